# baseline (speedup 1.0000x reference)
import functools

import jax
import jax.numpy as jnp
from jax import lax
from jax.experimental import pallas as pl
from jax.experimental.pallas import tpu as pltpu
from jax.experimental.pallas import tpu_sc as plsc

N = 10000
NM = 5000
EH = 128
NREL = 8
HID = 256
TOK = 768
BB, LE, LT = 16, 32, 64

NC, NS, LANE = 2, 16, 16
NW = NC * NS

NB_ENT = 10240
NB_MOV = 5120
NB_SEG = 80128
EP_KG = 327680
EP_C = 163840
EP_S = 81920
CHUNK = 128

_f32 = jnp.float32
_i32 = jnp.int32


def _mesh():
    return plsc.VectorSubcoreMesh(core_axis_name="c", subcore_axis_name="s")


_SC_PARAMS = pltpu.CompilerParams(needs_layout_passes=False)


def _fill_zero_2d(buf, nrows):
    def body(i, _):
        for c8 in range(EH // LANE):
            buf[i, pl.ds(c8 * LANE, LANE)] = jnp.zeros((LANE,), _f32)
        return 0
    lax.fori_loop(0, nrows, body, 0)


def _fill_const_1d(buf, n, val):
    def body(i, _):
        buf[pl.ds(i * LANE, LANE)] = jnp.full((LANE,), val, _f32)
        return 0
    lax.fori_loop(0, n // LANE, body, 0)


def _make_cnt_kernel():
    ZROWS = NB_SEG // NS

    @functools.partial(
        pl.kernel, mesh=_mesh(), compiler_params=_SC_PARAMS,
        out_type=(
            jax.ShapeDtypeStruct((NC * NB_SEG,), _f32),
            jax.ShapeDtypeStruct((NC * NB_ENT,), _f32),
            jax.ShapeDtypeStruct((NC * NB_MOV,), _f32),
            jax.ShapeDtypeStruct((NC * NB_MOV,), _f32),
        ),
        scratch_types=[
            pltpu.VMEM((ZROWS,), _f32),
            pltpu.VMEM((CHUNK,), _f32),
            pltpu.VMEM((CHUNK,), _i32),
            pltpu.VMEM((CHUNK,), _i32),
            pltpu.VMEM((CHUNK,), _i32),
            pltpu.VMEM_SHARED((NB_SEG,), _f32),
            pltpu.VMEM_SHARED((NB_ENT,), _f32),
            pltpu.VMEM_SHARED((NB_MOV,), _f32),
            pltpu.VMEM_SHARED((NB_MOV,), _f32),
        ],
    )
    def k(kg_dst, kg_et, c_col, t_col, i_col,
          out_seg, out_c, out_t, out_i,
          zbuf, ones, dstb, etb, segb, b_seg, b_c, b_t, b_i):
        cid = lax.axis_index("c")
        sid = lax.axis_index("s")
        wid = sid * NC + cid
        _fill_const_1d(zbuf, ZROWS, 0.0)
        _fill_const_1d(ones, CHUNK, 1.0)
        for bins, nb in ((b_seg, NB_SEG), (b_c, NB_ENT), (b_t, NB_MOV), (b_i, NB_MOV)):
            per = nb // NS
            pltpu.sync_copy(zbuf.at[pl.ds(0, per)], bins.at[pl.ds(sid * per, per)])
        plsc.subcore_barrier()

        def kg_body(t, _):
            eb = wid * (EP_KG // NW) + t * CHUNK
            pltpu.sync_copy(kg_dst.at[pl.ds(eb, CHUNK)], dstb)
            pltpu.sync_copy(kg_et.at[pl.ds(eb, CHUNK)], etb)
            for c8 in range(CHUNK // LANE):
                sl = pl.ds(c8 * LANE, LANE)
                segb[sl] = dstb[sl] * NREL + etb[sl]
            pltpu.sync_copy(ones, b_seg.at[segb], add=True)
            return 0
        lax.fori_loop(0, EP_KG // NW // CHUNK, kg_body, 0)

        def deg_body(col_hbm, bins, ep):
            def body(t, _):
                eb = wid * (ep // NW) + t * CHUNK
                pltpu.sync_copy(col_hbm.at[pl.ds(eb, CHUNK)], dstb)
                pltpu.sync_copy(ones, bins.at[dstb], add=True)
                return 0
            lax.fori_loop(0, ep // NW // CHUNK, body, 0)
        deg_body(c_col, b_c, EP_C)
        deg_body(t_col, b_t, EP_S)
        deg_body(i_col, b_i, EP_S)
        plsc.subcore_barrier()
        for bins, out, nb in ((b_seg, out_seg, NB_SEG), (b_c, out_c, NB_ENT),
                              (b_t, out_t, NB_MOV), (b_i, out_i, NB_MOV)):
            per = nb // NS
            pltpu.sync_copy(bins.at[pl.ds(sid * per, per)], zbuf.at[pl.ds(0, per)])
            pltpu.sync_copy(zbuf.at[pl.ds(0, per)],
                            out.at[pl.ds(cid * nb + sid * per, per)])
    return k


def _make_scale_kernel():
    ETILE = EP_KG // NW
    NCHUNK = ETILE // CHUNK

    @functools.partial(
        pl.kernel, mesh=_mesh(), compiler_params=_SC_PARAMS,
        out_type=(
            jax.ShapeDtypeStruct((EP_KG,), _f32),
            jax.ShapeDtypeStruct((EP_KG,), _i32),
        ),
        scratch_types=[
            pltpu.VMEM((NB_SEG,), _f32),
            pltpu.VMEM((CHUNK,), _i32),
            pltpu.VMEM((CHUNK,), _i32),
            pltpu.VMEM((CHUNK,), _i32),
            pltpu.VMEM((CHUNK,), _i32),
            pltpu.VMEM((CHUNK,), _f32),
        ],
    )
    def k(inv_hbm, src_hbm, dst_hbm, et_hbm, scale_out, gidx_out,
          invt, srcb, dstb, etb, gidxb, scaleb):
        cid = lax.axis_index("c")
        sid = lax.axis_index("s")
        wid = sid * NC + cid
        pltpu.sync_copy(inv_hbm, invt)

        def body(t, _):
            eb = wid * ETILE + t * CHUNK
            pltpu.sync_copy(src_hbm.at[pl.ds(eb, CHUNK)], srcb)
            pltpu.sync_copy(dst_hbm.at[pl.ds(eb, CHUNK)], dstb)
            pltpu.sync_copy(et_hbm.at[pl.ds(eb, CHUNK)], etb)
            for c8 in range(CHUNK // LANE):
                sl = pl.ds(c8 * LANE, LANE)
                e16 = etb[sl]
                gidxb[sl] = e16 * N + srcb[sl]
                scaleb[sl] = plsc.load_gather(invt, [dstb[sl] * NREL + e16])
            pltpu.sync_copy(scaleb, scale_out.at[pl.ds(eb, CHUNK)])
            pltpu.sync_copy(gidxb, gidx_out.at[pl.ds(eb, CHUNK)])
            return 0
        lax.fori_loop(0, NCHUNK, body, 0)
    return k


def _make_rgcn_kernel():
    ETILE = EP_KG // NW
    NCHUNK = ETILE // CHUNK
    RPT = NB_ENT // NS
    ZR = 64

    NH = NCHUNK // 2

    @functools.partial(
        pl.kernel, mesh=_mesh(), compiler_params=_SC_PARAMS,
        out_type=jax.ShapeDtypeStruct((NC, NB_ENT, EH), _f32),
        scratch_types=[
            pltpu.VMEM((ZR, EH), _f32),
            pltpu.VMEM((CHUNK,), _i32), pltpu.VMEM((CHUNK,), _i32),
            pltpu.VMEM((CHUNK,), _i32), pltpu.VMEM((CHUNK,), _i32),
            pltpu.VMEM((CHUNK,), _f32), pltpu.VMEM((CHUNK,), _f32),
            pltpu.VMEM((CHUNK, EH), _f32), pltpu.VMEM((CHUNK, EH), _f32),
            pltpu.VMEM_SHARED((NB_ENT, EH), _f32),
            pltpu.SemaphoreType.DMA, pltpu.SemaphoreType.DMA,
            pltpu.SemaphoreType.DMA, pltpu.SemaphoreType.DMA,
            pltpu.SemaphoreType.DMA, pltpu.SemaphoreType.DMA,
        ],
    )
    def k(xr_hbm, scale_hbm, gidx_hbm, dst_hbm, out_hbm,
          zbuf, dstb0, dstb1, gidxb0, gidxb1, scaleb0, scaleb1, rows0, rows1,
          bins, semi0, semi1, semg0, semg1, sems0, sems1):
        cid = lax.axis_index("c")
        sid = lax.axis_index("s")
        wid = sid * NC + cid
        base = wid * ETILE
        dstb = (dstb0, dstb1)
        gidxb = (gidxb0, gidxb1)
        scaleb = (scaleb0, scaleb1)
        rows = (rows0, rows1)
        semi = (semi0, semi1)
        semg = (semg0, semg1)
        _fill_zero_2d(zbuf, ZR)
        for j in range(RPT // ZR):
            pltpu.sync_copy(zbuf, bins.at[pl.ds(sid * RPT + j * ZR, ZR)])
        plsc.subcore_barrier()

        def fire_idx(t, b):
            eb = base + t * CHUNK
            pltpu.async_copy(dst_hbm.at[pl.ds(eb, CHUNK)], dstb[b], semi[b])
            pltpu.async_copy(gidx_hbm.at[pl.ds(eb, CHUNK)], gidxb[b], semi[b])
            pltpu.async_copy(scale_hbm.at[pl.ds(eb, CHUNK)], scaleb[b], semi[b])

        def wait_idx(b):
            pltpu.make_async_copy(dst_hbm.at[pl.ds(0, CHUNK)], dstb[b],
                                  semi[b]).wait()
            pltpu.make_async_copy(gidx_hbm.at[pl.ds(0, CHUNK)], gidxb[b],
                                  semi[b]).wait()
            pltpu.make_async_copy(scale_hbm.at[pl.ds(0, CHUNK)], scaleb[b],
                                  semi[b]).wait()

        def fire_gather(b):
            pltpu.async_copy(xr_hbm.at[gidxb[b]], rows[b], semg[b])

        def wait_gather(b):
            pltpu.make_async_copy(xr_hbm.at[gidxb[b]], rows[b], semg[b]).wait()

        def scale_rows(b):
            def mul_body(j, _):
                splat = plsc.load_gather(
                    scaleb[b], [jnp.broadcast_to(j, (LANE,))])
                for c8 in range(EH // LANE):
                    sl = pl.ds(c8 * LANE, LANE)
                    rows[b][j, sl] = rows[b][j, sl] * splat
                return 0
            lax.fori_loop(0, CHUNK, mul_body, 0)

        fire_idx(0, 0)
        fire_idx(1, 1)
        wait_idx(0)
        fire_gather(0)

        def body(i, _):
            wait_idx(1)
            fire_gather(1)
            wait_gather(0)
            scale_rows(0)
            d = pltpu.async_copy(rows0, bins.at[dstb0], sems0, add=True)
            d.wait()
            @pl.when(i < NH - 1)
            def _():
                fire_idx(2 * i + 2, 0)
                wait_idx(0)
                fire_gather(0)
            wait_gather(1)
            scale_rows(1)
            d1 = pltpu.async_copy(rows1, bins.at[dstb1], sems1, add=True)
            d1.wait()
            @pl.when(i < NH - 1)
            def _():
                fire_idx(2 * i + 3, 1)
            return 0
        lax.fori_loop(0, NH, body, 0)
        plsc.subcore_barrier()
        def flush(j, _):
            pltpu.sync_copy(bins.at[pl.ds(sid * RPT + j * ZR, ZR)], zbuf)
            pltpu.sync_copy(zbuf, out_hbm.at[cid, pl.ds(sid * RPT + j * ZR, ZR)])
            return 0
        lax.fori_loop(0, RPT // ZR, flush, 0)
    return k


def _make_scat_kernel(ep, nbins):
    ETILE = ep // NW
    NCHUNK = ETILE // CHUNK
    RPT = nbins // NS
    ZR = 64

    NH = NCHUNK // 2

    @functools.partial(
        pl.kernel, mesh=_mesh(), compiler_params=_SC_PARAMS,
        out_type=jax.ShapeDtypeStruct((NC, nbins, EH), _f32),
        scratch_types=[
            pltpu.VMEM((ZR, EH), _f32),
            pltpu.VMEM((CHUNK,), _i32), pltpu.VMEM((CHUNK,), _i32),
            pltpu.VMEM((CHUNK,), _i32), pltpu.VMEM((CHUNK,), _i32),
            pltpu.VMEM((CHUNK, EH), _f32), pltpu.VMEM((CHUNK, EH), _f32),
            pltpu.VMEM_SHARED((nbins, EH), _f32),
            pltpu.SemaphoreType.DMA, pltpu.SemaphoreType.DMA,
            pltpu.SemaphoreType.DMA, pltpu.SemaphoreType.DMA,
            pltpu.SemaphoreType.DMA, pltpu.SemaphoreType.DMA,
        ],
    )
    def k(y_hbm, row_hbm, col_hbm, out_hbm, zbuf, rowb0, rowb1, colb0, colb1,
          rows0, rows1, bins, semi0, semi1, semg0, semg1, sems0, sems1):
        cid = lax.axis_index("c")
        sid = lax.axis_index("s")
        wid = sid * NC + cid
        base = wid * ETILE
        rowb = (rowb0, rowb1)
        colb = (colb0, colb1)
        rows = (rows0, rows1)
        semi = (semi0, semi1)
        semg = (semg0, semg1)
        sems = (sems0, sems1)
        _fill_zero_2d(zbuf, ZR)
        for j in range(RPT // ZR):
            pltpu.sync_copy(zbuf, bins.at[pl.ds(sid * RPT + j * ZR, ZR)])
        plsc.subcore_barrier()

        def fire_idx(t, b):
            eb = base + t * CHUNK
            pltpu.async_copy(row_hbm.at[pl.ds(eb, CHUNK)], rowb[b], semi[b])
            pltpu.async_copy(col_hbm.at[pl.ds(eb, CHUNK)], colb[b], semi[b])

        def wait_idx(b):
            pltpu.make_async_copy(row_hbm.at[pl.ds(0, CHUNK)], rowb[b],
                                  semi[b]).wait()
            pltpu.make_async_copy(col_hbm.at[pl.ds(0, CHUNK)], colb[b],
                                  semi[b]).wait()

        def fire_gather(b):
            pltpu.async_copy(y_hbm.at[rowb[b]], rows[b], semg[b])

        def wait_gather(b):
            pltpu.make_async_copy(y_hbm.at[rowb[b]], rows[b], semg[b]).wait()

        fire_idx(0, 0)
        fire_idx(1, 1)
        wait_idx(0)
        fire_gather(0)

        def body(i, _):
            wait_idx(1)
            fire_gather(1)
            wait_gather(0)
            d = pltpu.async_copy(rows0, bins.at[colb0], sems0, add=True)
            d.wait()
            @pl.when(i < NH - 1)
            def _():
                fire_idx(2 * i + 2, 0)
            @pl.when(i < NH - 1)
            def _():
                wait_idx(0)
                fire_gather(0)
            wait_gather(1)
            d1 = pltpu.async_copy(rows1, bins.at[colb1], sems1, add=True)
            d1.wait()
            @pl.when(i < NH - 1)
            def _():
                fire_idx(2 * i + 3, 1)
            return 0
        lax.fori_loop(0, NH, body, 0)
        plsc.subcore_barrier()
        def flush(j, _):
            pltpu.sync_copy(bins.at[pl.ds(sid * RPT + j * ZR, ZR)], zbuf)
            pltpu.sync_copy(zbuf, out_hbm.at[cid, pl.ds(sid * RPT + j * ZR, ZR)])
            return 0
        lax.fori_loop(0, RPT // ZR, flush, 0)
    return k


def _make_gather_kernel(ni, feat):
    KPT = ni // NW
    TR = next(t for t in (128, 80, 64, 32, 16, 8) if KPT % t == 0 and t <= KPT)
    NT = KPT // TR

    @functools.partial(
        pl.kernel, mesh=_mesh(), compiler_params=_SC_PARAMS,
        out_type=jax.ShapeDtypeStruct((ni, feat), _f32),
        scratch_types=[
            pltpu.VMEM((KPT,), _i32),
            pltpu.VMEM((TR, feat), _f32),
            pltpu.SemaphoreType.DMA,
        ],
    )
    def k(table_hbm, idx_hbm, out_hbm, idxb, rows, sem):
        cid = lax.axis_index("c")
        sid = lax.axis_index("s")
        wid = sid * NC + cid
        base = wid * KPT
        pltpu.sync_copy(idx_hbm.at[pl.ds(base, KPT)], idxb)
        for j in range(NT):
            pltpu.async_copy(table_hbm.at[idxb.at[pl.ds(j * TR, TR)]], rows,
                             sem).wait()
            pltpu.sync_copy(rows, out_hbm.at[pl.ds(base + j * TR, TR)])
    return k


def _make_merge_kernel():
    KPT = NB_MOV // NW
    TR = 32
    NT = KPT // TR
    HALF = NB_ENT // 2
    TRASH = NB_ENT - 8

    @functools.partial(
        pl.kernel, mesh=_mesh(), compiler_params=_SC_PARAMS,
        out_type=jax.ShapeDtypeStruct((NB_ENT, EH), _f32),
        scratch_types=[
            pltpu.VMEM((KPT,), _i32),
            pltpu.VMEM((NT, TR), _i32),
            pltpu.VMEM((KPT, EH), _f32),
            pltpu.VMEM((TR, EH), _f32),
            pltpu.VMEM((80, EH), _f32),
            pltpu.SemaphoreType.DMA,
        ],
    )
    def k(entA_hbm, mean_hbm, mi_hbm, out_hbm, mib, rib, meanb, rows, cbuf, sem):
        cid = lax.axis_index("c")
        sid = lax.axis_index("s")
        wid = sid * NC + cid
        def cp(i, _):
            ci = sid + i * NS
            nch = jnp.where(cid == 0, 64, 61)
            @pl.when(ci < nch)
            def _():
                r0 = cid * HALF + ci * 80
                pltpu.sync_copy(entA_hbm.at[pl.ds(r0, 80)], cbuf)
                pltpu.sync_copy(cbuf, out_hbm.at[pl.ds(r0, 80)])
            return 0
        lax.fori_loop(0, 4, cp, 0)
        plsc.subcore_barrier()
        base = wid * KPT
        pltpu.sync_copy(mi_hbm.at[pl.ds(base, KPT)], mib)
        pltpu.sync_copy(mean_hbm.at[pl.ds(base, KPT)], meanb)
        for c8 in range(KPT // LANE):
            sl = pl.ds((c8 % (TR // LANE)) * LANE, LANE)
            m16 = mib[pl.ds(c8 * LANE, LANE)]
            mine = jnp.where(cid == 0, m16 < HALF, m16 >= HALF)
            rib[c8 // (TR // LANE), sl] = jnp.where(mine, m16, TRASH)
        for j in range(NT):
            pltpu.async_copy(entA_hbm.at[mib.at[pl.ds(j * TR, TR)]], rows,
                             sem).wait()
            def addrow(i, _):
                for c8 in range(EH // LANE):
                    sl = pl.ds(c8 * LANE, LANE)
                    rows[i, sl] = rows[i, sl] + meanb[j * TR + i, sl]
                return 0
            lax.fori_loop(0, TR, addrow, 0)
            pltpu.sync_copy(rows, out_hbm.at[rib.at[j]])
    return k


def _dot(a, b):
    return jnp.dot(a, b, preferred_element_type=_f32)


def _tc_xr(node, basis, comp):
    def body(comp_ref, basis_ref, x_ref, o_ref):
        r = pl.program_id(0)
        w = comp_ref[r, 0] * basis_ref[0]
        for b in range(1, NREL):
            w = w + comp_ref[r, b] * basis_ref[b]
        o_ref[0] = _dot(x_ref[...], w)
    return pl.pallas_call(
        body,
        grid=(NREL,),
        in_specs=[
            pl.BlockSpec((NREL, NREL), lambda r: (0, 0)),
            pl.BlockSpec((NREL, EH, EH), lambda r: (0, 0, 0)),
            pl.BlockSpec((N, EH), lambda r: (0, 0)),
        ],
        out_specs=pl.BlockSpec((1, N, EH), lambda r: (r, 0, 0)),
        out_shape=jax.ShapeDtypeStruct((NREL, N, EH), _f32),
    )(comp, basis, node)


def _tc_transform_counts(cnt_seg, cnt_c, cnt_t, cnt_i):
    def body(s_ref, c_ref, t_ref, i_ref, inv_ref, dc_ref, dt_ref, di_ref):
        s = s_ref[0] + s_ref[1]
        inv_ref[...] = 1.0 / jnp.maximum(s, 1.0)
        dc_ref[...] = lax.rsqrt(c_ref[0] + c_ref[1] + 1.0)
        dt_ref[...] = lax.rsqrt(t_ref[0] + t_ref[1] + 1.0)
        di_ref[...] = lax.rsqrt(i_ref[0] + i_ref[1] + 1.0)
    r = lambda a: a.reshape(NC, -1, 128)
    outs = pl.pallas_call(
        body,
        out_shape=(
            jax.ShapeDtypeStruct((NB_SEG // 128, 128), _f32),
            jax.ShapeDtypeStruct((NB_ENT // 128, 128), _f32),
            jax.ShapeDtypeStruct((NB_MOV // 128, 128), _f32),
            jax.ShapeDtypeStruct((NB_MOV // 128, 128), _f32),
        ),
    )(r(cnt_seg), r(cnt_c), r(cnt_t), r(cnt_i))
    inv, dc, dt, di = outs
    return (inv.reshape(-1), dc.reshape(-1)[:N], dt.reshape(-1)[:NM],
            di.reshape(-1)[:NM])


def _tc_ent0(node, p0, p1, root_w, root_b, dinv_c):
    def body(x_ref, p0_ref, p1_ref, w_ref, b_ref, d_ref, e_ref, y_ref):
        e = p0_ref[...] + p1_ref[...] + _dot(x_ref[...], w_ref[...]) \
            + b_ref[...] + x_ref[...]
        e_ref[...] = e
        y_ref[...] = e * d_ref[...]
    blk = 2000
    return pl.pallas_call(
        body,
        grid=(N // blk,),
        in_specs=[
            pl.BlockSpec((blk, EH), lambda i: (i, 0)),
            pl.BlockSpec((blk, EH), lambda i: (i, 0)),
            pl.BlockSpec((blk, EH), lambda i: (i, 0)),
            pl.BlockSpec((EH, EH), lambda i: (0, 0)),
            pl.BlockSpec((1, EH), lambda i: (0, 0)),
            pl.BlockSpec((blk, 1), lambda i: (i, 0)),
        ],
        out_specs=[pl.BlockSpec((blk, EH), lambda i: (i, 0))] * 2,
        out_shape=[jax.ShapeDtypeStruct((N, EH), _f32)] * 2,
    )(node, p0, p1, root_w, root_b[None], dinv_c)


def _tc_gcn_step(n, s0, s1, y, dinv2):
    def body(s0_ref, s1_ref, y_ref, d_ref, o_ref, yn_ref):
        d = d_ref[...]
        o = d * (s0_ref[...] + s1_ref[...] + y_ref[...])
        o_ref[...] = o
        yn_ref[...] = d * o
    blk = 2000 if n == N else 1000
    return pl.pallas_call(
        body,
        grid=(n // blk,),
        in_specs=[pl.BlockSpec((blk, EH), lambda i: (i, 0))] * 3
        + [pl.BlockSpec((blk, 1), lambda i: (i, 0))],
        out_specs=[pl.BlockSpec((blk, EH), lambda i: (i, 0))] * 2,
        out_shape=[jax.ShapeDtypeStruct((n, EH), _f32)] * 2,
    )(s0, s1, y, dinv2)


def _tc_combine4(n, s0, s1, y, dinv2, a, b, c):
    def body(s0_ref, s1_ref, y_ref, d_ref, a_ref, b_ref, c_ref, o_ref):
        last = d_ref[...] * (s0_ref[...] + s1_ref[...] + y_ref[...])
        o_ref[...] = (a_ref[...] + b_ref[...] + c_ref[...] + last) * 0.25
    blk = 2000 if n == N else 1000
    return pl.pallas_call(
        body,
        grid=(n // blk,),
        in_specs=[pl.BlockSpec((blk, EH), lambda i: (i, 0))] * 3
        + [pl.BlockSpec((blk, 1), lambda i: (i, 0))]
        + [pl.BlockSpec((blk, EH), lambda i: (i, 0))] * 3,
        out_specs=pl.BlockSpec((blk, EH), lambda i: (i, 0)),
        out_shape=jax.ShapeDtypeStruct((n, EH), _f32),
    )(s0, s1, y, dinv2, a, b, c)


def _tc_prep_ti(nf, dt2, di2):
    def body(x_ref, dt_ref, di_ref, yt_ref, yi_ref):
        yt_ref[...] = x_ref[...] * dt_ref[...]
        yi_ref[...] = x_ref[...] * di_ref[...]
    blk = 1000
    return pl.pallas_call(
        body,
        grid=(NM // blk,),
        in_specs=[pl.BlockSpec((blk, EH), lambda i: (i, 0)),
                  pl.BlockSpec((blk, 1), lambda i: (i, 0)),
                  pl.BlockSpec((blk, 1), lambda i: (i, 0))],
        out_specs=[pl.BlockSpec((blk, EH), lambda i: (i, 0))] * 2,
        out_shape=[jax.ShapeDtypeStruct((NM, EH), _f32)] * 2,
    )(nf, dt2, di2)


def _tc_ent_mlp(entA, w1, b1, w2, b2, ew, eb):
    def body(x_ref, w1_ref, b1_ref, w2_ref, b2_ref, ew_ref, eb_ref, o_ref):
        x = x_ref[...]
        h = jnp.maximum(_dot(x, w1_ref[...]) + b1_ref[...], 0.0)
        m = _dot(h, w2_ref[...]) + b2_ref[...] + x
        o_ref[...] = _dot(m, ew_ref[...]) + eb_ref[...]
    blk = 2000
    return pl.pallas_call(
        body,
        grid=(N // blk,),
        in_specs=[
            pl.BlockSpec((blk, EH), lambda i: (i, 0)),
            pl.BlockSpec((EH, EH // 2), lambda i: (0, 0)),
            pl.BlockSpec((1, EH // 2), lambda i: (0, 0)),
            pl.BlockSpec((EH // 2, EH), lambda i: (0, 0)),
            pl.BlockSpec((1, EH), lambda i: (0, 0)),
            pl.BlockSpec((EH, HID), lambda i: (0, 0)),
            pl.BlockSpec((1, HID), lambda i: (0, 0)),
        ],
        out_specs=pl.BlockSpec((blk, HID), lambda i: (i, 0)),
        out_shape=jax.ShapeDtypeStruct((N, HID), _f32),
    )(entA, w1, b1[None], w2, b2[None], ew, eb[None])


def _tc_tok_mlp(tok, w1, b1, w2, b2, tw, tb):
    def body(t_ref, w1_ref, b1_ref, w2_ref, b2_ref, tw_ref, tb_ref, o_ref):
        t = t_ref[0]
        h = jnp.maximum(_dot(t, w1_ref[...]) + b1_ref[...], 0.0)
        m = _dot(h, w2_ref[...]) + b2_ref[...] + t
        o_ref[0] = _dot(m, tw_ref[...]) + tb_ref[...]
    return pl.pallas_call(
        body,
        grid=(BB,),
        in_specs=[
            pl.BlockSpec((1, LT, TOK), lambda i: (i, 0, 0)),
            pl.BlockSpec((TOK, TOK // 2), lambda i: (0, 0)),
            pl.BlockSpec((1, TOK // 2), lambda i: (0, 0)),
            pl.BlockSpec((TOK // 2, TOK), lambda i: (0, 0)),
            pl.BlockSpec((1, TOK), lambda i: (0, 0)),
            pl.BlockSpec((TOK, HID), lambda i: (0, 0)),
            pl.BlockSpec((1, HID), lambda i: (0, 0)),
        ],
        out_specs=pl.BlockSpec((1, LT, HID), lambda i: (i, 0, 0)),
        out_shape=jax.ShapeDtypeStruct((BB, LT, HID), _f32),
    )(tok, w1, b1[None], w2, b2[None], tw, tb[None])


def _tc_attn(tt, e, cross_w):
    def body(t_ref, e_ref, w_ref, o_ref):
        t = t_ref[0]
        e2 = e_ref[0]
        q = _dot(t, w_ref[...])
        a = lax.dot_general(q, e2, (((1,), (1,)), ((), ())),
                            preferred_element_type=_f32) * (1.0 / HID)
        a = a - jnp.max(a, axis=1, keepdims=True)
        ex = jnp.exp(a)
        ew = ex / jnp.sum(ex, axis=1, keepdims=True)
        o_ref[0] = _dot(ew, e2) + t
    return pl.pallas_call(
        body,
        grid=(BB,),
        in_specs=[
            pl.BlockSpec((1, LT, HID), lambda i: (i, 0, 0)),
            pl.BlockSpec((1, LE, HID), lambda i: (i, 0, 0)),
            pl.BlockSpec((HID, HID), lambda i: (0, 0)),
        ],
        out_specs=pl.BlockSpec((1, LT, HID), lambda i: (i, 0, 0)),
        out_shape=jax.ShapeDtypeStruct((BB, LT, HID), _f32),
    )(tt, e, cross_w)


def _pad1(a, n, val):
    return jnp.concatenate([a, jnp.full((n - a.shape[0],), val, a.dtype)])


def _pad_spread(a, n, lo, nspread):
    npad = n - a.shape[0]
    pad = lo + (jnp.arange(npad, dtype=a.dtype) % nspread)
    return jnp.concatenate([a, pad])


def kernel(node_embeds, basis, comp, root_w, root_b, ep1_w1, ep1_b1, ep1_w2,
           ep1_b2, ep2_w, ep2_b, tp1_w1, tp1_b1, tp1_w2, tp1_b2, tp2_w, tp2_b,
           cross_w, token_embeds, entity_ids, edge_index, edge_type,
           edge_index_c, edge_index_t_s, edge_index_i_s, movie_indices):
    kg_src = _pad1(edge_index[0], EP_KG, 0)
    kg_dst = _pad_spread(edge_index[1], EP_KG, N, 16)
    kg_et = _pad1(edge_type, EP_KG, 0)
    c_row = _pad1(edge_index_c[0], EP_C, 0)
    c_col = _pad_spread(edge_index_c[1], EP_C, N, NB_ENT - N)
    t_row = _pad1(edge_index_t_s[0], EP_S, 0)
    t_col = _pad_spread(edge_index_t_s[1], EP_S, NM, NB_MOV - NM)
    i_row = _pad1(edge_index_i_s[0], EP_S, 0)
    i_col = _pad_spread(edge_index_i_s[1], EP_S, NM, NB_MOV - NM)
    mi_pad = _pad1(movie_indices, NB_MOV, 0)
    eids = entity_ids.reshape(-1)

    cnt_seg, cnt_c, cnt_t, cnt_i = _make_cnt_kernel()(
        kg_dst, kg_et, c_col, t_col, i_col)
    inv, dinv_c, dinv_t, dinv_i = _tc_transform_counts(
        cnt_seg, cnt_c, cnt_t, cnt_i)
    dc2, dt2, di2 = dinv_c[:, None], dinv_t[:, None], dinv_i[:, None]

    xr = _tc_xr(node_embeds, basis, comp).reshape(NREL * N, EH)
    scale, gidx = _make_scale_kernel()(inv, kg_src, kg_dst, kg_et)
    rg = _make_rgcn_kernel()(xr, scale, gidx, kg_dst)
    ent0, y1c = _tc_ent0(node_embeds, rg[0, :N], rg[1, :N], root_w, root_b, dc2)

    scat_c = _make_scat_kernel(EP_C, NB_ENT)
    s1 = scat_c(y1c, c_row, c_col)
    c1, y2c = _tc_gcn_step(N, s1[0, :N], s1[1, :N], y1c, dc2)
    s2 = scat_c(y2c, c_row, c_col)
    c2, y3c = _tc_gcn_step(N, s2[0, :N], s2[1, :N], y2c, dc2)
    s3 = scat_c(y3c, c_row, c_col)
    entA = _tc_combine4(N, s3[0, :N], s3[1, :N], y3c, dc2, c1, c2, ent0)

    nf = _make_gather_kernel(NB_MOV, EH)(ent0, mi_pad)[:NM]
    y1t, y1i = _tc_prep_ti(nf, dt2, di2)
    scat_s = _make_scat_kernel(EP_S, NB_MOV)
    st1 = scat_s(y1t, t_row, t_col)
    t1, y2t = _tc_gcn_step(NM, st1[0, :NM], st1[1, :NM], y1t, dt2)
    st2 = scat_s(y2t, t_row, t_col)
    t2, _ = _tc_gcn_step(NM, st2[0, :NM], st2[1, :NM], y2t, dt2)
    si1 = scat_s(y1i, i_row, i_col)
    i1, y2i = _tc_gcn_step(NM, si1[0, :NM], si1[1, :NM], y1i, di2)
    si2 = scat_s(y2i, i_row, i_col)
    mean = _tc_combine4(NM, si2[0, :NM], si2[1, :NM], y2i, di2, t1, t2, i1)

    mean_pad = jnp.concatenate(
        [mean, jnp.zeros((NB_MOV - NM, EH), _f32)], axis=0)
    entA2 = _make_merge_kernel()(entA, mean_pad, mi_pad)[:N]
    entF = _tc_ent_mlp(entA2, ep1_w1, ep1_b1, ep1_w2, ep1_b2, ep2_w, ep2_b)

    e = _make_gather_kernel(BB * LE, HID)(entF, eids).reshape(BB, LE, HID)
    tt = _tc_tok_mlp(token_embeds, tp1_w1, tp1_b1, tp1_w2, tp1_b2, tp2_w, tp2_b)
    return _tc_attn(tt, e, cross_w)


# speedup vs baseline: 1.0964x; 1.0964x over previous
import functools

import jax
import jax.numpy as jnp
from jax import lax
from jax.experimental import pallas as pl
from jax.experimental.pallas import tpu as pltpu
from jax.experimental.pallas import tpu_sc as plsc

N = 10000
NM = 5000
EH = 128
NREL = 8
HID = 256
TOK = 768
BB, LE, LT = 16, 32, 64

NC, NS, LANE = 2, 16, 16
NW = NC * NS

NB_ENT = 10240
NB_MOV = 5120
NB_SEG = 80128
EP_KG = 327680
EP_C = 163840
EP_S = 81920
CHUNK = 128

_f32 = jnp.float32
_i32 = jnp.int32


def _mesh():
    return plsc.VectorSubcoreMesh(core_axis_name="c", subcore_axis_name="s")


_SC_PARAMS = pltpu.CompilerParams(needs_layout_passes=False)


def _fill_zero_2d(buf, nrows):
    def body(i, _):
        for c8 in range(EH // LANE):
            buf[i, pl.ds(c8 * LANE, LANE)] = jnp.zeros((LANE,), _f32)
        return 0
    lax.fori_loop(0, nrows, body, 0)


def _fill_const_1d(buf, n, val):
    def body(i, _):
        buf[pl.ds(i * LANE, LANE)] = jnp.full((LANE,), val, _f32)
        return 0
    lax.fori_loop(0, n // LANE, body, 0)


def _make_cnt_kernel():
    ZROWS = NB_SEG // NS

    @functools.partial(
        pl.kernel, mesh=_mesh(), compiler_params=_SC_PARAMS,
        out_type=(
            jax.ShapeDtypeStruct((NC * NB_SEG,), _f32),
            jax.ShapeDtypeStruct((NC * NB_ENT,), _f32),
            jax.ShapeDtypeStruct((NC * NB_MOV,), _f32),
            jax.ShapeDtypeStruct((NC * NB_MOV,), _f32),
        ),
        scratch_types=[
            pltpu.VMEM((ZROWS,), _f32),
            pltpu.VMEM((CHUNK,), _f32),
            pltpu.VMEM((CHUNK,), _i32),
            pltpu.VMEM((CHUNK,), _i32),
            pltpu.VMEM((CHUNK,), _i32),
            pltpu.VMEM_SHARED((NB_SEG,), _f32),
            pltpu.VMEM_SHARED((NB_ENT,), _f32),
            pltpu.VMEM_SHARED((NB_MOV,), _f32),
            pltpu.VMEM_SHARED((NB_MOV,), _f32),
        ],
    )
    def k(kg_dst, kg_et, c_col, t_col, i_col,
          out_seg, out_c, out_t, out_i,
          zbuf, ones, dstb, etb, segb, b_seg, b_c, b_t, b_i):
        cid = lax.axis_index("c")
        sid = lax.axis_index("s")
        wid = sid * NC + cid
        _fill_const_1d(zbuf, ZROWS, 0.0)
        _fill_const_1d(ones, CHUNK, 1.0)
        for bins, nb in ((b_seg, NB_SEG), (b_c, NB_ENT), (b_t, NB_MOV), (b_i, NB_MOV)):
            per = nb // NS
            pltpu.sync_copy(zbuf.at[pl.ds(0, per)], bins.at[pl.ds(sid * per, per)])
        plsc.subcore_barrier()

        def kg_body(t, _):
            eb = wid * (EP_KG // NW) + t * CHUNK
            pltpu.sync_copy(kg_dst.at[pl.ds(eb, CHUNK)], dstb)
            pltpu.sync_copy(kg_et.at[pl.ds(eb, CHUNK)], etb)
            for c8 in range(CHUNK // LANE):
                sl = pl.ds(c8 * LANE, LANE)
                segb[sl] = dstb[sl] * NREL + etb[sl]
            pltpu.sync_copy(ones, b_seg.at[segb], add=True)
            return 0
        lax.fori_loop(0, EP_KG // NW // CHUNK, kg_body, 0)

        def deg_body(col_hbm, bins, ep):
            def body(t, _):
                eb = wid * (ep // NW) + t * CHUNK
                pltpu.sync_copy(col_hbm.at[pl.ds(eb, CHUNK)], dstb)
                pltpu.sync_copy(ones, bins.at[dstb], add=True)
                return 0
            lax.fori_loop(0, ep // NW // CHUNK, body, 0)
        deg_body(c_col, b_c, EP_C)
        deg_body(t_col, b_t, EP_S)
        deg_body(i_col, b_i, EP_S)
        plsc.subcore_barrier()
        for bins, out, nb in ((b_seg, out_seg, NB_SEG), (b_c, out_c, NB_ENT),
                              (b_t, out_t, NB_MOV), (b_i, out_i, NB_MOV)):
            per = nb // NS
            pltpu.sync_copy(bins.at[pl.ds(sid * per, per)], zbuf.at[pl.ds(0, per)])
            pltpu.sync_copy(zbuf.at[pl.ds(0, per)],
                            out.at[pl.ds(cid * nb + sid * per, per)])
    return k


def _make_scale_kernel():
    ETILE = EP_KG // NW
    NCHUNK = ETILE // CHUNK

    @functools.partial(
        pl.kernel, mesh=_mesh(), compiler_params=_SC_PARAMS,
        out_type=(
            jax.ShapeDtypeStruct((EP_KG,), _f32),
            jax.ShapeDtypeStruct((EP_KG,), _i32),
        ),
        scratch_types=[
            pltpu.VMEM((NB_SEG,), _f32),
            pltpu.VMEM((CHUNK,), _i32),
            pltpu.VMEM((CHUNK,), _i32),
            pltpu.VMEM((CHUNK,), _i32),
            pltpu.VMEM((CHUNK,), _i32),
            pltpu.VMEM((CHUNK,), _f32),
        ],
    )
    def k(inv_hbm, src_hbm, dst_hbm, et_hbm, scale_out, gidx_out,
          invt, srcb, dstb, etb, gidxb, scaleb):
        cid = lax.axis_index("c")
        sid = lax.axis_index("s")
        wid = sid * NC + cid
        pltpu.sync_copy(inv_hbm, invt)

        def body(t, _):
            eb = wid * ETILE + t * CHUNK
            pltpu.sync_copy(src_hbm.at[pl.ds(eb, CHUNK)], srcb)
            pltpu.sync_copy(dst_hbm.at[pl.ds(eb, CHUNK)], dstb)
            pltpu.sync_copy(et_hbm.at[pl.ds(eb, CHUNK)], etb)
            for c8 in range(CHUNK // LANE):
                sl = pl.ds(c8 * LANE, LANE)
                e16 = etb[sl]
                gidxb[sl] = e16 * N + srcb[sl]
                scaleb[sl] = plsc.load_gather(invt, [dstb[sl] * NREL + e16])
            pltpu.sync_copy(scaleb, scale_out.at[pl.ds(eb, CHUNK)])
            pltpu.sync_copy(gidxb, gidx_out.at[pl.ds(eb, CHUNK)])
            return 0
        lax.fori_loop(0, NCHUNK, body, 0)
    return k


def _make_rgcn_kernel():
    ETILE = EP_KG // NW
    NCHUNK = ETILE // CHUNK
    RPT = NB_ENT // NS
    ZR = 64

    NH = NCHUNK // 2

    @functools.partial(
        pl.kernel, mesh=_mesh(), compiler_params=_SC_PARAMS,
        out_type=jax.ShapeDtypeStruct((NC, NB_ENT, EH), _f32),
        scratch_types=[
            pltpu.VMEM((ZR, EH), _f32),
            pltpu.VMEM((CHUNK,), _i32), pltpu.VMEM((CHUNK,), _i32),
            pltpu.VMEM((CHUNK,), _i32), pltpu.VMEM((CHUNK,), _i32),
            pltpu.VMEM((CHUNK,), _f32), pltpu.VMEM((CHUNK,), _f32),
            pltpu.VMEM((CHUNK, EH), _f32), pltpu.VMEM((CHUNK, EH), _f32),
            pltpu.VMEM_SHARED((NB_ENT, EH), _f32),
            pltpu.SemaphoreType.DMA, pltpu.SemaphoreType.DMA,
            pltpu.SemaphoreType.DMA, pltpu.SemaphoreType.DMA,
            pltpu.SemaphoreType.DMA, pltpu.SemaphoreType.DMA,
        ],
    )
    def k(xr_hbm, scale_hbm, gidx_hbm, dst_hbm, out_hbm,
          zbuf, dstb0, dstb1, gidxb0, gidxb1, scaleb0, scaleb1, rows0, rows1,
          bins, semi0, semi1, semg0, semg1, sems0, sems1):
        cid = lax.axis_index("c")
        sid = lax.axis_index("s")
        wid = sid * NC + cid
        base = wid * ETILE
        dstb = (dstb0, dstb1)
        gidxb = (gidxb0, gidxb1)
        scaleb = (scaleb0, scaleb1)
        rows = (rows0, rows1)
        semi = (semi0, semi1)
        semg = (semg0, semg1)
        _fill_zero_2d(zbuf, ZR)
        for j in range(RPT // ZR):
            pltpu.sync_copy(zbuf, bins.at[pl.ds(sid * RPT + j * ZR, ZR)])
        plsc.subcore_barrier()

        def fire_idx(t, b):
            eb = base + t * CHUNK
            pltpu.async_copy(dst_hbm.at[pl.ds(eb, CHUNK)], dstb[b], semi[b])
            pltpu.async_copy(gidx_hbm.at[pl.ds(eb, CHUNK)], gidxb[b], semi[b])
            pltpu.async_copy(scale_hbm.at[pl.ds(eb, CHUNK)], scaleb[b], semi[b])

        def wait_idx(b):
            pltpu.make_async_copy(dst_hbm.at[pl.ds(0, CHUNK)], dstb[b],
                                  semi[b]).wait()
            pltpu.make_async_copy(gidx_hbm.at[pl.ds(0, CHUNK)], gidxb[b],
                                  semi[b]).wait()
            pltpu.make_async_copy(scale_hbm.at[pl.ds(0, CHUNK)], scaleb[b],
                                  semi[b]).wait()

        def fire_gather(b):
            pltpu.async_copy(xr_hbm.at[gidxb[b]], rows[b], semg[b])

        def wait_gather(b):
            pltpu.make_async_copy(xr_hbm.at[gidxb[b]], rows[b], semg[b]).wait()

        def scale_rows(b):
            def mul_body(j, _):
                splat = plsc.load_gather(
                    scaleb[b], [jnp.broadcast_to(j, (LANE,))])
                for c8 in range(EH // LANE):
                    sl = pl.ds(c8 * LANE, LANE)
                    rows[b][j, sl] = rows[b][j, sl] * splat
                return 0
            lax.fori_loop(0, CHUNK, mul_body, 0)

        fire_idx(0, 0)
        fire_idx(1, 1)
        wait_idx(0)
        fire_gather(0)

        def body(i, _):
            wait_idx(1)
            fire_gather(1)
            wait_gather(0)
            scale_rows(0)
            d = pltpu.async_copy(rows0, bins.at[dstb0], sems0, add=True)
            d.wait()
            @pl.when(i < NH - 1)
            def _():
                fire_idx(2 * i + 2, 0)
                wait_idx(0)
                fire_gather(0)
            wait_gather(1)
            scale_rows(1)
            d1 = pltpu.async_copy(rows1, bins.at[dstb1], sems1, add=True)
            d1.wait()
            @pl.when(i < NH - 1)
            def _():
                fire_idx(2 * i + 3, 1)
            return 0
        lax.fori_loop(0, NH, body, 0)
        plsc.subcore_barrier()
        def flush(j, _):
            pltpu.sync_copy(bins.at[pl.ds(sid * RPT + j * ZR, ZR)], zbuf)
            pltpu.sync_copy(zbuf, out_hbm.at[cid, pl.ds(sid * RPT + j * ZR, ZR)])
            return 0
        lax.fori_loop(0, RPT // ZR, flush, 0)
    return k


def _make_scat_kernel(ep, nbins):
    ETILE = ep // NW
    NCHUNK = ETILE // CHUNK
    RPT = nbins // NS
    ZR = 64

    NH = NCHUNK // 2

    @functools.partial(
        pl.kernel, mesh=_mesh(), compiler_params=_SC_PARAMS,
        out_type=jax.ShapeDtypeStruct((NC, nbins, EH), _f32),
        scratch_types=[
            pltpu.VMEM((ZR, EH), _f32),
            pltpu.VMEM((CHUNK,), _i32), pltpu.VMEM((CHUNK,), _i32),
            pltpu.VMEM((CHUNK,), _i32), pltpu.VMEM((CHUNK,), _i32),
            pltpu.VMEM((CHUNK, EH), _f32), pltpu.VMEM((CHUNK, EH), _f32),
            pltpu.VMEM_SHARED((nbins, EH), _f32),
            pltpu.SemaphoreType.DMA, pltpu.SemaphoreType.DMA,
            pltpu.SemaphoreType.DMA, pltpu.SemaphoreType.DMA,
            pltpu.SemaphoreType.DMA, pltpu.SemaphoreType.DMA,
        ],
    )
    def k(y_hbm, row_hbm, col_hbm, out_hbm, zbuf, rowb0, rowb1, colb0, colb1,
          rows0, rows1, bins, semi0, semi1, semg0, semg1, sems0, sems1):
        cid = lax.axis_index("c")
        sid = lax.axis_index("s")
        wid = sid * NC + cid
        base = wid * ETILE
        rowb = (rowb0, rowb1)
        colb = (colb0, colb1)
        rows = (rows0, rows1)
        semi = (semi0, semi1)
        semg = (semg0, semg1)
        sems = (sems0, sems1)
        _fill_zero_2d(zbuf, ZR)
        for j in range(RPT // ZR):
            pltpu.sync_copy(zbuf, bins.at[pl.ds(sid * RPT + j * ZR, ZR)])
        plsc.subcore_barrier()

        def fire_idx(t, b):
            eb = (wid + t * NW) * CHUNK
            pltpu.async_copy(row_hbm.at[pl.ds(eb, CHUNK)], rowb[b], semi[b])
            pltpu.async_copy(col_hbm.at[pl.ds(eb, CHUNK)], colb[b], semi[b])

        def wait_idx(b):
            pltpu.make_async_copy(row_hbm.at[pl.ds(0, CHUNK)], rowb[b],
                                  semi[b]).wait()
            pltpu.make_async_copy(col_hbm.at[pl.ds(0, CHUNK)], colb[b],
                                  semi[b]).wait()

        def fire_gather(b):
            pltpu.async_copy(y_hbm.at[rowb[b]], rows[b], semg[b])

        def wait_gather(b):
            pltpu.make_async_copy(y_hbm.at[rowb[b]], rows[b], semg[b]).wait()

        fire_idx(0, 0)
        fire_idx(1, 1)
        wait_idx(0)
        fire_gather(0)

        def body(i, _):
            wait_idx(1)
            fire_gather(1)
            wait_gather(0)
            d = pltpu.async_copy(rows0, bins.at[colb0], sems0, add=True)
            d.wait()
            @pl.when(i < NH - 1)
            def _():
                fire_idx(2 * i + 2, 0)
            @pl.when(i < NH - 1)
            def _():
                wait_idx(0)
                fire_gather(0)
            wait_gather(1)
            d1 = pltpu.async_copy(rows1, bins.at[colb1], sems1, add=True)
            d1.wait()
            @pl.when(i < NH - 1)
            def _():
                fire_idx(2 * i + 3, 1)
            return 0
        lax.fori_loop(0, NH, body, 0)
        plsc.subcore_barrier()
        def flush(j, _):
            pltpu.sync_copy(bins.at[pl.ds(sid * RPT + j * ZR, ZR)], zbuf)
            pltpu.sync_copy(zbuf, out_hbm.at[cid, pl.ds(sid * RPT + j * ZR, ZR)])
            return 0
        lax.fori_loop(0, RPT // ZR, flush, 0)
    return k


def _make_gather_kernel(ni, feat):
    KPT = ni // NW
    TR = next(t for t in (128, 80, 64, 32, 16, 8) if KPT % t == 0 and t <= KPT)
    NT = KPT // TR

    @functools.partial(
        pl.kernel, mesh=_mesh(), compiler_params=_SC_PARAMS,
        out_type=jax.ShapeDtypeStruct((ni, feat), _f32),
        scratch_types=[
            pltpu.VMEM((KPT,), _i32),
            pltpu.VMEM((TR, feat), _f32),
            pltpu.SemaphoreType.DMA,
        ],
    )
    def k(table_hbm, idx_hbm, out_hbm, idxb, rows, sem):
        cid = lax.axis_index("c")
        sid = lax.axis_index("s")
        wid = sid * NC + cid
        base = wid * KPT
        pltpu.sync_copy(idx_hbm.at[pl.ds(base, KPT)], idxb)
        for j in range(NT):
            pltpu.async_copy(table_hbm.at[idxb.at[pl.ds(j * TR, TR)]], rows,
                             sem).wait()
            pltpu.sync_copy(rows, out_hbm.at[pl.ds(base + j * TR, TR)])
    return k


def _make_merge_kernel():
    KPT = NB_MOV // NW
    TR = 32
    NT = KPT // TR
    HALF = NB_ENT // 2
    TRASH = NB_ENT - 8

    @functools.partial(
        pl.kernel, mesh=_mesh(), compiler_params=_SC_PARAMS,
        out_type=jax.ShapeDtypeStruct((NB_ENT, EH), _f32),
        scratch_types=[
            pltpu.VMEM((KPT,), _i32),
            pltpu.VMEM((NT, TR), _i32),
            pltpu.VMEM((KPT, EH), _f32),
            pltpu.VMEM((TR, EH), _f32),
            pltpu.VMEM((80, EH), _f32),
            pltpu.SemaphoreType.DMA,
        ],
    )
    def k(entA_hbm, mean_hbm, mi_hbm, out_hbm, mib, rib, meanb, rows, cbuf, sem):
        cid = lax.axis_index("c")
        sid = lax.axis_index("s")
        wid = sid * NC + cid
        def cp(i, _):
            ci = sid + i * NS
            nch = jnp.where(cid == 0, 64, 61)
            @pl.when(ci < nch)
            def _():
                r0 = cid * HALF + ci * 80
                pltpu.sync_copy(entA_hbm.at[pl.ds(r0, 80)], cbuf)
                pltpu.sync_copy(cbuf, out_hbm.at[pl.ds(r0, 80)])
            return 0
        lax.fori_loop(0, 4, cp, 0)
        plsc.subcore_barrier()
        base = wid * KPT
        pltpu.sync_copy(mi_hbm.at[pl.ds(base, KPT)], mib)
        pltpu.sync_copy(mean_hbm.at[pl.ds(base, KPT)], meanb)
        for c8 in range(KPT // LANE):
            sl = pl.ds((c8 % (TR // LANE)) * LANE, LANE)
            m16 = mib[pl.ds(c8 * LANE, LANE)]
            mine = jnp.where(cid == 0, m16 < HALF, m16 >= HALF)
            rib[c8 // (TR // LANE), sl] = jnp.where(mine, m16, TRASH)
        for j in range(NT):
            pltpu.async_copy(entA_hbm.at[mib.at[pl.ds(j * TR, TR)]], rows,
                             sem).wait()
            def addrow(i, _):
                for c8 in range(EH // LANE):
                    sl = pl.ds(c8 * LANE, LANE)
                    rows[i, sl] = rows[i, sl] + meanb[j * TR + i, sl]
                return 0
            lax.fori_loop(0, TR, addrow, 0)
            pltpu.sync_copy(rows, out_hbm.at[rib.at[j]])
    return k


def _dot(a, b):
    return jnp.dot(a, b, preferred_element_type=_f32)


def _tc_xr(node, basis, comp):
    def body(comp_ref, basis_ref, x_ref, o_ref):
        r = pl.program_id(0)
        w = comp_ref[r, 0] * basis_ref[0]
        for b in range(1, NREL):
            w = w + comp_ref[r, b] * basis_ref[b]
        o_ref[0] = _dot(x_ref[...], w)
    return pl.pallas_call(
        body,
        grid=(NREL,),
        in_specs=[
            pl.BlockSpec((NREL, NREL), lambda r: (0, 0)),
            pl.BlockSpec((NREL, EH, EH), lambda r: (0, 0, 0)),
            pl.BlockSpec((N, EH), lambda r: (0, 0)),
        ],
        out_specs=pl.BlockSpec((1, N, EH), lambda r: (r, 0, 0)),
        out_shape=jax.ShapeDtypeStruct((NREL, N, EH), _f32),
    )(comp, basis, node)


def _tc_transform_counts(cnt_seg, cnt_c, cnt_t, cnt_i):
    def body(s_ref, c_ref, t_ref, i_ref, inv_ref, dc_ref, dt_ref, di_ref):
        s = s_ref[0] + s_ref[1]
        inv_ref[...] = 1.0 / jnp.maximum(s, 1.0)
        dc_ref[...] = lax.rsqrt(c_ref[0] + c_ref[1] + 1.0)
        dt_ref[...] = lax.rsqrt(t_ref[0] + t_ref[1] + 1.0)
        di_ref[...] = lax.rsqrt(i_ref[0] + i_ref[1] + 1.0)
    r = lambda a: a.reshape(NC, -1, 128)
    outs = pl.pallas_call(
        body,
        out_shape=(
            jax.ShapeDtypeStruct((NB_SEG // 128, 128), _f32),
            jax.ShapeDtypeStruct((NB_ENT // 128, 128), _f32),
            jax.ShapeDtypeStruct((NB_MOV // 128, 128), _f32),
            jax.ShapeDtypeStruct((NB_MOV // 128, 128), _f32),
        ),
    )(r(cnt_seg), r(cnt_c), r(cnt_t), r(cnt_i))
    inv, dc, dt, di = outs
    return (inv.reshape(-1), dc.reshape(-1)[:N], dt.reshape(-1)[:NM],
            di.reshape(-1)[:NM])


def _tc_ent0(node, p0, p1, root_w, root_b, dinv_c):
    def body(x_ref, p0_ref, p1_ref, w_ref, b_ref, d_ref, e_ref, y_ref):
        e = p0_ref[...] + p1_ref[...] + _dot(x_ref[...], w_ref[...]) \
            + b_ref[...] + x_ref[...]
        e_ref[...] = e
        y_ref[...] = e * d_ref[...]
    blk = 2000
    return pl.pallas_call(
        body,
        grid=(N // blk,),
        in_specs=[
            pl.BlockSpec((blk, EH), lambda i: (i, 0)),
            pl.BlockSpec((blk, EH), lambda i: (i, 0)),
            pl.BlockSpec((blk, EH), lambda i: (i, 0)),
            pl.BlockSpec((EH, EH), lambda i: (0, 0)),
            pl.BlockSpec((1, EH), lambda i: (0, 0)),
            pl.BlockSpec((blk, 1), lambda i: (i, 0)),
        ],
        out_specs=[pl.BlockSpec((blk, EH), lambda i: (i, 0))] * 2,
        out_shape=[jax.ShapeDtypeStruct((N, EH), _f32)] * 2,
    )(node, p0, p1, root_w, root_b[None], dinv_c)


def _tc_gcn_step(n, s0, s1, y, dinv2):
    def body(s0_ref, s1_ref, y_ref, d_ref, o_ref, yn_ref):
        d = d_ref[...]
        o = d * (s0_ref[...] + s1_ref[...] + y_ref[...])
        o_ref[...] = o
        yn_ref[...] = d * o
    blk = 2000 if n == N else 1000
    return pl.pallas_call(
        body,
        grid=(n // blk,),
        in_specs=[pl.BlockSpec((blk, EH), lambda i: (i, 0))] * 3
        + [pl.BlockSpec((blk, 1), lambda i: (i, 0))],
        out_specs=[pl.BlockSpec((blk, EH), lambda i: (i, 0))] * 2,
        out_shape=[jax.ShapeDtypeStruct((n, EH), _f32)] * 2,
    )(s0, s1, y, dinv2)


def _tc_combine4(n, s0, s1, y, dinv2, a, b, c):
    def body(s0_ref, s1_ref, y_ref, d_ref, a_ref, b_ref, c_ref, o_ref):
        last = d_ref[...] * (s0_ref[...] + s1_ref[...] + y_ref[...])
        o_ref[...] = (a_ref[...] + b_ref[...] + c_ref[...] + last) * 0.25
    blk = 2000 if n == N else 1000
    return pl.pallas_call(
        body,
        grid=(n // blk,),
        in_specs=[pl.BlockSpec((blk, EH), lambda i: (i, 0))] * 3
        + [pl.BlockSpec((blk, 1), lambda i: (i, 0))]
        + [pl.BlockSpec((blk, EH), lambda i: (i, 0))] * 3,
        out_specs=pl.BlockSpec((blk, EH), lambda i: (i, 0)),
        out_shape=jax.ShapeDtypeStruct((n, EH), _f32),
    )(s0, s1, y, dinv2, a, b, c)


def _tc_prep_ti(nf, dt2, di2):
    def body(x_ref, dt_ref, di_ref, yt_ref, yi_ref):
        yt_ref[...] = x_ref[...] * dt_ref[...]
        yi_ref[...] = x_ref[...] * di_ref[...]
    blk = 1000
    return pl.pallas_call(
        body,
        grid=(NM // blk,),
        in_specs=[pl.BlockSpec((blk, EH), lambda i: (i, 0)),
                  pl.BlockSpec((blk, 1), lambda i: (i, 0)),
                  pl.BlockSpec((blk, 1), lambda i: (i, 0))],
        out_specs=[pl.BlockSpec((blk, EH), lambda i: (i, 0))] * 2,
        out_shape=[jax.ShapeDtypeStruct((NM, EH), _f32)] * 2,
    )(nf, dt2, di2)


def _tc_ent_mlp(entA, w1, b1, w2, b2, ew, eb):
    def body(x_ref, w1_ref, b1_ref, w2_ref, b2_ref, ew_ref, eb_ref, o_ref):
        x = x_ref[...]
        h = jnp.maximum(_dot(x, w1_ref[...]) + b1_ref[...], 0.0)
        m = _dot(h, w2_ref[...]) + b2_ref[...] + x
        o_ref[...] = _dot(m, ew_ref[...]) + eb_ref[...]
    blk = 2000
    return pl.pallas_call(
        body,
        grid=(N // blk,),
        in_specs=[
            pl.BlockSpec((blk, EH), lambda i: (i, 0)),
            pl.BlockSpec((EH, EH // 2), lambda i: (0, 0)),
            pl.BlockSpec((1, EH // 2), lambda i: (0, 0)),
            pl.BlockSpec((EH // 2, EH), lambda i: (0, 0)),
            pl.BlockSpec((1, EH), lambda i: (0, 0)),
            pl.BlockSpec((EH, HID), lambda i: (0, 0)),
            pl.BlockSpec((1, HID), lambda i: (0, 0)),
        ],
        out_specs=pl.BlockSpec((blk, HID), lambda i: (i, 0)),
        out_shape=jax.ShapeDtypeStruct((N, HID), _f32),
    )(entA, w1, b1[None], w2, b2[None], ew, eb[None])


def _tc_tok_mlp(tok, w1, b1, w2, b2, tw, tb):
    def body(t_ref, w1_ref, b1_ref, w2_ref, b2_ref, tw_ref, tb_ref, o_ref):
        t = t_ref[0]
        h = jnp.maximum(_dot(t, w1_ref[...]) + b1_ref[...], 0.0)
        m = _dot(h, w2_ref[...]) + b2_ref[...] + t
        o_ref[0] = _dot(m, tw_ref[...]) + tb_ref[...]
    return pl.pallas_call(
        body,
        grid=(BB,),
        in_specs=[
            pl.BlockSpec((1, LT, TOK), lambda i: (i, 0, 0)),
            pl.BlockSpec((TOK, TOK // 2), lambda i: (0, 0)),
            pl.BlockSpec((1, TOK // 2), lambda i: (0, 0)),
            pl.BlockSpec((TOK // 2, TOK), lambda i: (0, 0)),
            pl.BlockSpec((1, TOK), lambda i: (0, 0)),
            pl.BlockSpec((TOK, HID), lambda i: (0, 0)),
            pl.BlockSpec((1, HID), lambda i: (0, 0)),
        ],
        out_specs=pl.BlockSpec((1, LT, HID), lambda i: (i, 0, 0)),
        out_shape=jax.ShapeDtypeStruct((BB, LT, HID), _f32),
    )(tok, w1, b1[None], w2, b2[None], tw, tb[None])


def _tc_attn(tt, e, cross_w):
    def body(t_ref, e_ref, w_ref, o_ref):
        t = t_ref[0]
        e2 = e_ref[0]
        q = _dot(t, w_ref[...])
        a = lax.dot_general(q, e2, (((1,), (1,)), ((), ())),
                            preferred_element_type=_f32) * (1.0 / HID)
        a = a - jnp.max(a, axis=1, keepdims=True)
        ex = jnp.exp(a)
        ew = ex / jnp.sum(ex, axis=1, keepdims=True)
        o_ref[0] = _dot(ew, e2) + t
    return pl.pallas_call(
        body,
        grid=(BB,),
        in_specs=[
            pl.BlockSpec((1, LT, HID), lambda i: (i, 0, 0)),
            pl.BlockSpec((1, LE, HID), lambda i: (i, 0, 0)),
            pl.BlockSpec((HID, HID), lambda i: (0, 0)),
        ],
        out_specs=pl.BlockSpec((1, LT, HID), lambda i: (i, 0, 0)),
        out_shape=jax.ShapeDtypeStruct((BB, LT, HID), _f32),
    )(tt, e, cross_w)


def _pad1(a, n, val):
    return jnp.concatenate([a, jnp.full((n - a.shape[0],), val, a.dtype)])


def _pad_spread(a, n, lo, nspread):
    npad = n - a.shape[0]
    pad = lo + (jnp.arange(npad, dtype=a.dtype) % nspread)
    return jnp.concatenate([a, pad])


def kernel(node_embeds, basis, comp, root_w, root_b, ep1_w1, ep1_b1, ep1_w2,
           ep1_b2, ep2_w, ep2_b, tp1_w1, tp1_b1, tp1_w2, tp1_b2, tp2_w, tp2_b,
           cross_w, token_embeds, entity_ids, edge_index, edge_type,
           edge_index_c, edge_index_t_s, edge_index_i_s, movie_indices):
    kg_src = _pad1(edge_index[0], EP_KG, 0)
    kg_dst = _pad_spread(edge_index[1], EP_KG, N, 16)
    kg_et = _pad1(edge_type, EP_KG, 0)
    c_row = _pad1(edge_index_c[0], EP_C, 0)
    c_col = _pad_spread(edge_index_c[1], EP_C, N, NB_ENT - N)
    t_row = _pad1(edge_index_t_s[0], EP_S, 0)
    t_col = _pad_spread(edge_index_t_s[1], EP_S, NM, NB_MOV - NM)
    i_row = _pad1(edge_index_i_s[0], EP_S, 0)
    i_col = _pad_spread(edge_index_i_s[1], EP_S, NM, NB_MOV - NM)
    mi_pad = _pad1(movie_indices, NB_MOV, 0)
    eids = entity_ids.reshape(-1)

    cnt_seg, cnt_c, cnt_t, cnt_i = _make_cnt_kernel()(
        kg_dst, kg_et, c_col, t_col, i_col)
    inv, dinv_c, dinv_t, dinv_i = _tc_transform_counts(
        cnt_seg, cnt_c, cnt_t, cnt_i)
    dc2, dt2, di2 = dinv_c[:, None], dinv_t[:, None], dinv_i[:, None]

    xr = _tc_xr(node_embeds, basis, comp).reshape(NREL * N, EH)
    scale, gidx = _make_scale_kernel()(inv, kg_src, kg_dst, kg_et)
    rg = _make_rgcn_kernel()(xr, scale, gidx, kg_dst)
    ent0, y1c = _tc_ent0(node_embeds, rg[0, :N], rg[1, :N], root_w, root_b, dc2)

    scat_c = _make_scat_kernel(EP_C, NB_ENT)
    s1 = scat_c(y1c, c_row, c_col)
    c1, y2c = _tc_gcn_step(N, s1[0, :N], s1[1, :N], y1c, dc2)
    s2 = scat_c(y2c, c_row, c_col)
    c2, y3c = _tc_gcn_step(N, s2[0, :N], s2[1, :N], y2c, dc2)
    s3 = scat_c(y3c, c_row, c_col)
    entA = _tc_combine4(N, s3[0, :N], s3[1, :N], y3c, dc2, c1, c2, ent0)

    nf = _make_gather_kernel(NB_MOV, EH)(ent0, mi_pad)[:NM]
    y1t, y1i = _tc_prep_ti(nf, dt2, di2)
    scat_s = _make_scat_kernel(EP_S, NB_MOV)
    st1 = scat_s(y1t, t_row, t_col)
    t1, y2t = _tc_gcn_step(NM, st1[0, :NM], st1[1, :NM], y1t, dt2)
    st2 = scat_s(y2t, t_row, t_col)
    t2, _ = _tc_gcn_step(NM, st2[0, :NM], st2[1, :NM], y2t, dt2)
    si1 = scat_s(y1i, i_row, i_col)
    i1, y2i = _tc_gcn_step(NM, si1[0, :NM], si1[1, :NM], y1i, di2)
    si2 = scat_s(y2i, i_row, i_col)
    mean = _tc_combine4(NM, si2[0, :NM], si2[1, :NM], y2i, di2, t1, t2, i1)

    mean_pad = jnp.concatenate(
        [mean, jnp.zeros((NB_MOV - NM, EH), _f32)], axis=0)
    entA2 = _make_merge_kernel()(entA, mean_pad, mi_pad)[:N]
    entF = _tc_ent_mlp(entA2, ep1_w1, ep1_b1, ep1_w2, ep1_b2, ep2_w, ep2_b)

    e = _make_gather_kernel(BB * LE, HID)(entF, eids).reshape(BB, LE, HID)
    tt = _tc_tok_mlp(token_embeds, tp1_w1, tp1_b1, tp1_w2, tp1_b2, tp2_w, tp2_b)
    return _tc_attn(tt, e, cross_w)


# speedup vs baseline: 1.1813x; 1.0774x over previous
import functools

import jax
import jax.numpy as jnp
from jax import lax
from jax.experimental import pallas as pl
from jax.experimental.pallas import tpu as pltpu
from jax.experimental.pallas import tpu_sc as plsc

N = 10000
NM = 5000
EH = 128
NREL = 8
HID = 256
TOK = 768
BB, LE, LT = 16, 32, 64

NC, NS, LANE = 2, 16, 16
NW = NC * NS

NB_ENT = 10240
NB_MOV = 5120
NB_SEG = 80128
EP_KG = 327680
EP_C = 163840
EP_S = 81920
CHUNK = 128

_f32 = jnp.float32
_i32 = jnp.int32


def _mesh():
    return plsc.VectorSubcoreMesh(core_axis_name="c", subcore_axis_name="s")


_SC_PARAMS = pltpu.CompilerParams(needs_layout_passes=False)


def _fill_zero_2d(buf, nrows):
    def body(i, _):
        for c8 in range(EH // LANE):
            buf[i, pl.ds(c8 * LANE, LANE)] = jnp.zeros((LANE,), _f32)
        return 0
    lax.fori_loop(0, nrows, body, 0)


def _fill_const_1d(buf, n, val):
    def body(i, _):
        buf[pl.ds(i * LANE, LANE)] = jnp.full((LANE,), val, _f32)
        return 0
    lax.fori_loop(0, n // LANE, body, 0)


def _make_cnt_kernel():
    ZROWS = NB_SEG // NS

    @functools.partial(
        pl.kernel, mesh=_mesh(), compiler_params=_SC_PARAMS,
        out_type=(
            jax.ShapeDtypeStruct((NC * NB_SEG,), _f32),
            jax.ShapeDtypeStruct((NC * NB_ENT,), _f32),
            jax.ShapeDtypeStruct((NC * NB_MOV,), _f32),
            jax.ShapeDtypeStruct((NC * NB_MOV,), _f32),
        ),
        scratch_types=[
            pltpu.VMEM((ZROWS,), _f32),
            pltpu.VMEM((CHUNK,), _f32),
            pltpu.VMEM((CHUNK,), _i32),
            pltpu.VMEM((CHUNK,), _i32),
            pltpu.VMEM((CHUNK,), _i32),
            pltpu.VMEM_SHARED((NB_SEG,), _f32),
            pltpu.VMEM_SHARED((NB_ENT,), _f32),
            pltpu.VMEM_SHARED((NB_MOV,), _f32),
            pltpu.VMEM_SHARED((NB_MOV,), _f32),
        ],
    )
    def k(kg_dst, kg_et, c_col, t_col, i_col,
          out_seg, out_c, out_t, out_i,
          zbuf, ones, dstb, etb, segb, b_seg, b_c, b_t, b_i):
        cid = lax.axis_index("c")
        sid = lax.axis_index("s")
        wid = sid * NC + cid
        _fill_const_1d(zbuf, ZROWS, 0.0)
        _fill_const_1d(ones, CHUNK, 1.0)
        for bins, nb in ((b_seg, NB_SEG), (b_c, NB_ENT), (b_t, NB_MOV), (b_i, NB_MOV)):
            per = nb // NS
            pltpu.sync_copy(zbuf.at[pl.ds(0, per)], bins.at[pl.ds(sid * per, per)])
        plsc.subcore_barrier()

        def kg_body(t, _):
            eb = wid * (EP_KG // NW) + t * CHUNK
            pltpu.sync_copy(kg_dst.at[pl.ds(eb, CHUNK)], dstb)
            pltpu.sync_copy(kg_et.at[pl.ds(eb, CHUNK)], etb)
            for c8 in range(CHUNK // LANE):
                sl = pl.ds(c8 * LANE, LANE)
                segb[sl] = dstb[sl] * NREL + etb[sl]
            pltpu.sync_copy(ones, b_seg.at[segb], add=True)
            return 0
        lax.fori_loop(0, EP_KG // NW // CHUNK, kg_body, 0)

        def deg_body(col_hbm, bins, ep):
            def body(t, _):
                eb = wid * (ep // NW) + t * CHUNK
                pltpu.sync_copy(col_hbm.at[pl.ds(eb, CHUNK)], dstb)
                pltpu.sync_copy(ones, bins.at[dstb], add=True)
                return 0
            lax.fori_loop(0, ep // NW // CHUNK, body, 0)
        deg_body(c_col, b_c, EP_C)
        deg_body(t_col, b_t, EP_S)
        deg_body(i_col, b_i, EP_S)
        plsc.subcore_barrier()
        for bins, out, nb in ((b_seg, out_seg, NB_SEG), (b_c, out_c, NB_ENT),
                              (b_t, out_t, NB_MOV), (b_i, out_i, NB_MOV)):
            per = nb // NS
            pltpu.sync_copy(bins.at[pl.ds(sid * per, per)], zbuf.at[pl.ds(0, per)])
            pltpu.sync_copy(zbuf.at[pl.ds(0, per)],
                            out.at[pl.ds(cid * nb + sid * per, per)])
    return k


def _make_scale_kernel():
    ETILE = EP_KG // NW
    NCHUNK = ETILE // CHUNK

    @functools.partial(
        pl.kernel, mesh=_mesh(), compiler_params=_SC_PARAMS,
        out_type=(
            jax.ShapeDtypeStruct((EP_KG,), _f32),
            jax.ShapeDtypeStruct((EP_KG,), _i32),
        ),
        scratch_types=[
            pltpu.VMEM((NB_SEG,), _f32),
            pltpu.VMEM((CHUNK,), _i32),
            pltpu.VMEM((CHUNK,), _i32),
            pltpu.VMEM((CHUNK,), _i32),
            pltpu.VMEM((CHUNK,), _i32),
            pltpu.VMEM((CHUNK,), _f32),
        ],
    )
    def k(inv_hbm, src_hbm, dst_hbm, et_hbm, scale_out, gidx_out,
          invt, srcb, dstb, etb, gidxb, scaleb):
        cid = lax.axis_index("c")
        sid = lax.axis_index("s")
        wid = sid * NC + cid
        pltpu.sync_copy(inv_hbm, invt)

        def body(t, _):
            eb = wid * ETILE + t * CHUNK
            pltpu.sync_copy(src_hbm.at[pl.ds(eb, CHUNK)], srcb)
            pltpu.sync_copy(dst_hbm.at[pl.ds(eb, CHUNK)], dstb)
            pltpu.sync_copy(et_hbm.at[pl.ds(eb, CHUNK)], etb)
            for c8 in range(CHUNK // LANE):
                sl = pl.ds(c8 * LANE, LANE)
                e16 = etb[sl]
                gidxb[sl] = e16 * N + srcb[sl]
                scaleb[sl] = plsc.load_gather(invt, [dstb[sl] * NREL + e16])
            pltpu.sync_copy(scaleb, scale_out.at[pl.ds(eb, CHUNK)])
            pltpu.sync_copy(gidxb, gidx_out.at[pl.ds(eb, CHUNK)])
            return 0
        lax.fori_loop(0, NCHUNK, body, 0)
    return k


def _make_rgcn_kernel():
    ETILE = EP_KG // NW
    NCHUNK = ETILE // CHUNK
    RPT = NB_ENT // NS
    ZR = 64

    NH = NCHUNK // 2

    @functools.partial(
        pl.kernel, mesh=_mesh(), compiler_params=_SC_PARAMS,
        out_type=jax.ShapeDtypeStruct((NC, NB_ENT, EH), _f32),
        scratch_types=[
            pltpu.VMEM((ZR, EH), _f32),
            pltpu.VMEM((CHUNK,), _i32), pltpu.VMEM((CHUNK,), _i32),
            pltpu.VMEM((CHUNK,), _i32), pltpu.VMEM((CHUNK,), _i32),
            pltpu.VMEM((CHUNK,), _f32), pltpu.VMEM((CHUNK,), _f32),
            pltpu.VMEM((CHUNK, EH), _f32), pltpu.VMEM((CHUNK, EH), _f32),
            pltpu.VMEM_SHARED((NB_ENT, EH), _f32),
            pltpu.SemaphoreType.DMA, pltpu.SemaphoreType.DMA,
            pltpu.SemaphoreType.DMA, pltpu.SemaphoreType.DMA,
            pltpu.SemaphoreType.DMA, pltpu.SemaphoreType.DMA,
        ],
    )
    def k(xr_hbm, scale_hbm, gidx_hbm, dst_hbm, out_hbm,
          zbuf, dstb0, dstb1, gidxb0, gidxb1, scaleb0, scaleb1, rows0, rows1,
          bins, semi0, semi1, semg0, semg1, sems0, sems1):
        cid = lax.axis_index("c")
        sid = lax.axis_index("s")
        wid = sid * NC + cid
        base = wid * ETILE
        dstb = (dstb0, dstb1)
        gidxb = (gidxb0, gidxb1)
        scaleb = (scaleb0, scaleb1)
        rows = (rows0, rows1)
        semi = (semi0, semi1)
        semg = (semg0, semg1)
        _fill_zero_2d(zbuf, ZR)
        for j in range(RPT // ZR):
            pltpu.sync_copy(zbuf, bins.at[pl.ds(sid * RPT + j * ZR, ZR)])
        plsc.subcore_barrier()

        def fire_idx(t, b):
            eb = (wid + t * NW) * CHUNK
            pltpu.async_copy(dst_hbm.at[pl.ds(eb, CHUNK)], dstb[b], semi[b])
            pltpu.async_copy(gidx_hbm.at[pl.ds(eb, CHUNK)], gidxb[b], semi[b])
            pltpu.async_copy(scale_hbm.at[pl.ds(eb, CHUNK)], scaleb[b], semi[b])

        def wait_idx(b):
            pltpu.make_async_copy(dst_hbm.at[pl.ds(0, CHUNK)], dstb[b],
                                  semi[b]).wait()
            pltpu.make_async_copy(gidx_hbm.at[pl.ds(0, CHUNK)], gidxb[b],
                                  semi[b]).wait()
            pltpu.make_async_copy(scale_hbm.at[pl.ds(0, CHUNK)], scaleb[b],
                                  semi[b]).wait()

        def fire_gather(b):
            pltpu.async_copy(xr_hbm.at[gidxb[b]], rows[b], semg[b])

        def wait_gather(b):
            pltpu.make_async_copy(xr_hbm.at[gidxb[b]], rows[b], semg[b]).wait()

        def scale_rows(b):
            def mul_body(j, _):
                splat = plsc.load_gather(
                    scaleb[b], [jnp.broadcast_to(j, (LANE,))])
                for c8 in range(EH // LANE):
                    sl = pl.ds(c8 * LANE, LANE)
                    rows[b][j, sl] = rows[b][j, sl] * splat
                return 0
            lax.fori_loop(0, CHUNK, mul_body, 0)

        fire_idx(0, 0)
        fire_idx(1, 1)
        wait_idx(0)
        fire_gather(0)

        def body(i, _):
            wait_idx(1)
            fire_gather(1)
            wait_gather(0)
            scale_rows(0)
            d = pltpu.async_copy(rows0, bins.at[dstb0], sems0, add=True)
            d.wait()
            @pl.when(i < NH - 1)
            def _():
                fire_idx(2 * i + 2, 0)
                wait_idx(0)
                fire_gather(0)
            wait_gather(1)
            scale_rows(1)
            d1 = pltpu.async_copy(rows1, bins.at[dstb1], sems1, add=True)
            d1.wait()
            @pl.when(i < NH - 1)
            def _():
                fire_idx(2 * i + 3, 1)
            return 0
        lax.fori_loop(0, NH, body, 0)
        plsc.subcore_barrier()
        def flush(j, _):
            pltpu.sync_copy(bins.at[pl.ds(sid * RPT + j * ZR, ZR)], zbuf)
            pltpu.sync_copy(zbuf, out_hbm.at[cid, pl.ds(sid * RPT + j * ZR, ZR)])
            return 0
        lax.fori_loop(0, RPT // ZR, flush, 0)
    return k


def _make_scat_kernel(ep, nbins, stage_y=False, ny=0):
    ETILE = ep // NW
    NCHUNK = ETILE // CHUNK
    RPT = nbins // NS
    ZR = 64

    NH = NCHUNK // 2
    ytab_scratch = [pltpu.VMEM_SHARED((nbins, EH), _f32)] if stage_y else []

    @functools.partial(
        pl.kernel, mesh=_mesh(), compiler_params=_SC_PARAMS,
        out_type=jax.ShapeDtypeStruct((NC, nbins, EH), _f32),
        scratch_types=[
            pltpu.VMEM((ZR, EH), _f32),
            pltpu.VMEM((CHUNK,), _i32), pltpu.VMEM((CHUNK,), _i32),
            pltpu.VMEM((CHUNK,), _i32), pltpu.VMEM((CHUNK,), _i32),
            pltpu.VMEM((CHUNK, EH), _f32), pltpu.VMEM((CHUNK, EH), _f32),
            pltpu.VMEM_SHARED((nbins, EH), _f32),
        ] + ytab_scratch + [
            pltpu.SemaphoreType.DMA, pltpu.SemaphoreType.DMA,
            pltpu.SemaphoreType.DMA, pltpu.SemaphoreType.DMA,
            pltpu.SemaphoreType.DMA, pltpu.SemaphoreType.DMA,
        ],
    )
    def k(y_hbm, row_hbm, col_hbm, out_hbm, zbuf, rowb0, rowb1, colb0, colb1,
          rows0, rows1, bins, *rest):
        if stage_y:
            ytab = rest[0]
            semi0, semi1, semg0, semg1, sems0, sems1 = rest[1:]
        else:
            ytab = y_hbm
            semi0, semi1, semg0, semg1, sems0, sems1 = rest
        cid = lax.axis_index("c")
        sid = lax.axis_index("s")
        wid = sid * NC + cid
        base = wid * ETILE
        rowb = (rowb0, rowb1)
        colb = (colb0, colb1)
        rows = (rows0, rows1)
        semi = (semi0, semi1)
        semg = (semg0, semg1)
        sems = (sems0, sems1)
        _fill_zero_2d(zbuf, ZR)
        for j in range(RPT // ZR):
            pltpu.sync_copy(zbuf, bins.at[pl.ds(sid * RPT + j * ZR, ZR)])
        plsc.subcore_barrier()

        def fire_idx(t, b):
            eb = (wid + t * NW) * CHUNK
            pltpu.async_copy(row_hbm.at[pl.ds(eb, CHUNK)], rowb[b], semi[b])
            pltpu.async_copy(col_hbm.at[pl.ds(eb, CHUNK)], colb[b], semi[b])

        def wait_idx(b):
            pltpu.make_async_copy(row_hbm.at[pl.ds(0, CHUNK)], rowb[b],
                                  semi[b]).wait()
            pltpu.make_async_copy(col_hbm.at[pl.ds(0, CHUNK)], colb[b],
                                  semi[b]).wait()

        def fire_gather(b):
            pltpu.async_copy(ytab.at[rowb[b]], rows[b], semg[b])

        def wait_gather(b):
            pltpu.make_async_copy(ytab.at[rowb[b]], rows[b], semg[b]).wait()

        if stage_y:
            nfull = ny // 80
            tail = ny - nfull * 80
            def stage(i, _):
                ci = sid + i * NS
                @pl.when(ci < nfull)
                def _():
                    r0 = ci * 80
                    pltpu.sync_copy(y_hbm.at[pl.ds(r0, 80)],
                                    rows0.at[pl.ds(0, 80)])
                    pltpu.sync_copy(rows0.at[pl.ds(0, 80)],
                                    ytab.at[pl.ds(r0, 80)])
                return 0
            lax.fori_loop(0, (nfull + NS - 1) // NS, stage, 0)
            if tail:
                @pl.when(sid == NS - 1)
                def _():
                    r0 = nfull * 80
                    pltpu.sync_copy(y_hbm.at[pl.ds(r0, tail)],
                                    rows1.at[pl.ds(0, tail)])
                    pltpu.sync_copy(rows1.at[pl.ds(0, tail)],
                                    ytab.at[pl.ds(r0, tail)])
            plsc.subcore_barrier()

        fire_idx(0, 0)
        fire_idx(1, 1)
        wait_idx(0)
        fire_gather(0)

        def body(i, _):
            wait_idx(1)
            fire_gather(1)
            wait_gather(0)
            d = pltpu.async_copy(rows0, bins.at[colb0], sems0, add=True)
            d.wait()
            @pl.when(i < NH - 1)
            def _():
                fire_idx(2 * i + 2, 0)
            @pl.when(i < NH - 1)
            def _():
                wait_idx(0)
                fire_gather(0)
            wait_gather(1)
            d1 = pltpu.async_copy(rows1, bins.at[colb1], sems1, add=True)
            d1.wait()
            @pl.when(i < NH - 1)
            def _():
                fire_idx(2 * i + 3, 1)
            return 0
        lax.fori_loop(0, NH, body, 0)
        plsc.subcore_barrier()
        def flush(j, _):
            pltpu.sync_copy(bins.at[pl.ds(sid * RPT + j * ZR, ZR)], zbuf)
            pltpu.sync_copy(zbuf, out_hbm.at[cid, pl.ds(sid * RPT + j * ZR, ZR)])
            return 0
        lax.fori_loop(0, RPT // ZR, flush, 0)
    return k


def _make_gather_kernel(ni, feat):
    KPT = ni // NW
    TR = next(t for t in (128, 80, 64, 32, 16, 8) if KPT % t == 0 and t <= KPT)
    NT = KPT // TR

    @functools.partial(
        pl.kernel, mesh=_mesh(), compiler_params=_SC_PARAMS,
        out_type=jax.ShapeDtypeStruct((ni, feat), _f32),
        scratch_types=[
            pltpu.VMEM((KPT,), _i32),
            pltpu.VMEM((TR, feat), _f32),
            pltpu.SemaphoreType.DMA,
        ],
    )
    def k(table_hbm, idx_hbm, out_hbm, idxb, rows, sem):
        cid = lax.axis_index("c")
        sid = lax.axis_index("s")
        wid = sid * NC + cid
        base = wid * KPT
        pltpu.sync_copy(idx_hbm.at[pl.ds(base, KPT)], idxb)
        for j in range(NT):
            pltpu.async_copy(table_hbm.at[idxb.at[pl.ds(j * TR, TR)]], rows,
                             sem).wait()
            pltpu.sync_copy(rows, out_hbm.at[pl.ds(base + j * TR, TR)])
    return k


def _make_merge_kernel():
    KPT = NB_MOV // NW
    TR = 32
    NT = KPT // TR
    HALF = NB_ENT // 2
    TRASH = NB_ENT - 8

    @functools.partial(
        pl.kernel, mesh=_mesh(), compiler_params=_SC_PARAMS,
        out_type=jax.ShapeDtypeStruct((NB_ENT, EH), _f32),
        scratch_types=[
            pltpu.VMEM((KPT,), _i32),
            pltpu.VMEM((NT, TR), _i32),
            pltpu.VMEM((KPT, EH), _f32),
            pltpu.VMEM((TR, EH), _f32),
            pltpu.VMEM((80, EH), _f32),
            pltpu.SemaphoreType.DMA,
        ],
    )
    def k(entA_hbm, mean_hbm, mi_hbm, out_hbm, mib, rib, meanb, rows, cbuf, sem):
        cid = lax.axis_index("c")
        sid = lax.axis_index("s")
        wid = sid * NC + cid
        def cp(i, _):
            ci = sid + i * NS
            nch = jnp.where(cid == 0, 64, 61)
            @pl.when(ci < nch)
            def _():
                r0 = cid * HALF + ci * 80
                pltpu.sync_copy(entA_hbm.at[pl.ds(r0, 80)], cbuf)
                pltpu.sync_copy(cbuf, out_hbm.at[pl.ds(r0, 80)])
            return 0
        lax.fori_loop(0, 4, cp, 0)
        plsc.subcore_barrier()
        base = wid * KPT
        pltpu.sync_copy(mi_hbm.at[pl.ds(base, KPT)], mib)
        pltpu.sync_copy(mean_hbm.at[pl.ds(base, KPT)], meanb)
        for c8 in range(KPT // LANE):
            sl = pl.ds((c8 % (TR // LANE)) * LANE, LANE)
            m16 = mib[pl.ds(c8 * LANE, LANE)]
            mine = jnp.where(cid == 0, m16 < HALF, m16 >= HALF)
            rib[c8 // (TR // LANE), sl] = jnp.where(mine, m16, TRASH)
        for j in range(NT):
            pltpu.async_copy(entA_hbm.at[mib.at[pl.ds(j * TR, TR)]], rows,
                             sem).wait()
            def addrow(i, _):
                for c8 in range(EH // LANE):
                    sl = pl.ds(c8 * LANE, LANE)
                    rows[i, sl] = rows[i, sl] + meanb[j * TR + i, sl]
                return 0
            lax.fori_loop(0, TR, addrow, 0)
            pltpu.sync_copy(rows, out_hbm.at[rib.at[j]])
    return k


def _dot(a, b):
    return jnp.dot(a, b, preferred_element_type=_f32)


def _tc_xr(node, basis, comp):
    def body(comp_ref, basis_ref, x_ref, o_ref):
        r = pl.program_id(0)
        w = comp_ref[r, 0] * basis_ref[0]
        for b in range(1, NREL):
            w = w + comp_ref[r, b] * basis_ref[b]
        o_ref[0] = _dot(x_ref[...], w)
    return pl.pallas_call(
        body,
        grid=(NREL,),
        in_specs=[
            pl.BlockSpec((NREL, NREL), lambda r: (0, 0)),
            pl.BlockSpec((NREL, EH, EH), lambda r: (0, 0, 0)),
            pl.BlockSpec((N, EH), lambda r: (0, 0)),
        ],
        out_specs=pl.BlockSpec((1, N, EH), lambda r: (r, 0, 0)),
        out_shape=jax.ShapeDtypeStruct((NREL, N, EH), _f32),
    )(comp, basis, node)


def _tc_transform_counts(cnt_seg, cnt_c, cnt_t, cnt_i):
    def body(s_ref, c_ref, t_ref, i_ref, inv_ref, dc_ref, dt_ref, di_ref):
        s = s_ref[0] + s_ref[1]
        inv_ref[...] = 1.0 / jnp.maximum(s, 1.0)
        dc_ref[...] = lax.rsqrt(c_ref[0] + c_ref[1] + 1.0)
        dt_ref[...] = lax.rsqrt(t_ref[0] + t_ref[1] + 1.0)
        di_ref[...] = lax.rsqrt(i_ref[0] + i_ref[1] + 1.0)
    r = lambda a: a.reshape(NC, -1, 128)
    outs = pl.pallas_call(
        body,
        out_shape=(
            jax.ShapeDtypeStruct((NB_SEG // 128, 128), _f32),
            jax.ShapeDtypeStruct((NB_ENT // 128, 128), _f32),
            jax.ShapeDtypeStruct((NB_MOV // 128, 128), _f32),
            jax.ShapeDtypeStruct((NB_MOV // 128, 128), _f32),
        ),
    )(r(cnt_seg), r(cnt_c), r(cnt_t), r(cnt_i))
    inv, dc, dt, di = outs
    return (inv.reshape(-1), dc.reshape(-1)[:N], dt.reshape(-1)[:NM],
            di.reshape(-1)[:NM])


def _tc_ent0(node, p0, p1, root_w, root_b, dinv_c):
    def body(x_ref, p0_ref, p1_ref, w_ref, b_ref, d_ref, e_ref, y_ref):
        e = p0_ref[...] + p1_ref[...] + _dot(x_ref[...], w_ref[...]) \
            + b_ref[...] + x_ref[...]
        e_ref[...] = e
        y_ref[...] = e * d_ref[...]
    blk = 2000
    return pl.pallas_call(
        body,
        grid=(N // blk,),
        in_specs=[
            pl.BlockSpec((blk, EH), lambda i: (i, 0)),
            pl.BlockSpec((blk, EH), lambda i: (i, 0)),
            pl.BlockSpec((blk, EH), lambda i: (i, 0)),
            pl.BlockSpec((EH, EH), lambda i: (0, 0)),
            pl.BlockSpec((1, EH), lambda i: (0, 0)),
            pl.BlockSpec((blk, 1), lambda i: (i, 0)),
        ],
        out_specs=[pl.BlockSpec((blk, EH), lambda i: (i, 0))] * 2,
        out_shape=[jax.ShapeDtypeStruct((N, EH), _f32)] * 2,
    )(node, p0, p1, root_w, root_b[None], dinv_c)


def _tc_gcn_step(n, s0, s1, y, dinv2):
    def body(s0_ref, s1_ref, y_ref, d_ref, o_ref, yn_ref):
        d = d_ref[...]
        o = d * (s0_ref[...] + s1_ref[...] + y_ref[...])
        o_ref[...] = o
        yn_ref[...] = d * o
    blk = 2000 if n == N else 1000
    return pl.pallas_call(
        body,
        grid=(n // blk,),
        in_specs=[pl.BlockSpec((blk, EH), lambda i: (i, 0))] * 3
        + [pl.BlockSpec((blk, 1), lambda i: (i, 0))],
        out_specs=[pl.BlockSpec((blk, EH), lambda i: (i, 0))] * 2,
        out_shape=[jax.ShapeDtypeStruct((n, EH), _f32)] * 2,
    )(s0, s1, y, dinv2)


def _tc_combine4(n, s0, s1, y, dinv2, a, b, c):
    def body(s0_ref, s1_ref, y_ref, d_ref, a_ref, b_ref, c_ref, o_ref):
        last = d_ref[...] * (s0_ref[...] + s1_ref[...] + y_ref[...])
        o_ref[...] = (a_ref[...] + b_ref[...] + c_ref[...] + last) * 0.25
    blk = 2000 if n == N else 1000
    return pl.pallas_call(
        body,
        grid=(n // blk,),
        in_specs=[pl.BlockSpec((blk, EH), lambda i: (i, 0))] * 3
        + [pl.BlockSpec((blk, 1), lambda i: (i, 0))]
        + [pl.BlockSpec((blk, EH), lambda i: (i, 0))] * 3,
        out_specs=pl.BlockSpec((blk, EH), lambda i: (i, 0)),
        out_shape=jax.ShapeDtypeStruct((n, EH), _f32),
    )(s0, s1, y, dinv2, a, b, c)


def _tc_prep_ti(nf, dt2, di2):
    def body(x_ref, dt_ref, di_ref, yt_ref, yi_ref):
        yt_ref[...] = x_ref[...] * dt_ref[...]
        yi_ref[...] = x_ref[...] * di_ref[...]
    blk = 1000
    return pl.pallas_call(
        body,
        grid=(NM // blk,),
        in_specs=[pl.BlockSpec((blk, EH), lambda i: (i, 0)),
                  pl.BlockSpec((blk, 1), lambda i: (i, 0)),
                  pl.BlockSpec((blk, 1), lambda i: (i, 0))],
        out_specs=[pl.BlockSpec((blk, EH), lambda i: (i, 0))] * 2,
        out_shape=[jax.ShapeDtypeStruct((NM, EH), _f32)] * 2,
    )(nf, dt2, di2)


def _tc_ent_mlp(entA, w1, b1, w2, b2, ew, eb):
    def body(x_ref, w1_ref, b1_ref, w2_ref, b2_ref, ew_ref, eb_ref, o_ref):
        x = x_ref[...]
        h = jnp.maximum(_dot(x, w1_ref[...]) + b1_ref[...], 0.0)
        m = _dot(h, w2_ref[...]) + b2_ref[...] + x
        o_ref[...] = _dot(m, ew_ref[...]) + eb_ref[...]
    blk = 2000
    return pl.pallas_call(
        body,
        grid=(N // blk,),
        in_specs=[
            pl.BlockSpec((blk, EH), lambda i: (i, 0)),
            pl.BlockSpec((EH, EH // 2), lambda i: (0, 0)),
            pl.BlockSpec((1, EH // 2), lambda i: (0, 0)),
            pl.BlockSpec((EH // 2, EH), lambda i: (0, 0)),
            pl.BlockSpec((1, EH), lambda i: (0, 0)),
            pl.BlockSpec((EH, HID), lambda i: (0, 0)),
            pl.BlockSpec((1, HID), lambda i: (0, 0)),
        ],
        out_specs=pl.BlockSpec((blk, HID), lambda i: (i, 0)),
        out_shape=jax.ShapeDtypeStruct((N, HID), _f32),
    )(entA, w1, b1[None], w2, b2[None], ew, eb[None])


def _tc_tok_mlp(tok, w1, b1, w2, b2, tw, tb):
    def body(t_ref, w1_ref, b1_ref, w2_ref, b2_ref, tw_ref, tb_ref, o_ref):
        t = t_ref[0]
        h = jnp.maximum(_dot(t, w1_ref[...]) + b1_ref[...], 0.0)
        m = _dot(h, w2_ref[...]) + b2_ref[...] + t
        o_ref[0] = _dot(m, tw_ref[...]) + tb_ref[...]
    return pl.pallas_call(
        body,
        grid=(BB,),
        in_specs=[
            pl.BlockSpec((1, LT, TOK), lambda i: (i, 0, 0)),
            pl.BlockSpec((TOK, TOK // 2), lambda i: (0, 0)),
            pl.BlockSpec((1, TOK // 2), lambda i: (0, 0)),
            pl.BlockSpec((TOK // 2, TOK), lambda i: (0, 0)),
            pl.BlockSpec((1, TOK), lambda i: (0, 0)),
            pl.BlockSpec((TOK, HID), lambda i: (0, 0)),
            pl.BlockSpec((1, HID), lambda i: (0, 0)),
        ],
        out_specs=pl.BlockSpec((1, LT, HID), lambda i: (i, 0, 0)),
        out_shape=jax.ShapeDtypeStruct((BB, LT, HID), _f32),
    )(tok, w1, b1[None], w2, b2[None], tw, tb[None])


def _tc_attn(tt, e, cross_w):
    def body(t_ref, e_ref, w_ref, o_ref):
        t = t_ref[0]
        e2 = e_ref[0]
        q = _dot(t, w_ref[...])
        a = lax.dot_general(q, e2, (((1,), (1,)), ((), ())),
                            preferred_element_type=_f32) * (1.0 / HID)
        a = a - jnp.max(a, axis=1, keepdims=True)
        ex = jnp.exp(a)
        ew = ex / jnp.sum(ex, axis=1, keepdims=True)
        o_ref[0] = _dot(ew, e2) + t
    return pl.pallas_call(
        body,
        grid=(BB,),
        in_specs=[
            pl.BlockSpec((1, LT, HID), lambda i: (i, 0, 0)),
            pl.BlockSpec((1, LE, HID), lambda i: (i, 0, 0)),
            pl.BlockSpec((HID, HID), lambda i: (0, 0)),
        ],
        out_specs=pl.BlockSpec((1, LT, HID), lambda i: (i, 0, 0)),
        out_shape=jax.ShapeDtypeStruct((BB, LT, HID), _f32),
    )(tt, e, cross_w)


def _pad1(a, n, val):
    return jnp.concatenate([a, jnp.full((n - a.shape[0],), val, a.dtype)])


def _pad_spread(a, n, lo, nspread):
    npad = n - a.shape[0]
    pad = lo + (jnp.arange(npad, dtype=a.dtype) % nspread)
    return jnp.concatenate([a, pad])


def kernel(node_embeds, basis, comp, root_w, root_b, ep1_w1, ep1_b1, ep1_w2,
           ep1_b2, ep2_w, ep2_b, tp1_w1, tp1_b1, tp1_w2, tp1_b2, tp2_w, tp2_b,
           cross_w, token_embeds, entity_ids, edge_index, edge_type,
           edge_index_c, edge_index_t_s, edge_index_i_s, movie_indices):
    kg_src = _pad1(edge_index[0], EP_KG, 0)
    kg_dst = _pad_spread(edge_index[1], EP_KG, N, 16)
    kg_et = _pad1(edge_type, EP_KG, 0)
    c_row = _pad1(edge_index_c[0], EP_C, 0)
    c_col = _pad_spread(edge_index_c[1], EP_C, N, NB_ENT - N)
    t_row = _pad1(edge_index_t_s[0], EP_S, 0)
    t_col = _pad_spread(edge_index_t_s[1], EP_S, NM, NB_MOV - NM)
    i_row = _pad1(edge_index_i_s[0], EP_S, 0)
    i_col = _pad_spread(edge_index_i_s[1], EP_S, NM, NB_MOV - NM)
    mi_pad = _pad1(movie_indices, NB_MOV, 0)
    eids = entity_ids.reshape(-1)

    cnt_seg, cnt_c, cnt_t, cnt_i = _make_cnt_kernel()(
        kg_dst, kg_et, c_col, t_col, i_col)
    inv, dinv_c, dinv_t, dinv_i = _tc_transform_counts(
        cnt_seg, cnt_c, cnt_t, cnt_i)
    dc2, dt2, di2 = dinv_c[:, None], dinv_t[:, None], dinv_i[:, None]

    xr = _tc_xr(node_embeds, basis, comp).reshape(NREL * N, EH)
    scale, gidx = _make_scale_kernel()(inv, kg_src, kg_dst, kg_et)
    rg = _make_rgcn_kernel()(xr, scale, gidx, kg_dst)
    ent0, y1c = _tc_ent0(node_embeds, rg[0, :N], rg[1, :N], root_w, root_b, dc2)

    scat_c = _make_scat_kernel(EP_C, NB_ENT)
    s1 = scat_c(y1c, c_row, c_col)
    c1, y2c = _tc_gcn_step(N, s1[0, :N], s1[1, :N], y1c, dc2)
    s2 = scat_c(y2c, c_row, c_col)
    c2, y3c = _tc_gcn_step(N, s2[0, :N], s2[1, :N], y2c, dc2)
    s3 = scat_c(y3c, c_row, c_col)
    entA = _tc_combine4(N, s3[0, :N], s3[1, :N], y3c, dc2, c1, c2, ent0)

    nf = _make_gather_kernel(NB_MOV, EH)(ent0, mi_pad)[:NM]
    y1t, y1i = _tc_prep_ti(nf, dt2, di2)
    scat_s = _make_scat_kernel(EP_S, NB_MOV, stage_y=True, ny=NM)
    st1 = scat_s(y1t, t_row, t_col)
    t1, y2t = _tc_gcn_step(NM, st1[0, :NM], st1[1, :NM], y1t, dt2)
    st2 = scat_s(y2t, t_row, t_col)
    t2, _ = _tc_gcn_step(NM, st2[0, :NM], st2[1, :NM], y2t, dt2)
    si1 = scat_s(y1i, i_row, i_col)
    i1, y2i = _tc_gcn_step(NM, si1[0, :NM], si1[1, :NM], y1i, di2)
    si2 = scat_s(y2i, i_row, i_col)
    mean = _tc_combine4(NM, si2[0, :NM], si2[1, :NM], y2i, di2, t1, t2, i1)

    mean_pad = jnp.concatenate(
        [mean, jnp.zeros((NB_MOV - NM, EH), _f32)], axis=0)
    entA2 = _make_merge_kernel()(entA, mean_pad, mi_pad)[:N]
    entF = _tc_ent_mlp(entA2, ep1_w1, ep1_b1, ep1_w2, ep1_b2, ep2_w, ep2_b)

    e = _make_gather_kernel(BB * LE, HID)(entF, eids).reshape(BB, LE, HID)
    tt = _tc_tok_mlp(token_embeds, tp1_w1, tp1_b1, tp1_w2, tp1_b2, tp2_w, tp2_b)
    return _tc_attn(tt, e, cross_w)


# speedup vs baseline: 1.3039x; 1.1038x over previous
import functools

import jax
import jax.numpy as jnp
from jax import lax
from jax.experimental import pallas as pl
from jax.experimental.pallas import tpu as pltpu
from jax.experimental.pallas import tpu_sc as plsc

N = 10000
NM = 5000
EH = 128
NREL = 8
HID = 256
TOK = 768
BB, LE, LT = 16, 32, 64

NC, NS, LANE = 2, 16, 16
NW = NC * NS

NB_ENT = 10240
NB_MOV = 5120
NB_SEG = 80128
EP_KG = 327680
EP_C = 163840
EP_S = 81920
CHUNK = 128

_f32 = jnp.float32
_i32 = jnp.int32


def _mesh():
    return plsc.VectorSubcoreMesh(core_axis_name="c", subcore_axis_name="s")


_SC_PARAMS = pltpu.CompilerParams(needs_layout_passes=False)


def _fill_zero_2d(buf, nrows):
    def body(i, _):
        for c8 in range(EH // LANE):
            buf[i, pl.ds(c8 * LANE, LANE)] = jnp.zeros((LANE,), _f32)
        return 0
    lax.fori_loop(0, nrows, body, 0)


def _fill_const_1d(buf, n, val):
    def body(i, _):
        buf[pl.ds(i * LANE, LANE)] = jnp.full((LANE,), val, _f32)
        return 0
    lax.fori_loop(0, n // LANE, body, 0)


def _make_cnt_kernel():
    ZROWS = NB_SEG // NS

    @functools.partial(
        pl.kernel, mesh=_mesh(), compiler_params=_SC_PARAMS,
        out_type=(
            jax.ShapeDtypeStruct((NC * NB_SEG,), _f32),
            jax.ShapeDtypeStruct((NC * NB_ENT,), _f32),
            jax.ShapeDtypeStruct((NC * NB_MOV,), _f32),
            jax.ShapeDtypeStruct((NC * NB_MOV,), _f32),
        ),
        scratch_types=[
            pltpu.VMEM((ZROWS,), _f32),
            pltpu.VMEM((CHUNK,), _f32),
            pltpu.VMEM((CHUNK,), _i32), pltpu.VMEM((CHUNK,), _i32),
            pltpu.VMEM((CHUNK,), _i32), pltpu.VMEM((CHUNK,), _i32),
            pltpu.VMEM((CHUNK,), _i32), pltpu.VMEM((CHUNK,), _i32),
            pltpu.VMEM_SHARED((NB_SEG,), _f32),
            pltpu.VMEM_SHARED((NB_ENT,), _f32),
            pltpu.VMEM_SHARED((NB_MOV,), _f32),
            pltpu.VMEM_SHARED((NB_MOV,), _f32),
            pltpu.SemaphoreType.DMA, pltpu.SemaphoreType.DMA,
            pltpu.SemaphoreType.DMA, pltpu.SemaphoreType.DMA,
        ],
    )
    def k(kg_dst, kg_et, c_col, t_col, i_col,
          out_seg, out_c, out_t, out_i,
          zbuf, ones, dstb0, dstb1, etb0, etb1, segb0, segb1,
          b_seg, b_c, b_t, b_i, semi0, semi1, sems0, sems1):
        cid = lax.axis_index("c")
        sid = lax.axis_index("s")
        wid = sid * NC + cid
        dstb = (dstb0, dstb1)
        etb = (etb0, etb1)
        segb = (segb0, segb1)
        semi = (semi0, semi1)
        sems = (sems0, sems1)
        _fill_const_1d(zbuf, ZROWS, 0.0)
        _fill_const_1d(ones, CHUNK, 1.0)
        for bins, nb in ((b_seg, NB_SEG), (b_c, NB_ENT), (b_t, NB_MOV), (b_i, NB_MOV)):
            per = nb // NS
            pltpu.sync_copy(zbuf.at[pl.ds(0, per)], bins.at[pl.ds(sid * per, per)])
        plsc.subcore_barrier()

        def run_phase(idx_hbms, bins, ep, with_seg):
            nch = ep // NW // CHUNK
            nh = nch // 2

            def fire_idx(t, b):
                eb = (wid + t * NW) * CHUNK
                for h, dst in zip(idx_hbms, (dstb[b], etb[b])):
                    pltpu.async_copy(h.at[pl.ds(eb, CHUNK)], dst, semi[b])

            def wait_idx(b):
                for h, dst in zip(idx_hbms, (dstb[b], etb[b])):
                    pltpu.make_async_copy(h.at[pl.ds(0, CHUNK)], dst,
                                          semi[b]).wait()

            def half(i, b):
                wait_idx(b)
                if with_seg:
                    for c8 in range(CHUNK // LANE):
                        sl = pl.ds(c8 * LANE, LANE)
                        segb[b][sl] = dstb[b][sl] * NREL + etb[b][sl]
                    key = segb[b]
                else:
                    key = dstb[b]
                d = pltpu.async_copy(ones, bins.at[key], sems[b], add=True)
                d.wait()
                @pl.when(i < nh - 1)
                def _():
                    fire_idx(2 * i + 2 + b, b)

            fire_idx(0, 0)
            fire_idx(1, 1)
            def body(i, _):
                half(i, 0)
                half(i, 1)
                return 0
            lax.fori_loop(0, nh, body, 0)

        run_phase((kg_dst, kg_et), b_seg, EP_KG, True)
        run_phase((c_col,), b_c, EP_C, False)
        run_phase((t_col,), b_t, EP_S, False)
        run_phase((i_col,), b_i, EP_S, False)
        plsc.subcore_barrier()
        for bins, out, nb in ((b_seg, out_seg, NB_SEG), (b_c, out_c, NB_ENT),
                              (b_t, out_t, NB_MOV), (b_i, out_i, NB_MOV)):
            per = nb // NS
            pltpu.sync_copy(bins.at[pl.ds(sid * per, per)], zbuf.at[pl.ds(0, per)])
            pltpu.sync_copy(zbuf.at[pl.ds(0, per)],
                            out.at[pl.ds(cid * nb + sid * per, per)])
    return k


def _make_scale_kernel():
    ETILE = EP_KG // NW
    NCHUNK = ETILE // CHUNK

    @functools.partial(
        pl.kernel, mesh=_mesh(), compiler_params=_SC_PARAMS,
        out_type=(
            jax.ShapeDtypeStruct((EP_KG,), _f32),
            jax.ShapeDtypeStruct((EP_KG,), _i32),
        ),
        scratch_types=[
            pltpu.VMEM((NB_SEG,), _f32),
            pltpu.VMEM((CHUNK,), _i32), pltpu.VMEM((CHUNK,), _i32),
            pltpu.VMEM((CHUNK,), _i32), pltpu.VMEM((CHUNK,), _i32),
            pltpu.VMEM((CHUNK,), _i32), pltpu.VMEM((CHUNK,), _i32),
            pltpu.VMEM((CHUNK,), _i32), pltpu.VMEM((CHUNK,), _i32),
            pltpu.VMEM((CHUNK,), _f32), pltpu.VMEM((CHUNK,), _f32),
            pltpu.SemaphoreType.DMA, pltpu.SemaphoreType.DMA,
            pltpu.SemaphoreType.DMA, pltpu.SemaphoreType.DMA,
        ],
    )
    def k(inv_hbm, src_hbm, dst_hbm, et_hbm, scale_out, gidx_out,
          invt, srcb0, srcb1, dstb0, dstb1, etb0, etb1, gidxb0, gidxb1,
          scaleb0, scaleb1, semi0, semi1, semo0, semo1):
        cid = lax.axis_index("c")
        sid = lax.axis_index("s")
        wid = sid * NC + cid
        srcb = (srcb0, srcb1)
        dstb = (dstb0, dstb1)
        etb = (etb0, etb1)
        gidxb = (gidxb0, gidxb1)
        scaleb = (scaleb0, scaleb1)
        semi = (semi0, semi1)
        semo = (semo0, semo1)
        pltpu.sync_copy(inv_hbm, invt)
        NH = NCHUNK // 2

        def fire_idx(t, b):
            eb = wid * ETILE + t * CHUNK
            pltpu.async_copy(src_hbm.at[pl.ds(eb, CHUNK)], srcb[b], semi[b])
            pltpu.async_copy(dst_hbm.at[pl.ds(eb, CHUNK)], dstb[b], semi[b])
            pltpu.async_copy(et_hbm.at[pl.ds(eb, CHUNK)], etb[b], semi[b])

        def wait_idx(b):
            for h, d in ((src_hbm, srcb[b]), (dst_hbm, dstb[b]),
                         (et_hbm, etb[b])):
                pltpu.make_async_copy(h.at[pl.ds(0, CHUNK)], d, semi[b]).wait()

        def wait_out(b):
            pltpu.make_async_copy(scaleb[b], scale_out.at[pl.ds(0, CHUNK)],
                                  semo[b]).wait()
            pltpu.make_async_copy(gidxb[b], gidx_out.at[pl.ds(0, CHUNK)],
                                  semo[b]).wait()

        def half(i, b):
            t = 2 * i + b
            wait_idx(b)
            @pl.when(i > 0)
            def _():
                wait_out(b)
            for c8 in range(CHUNK // LANE):
                sl = pl.ds(c8 * LANE, LANE)
                e16 = etb[b][sl]
                gidxb[b][sl] = e16 * N + srcb[b][sl]
                scaleb[b][sl] = plsc.load_gather(
                    invt, [dstb[b][sl] * NREL + e16])
            eb = wid * ETILE + t * CHUNK
            pltpu.async_copy(scaleb[b], scale_out.at[pl.ds(eb, CHUNK)], semo[b])
            pltpu.async_copy(gidxb[b], gidx_out.at[pl.ds(eb, CHUNK)], semo[b])
            @pl.when(i < NH - 1)
            def _():
                fire_idx(t + 2, b)

        fire_idx(0, 0)
        fire_idx(1, 1)
        def body(i, _):
            half(i, 0)
            half(i, 1)
            return 0
        lax.fori_loop(0, NH, body, 0)
        wait_out(0)
        wait_out(1)
    return k


def _make_rgcn_kernel():
    ETILE = EP_KG // NW
    NCHUNK = ETILE // CHUNK
    RPT = NB_ENT // NS
    ZR = 64

    NH = NCHUNK // 2

    @functools.partial(
        pl.kernel, mesh=_mesh(), compiler_params=_SC_PARAMS,
        out_type=jax.ShapeDtypeStruct((NC, NB_ENT, EH), _f32),
        scratch_types=[
            pltpu.VMEM((ZR, EH), _f32),
            pltpu.VMEM((CHUNK,), _i32), pltpu.VMEM((CHUNK,), _i32),
            pltpu.VMEM((CHUNK,), _i32), pltpu.VMEM((CHUNK,), _i32),
            pltpu.VMEM((CHUNK,), _f32), pltpu.VMEM((CHUNK,), _f32),
            pltpu.VMEM((CHUNK, EH), _f32), pltpu.VMEM((CHUNK, EH), _f32),
            pltpu.VMEM_SHARED((NB_ENT, EH), _f32),
            pltpu.SemaphoreType.DMA, pltpu.SemaphoreType.DMA,
            pltpu.SemaphoreType.DMA, pltpu.SemaphoreType.DMA,
            pltpu.SemaphoreType.DMA, pltpu.SemaphoreType.DMA,
        ],
    )
    def k(xr_hbm, scale_hbm, gidx_hbm, dst_hbm, out_hbm,
          zbuf, dstb0, dstb1, gidxb0, gidxb1, scaleb0, scaleb1, rows0, rows1,
          bins, semi0, semi1, semg0, semg1, sems0, sems1):
        cid = lax.axis_index("c")
        sid = lax.axis_index("s")
        wid = sid * NC + cid
        base = wid * ETILE
        dstb = (dstb0, dstb1)
        gidxb = (gidxb0, gidxb1)
        scaleb = (scaleb0, scaleb1)
        rows = (rows0, rows1)
        semi = (semi0, semi1)
        semg = (semg0, semg1)
        _fill_zero_2d(zbuf, ZR)
        for j in range(RPT // ZR):
            pltpu.sync_copy(zbuf, bins.at[pl.ds(sid * RPT + j * ZR, ZR)])
        plsc.subcore_barrier()

        def fire_idx(t, b):
            eb = (wid + t * NW) * CHUNK
            pltpu.async_copy(dst_hbm.at[pl.ds(eb, CHUNK)], dstb[b], semi[b])
            pltpu.async_copy(gidx_hbm.at[pl.ds(eb, CHUNK)], gidxb[b], semi[b])
            pltpu.async_copy(scale_hbm.at[pl.ds(eb, CHUNK)], scaleb[b], semi[b])

        def wait_idx(b):
            pltpu.make_async_copy(dst_hbm.at[pl.ds(0, CHUNK)], dstb[b],
                                  semi[b]).wait()
            pltpu.make_async_copy(gidx_hbm.at[pl.ds(0, CHUNK)], gidxb[b],
                                  semi[b]).wait()
            pltpu.make_async_copy(scale_hbm.at[pl.ds(0, CHUNK)], scaleb[b],
                                  semi[b]).wait()

        def fire_gather(b):
            pltpu.async_copy(xr_hbm.at[gidxb[b]], rows[b], semg[b])

        def wait_gather(b):
            pltpu.make_async_copy(xr_hbm.at[gidxb[b]], rows[b], semg[b]).wait()

        def scale_rows(b):
            def mul_body(j, _):
                splat = plsc.load_gather(
                    scaleb[b], [jnp.broadcast_to(j, (LANE,))])
                for c8 in range(EH // LANE):
                    sl = pl.ds(c8 * LANE, LANE)
                    rows[b][j, sl] = rows[b][j, sl] * splat
                return 0
            lax.fori_loop(0, CHUNK, mul_body, 0)

        fire_idx(0, 0)
        fire_idx(1, 1)
        wait_idx(0)
        fire_gather(0)

        def body(i, _):
            wait_idx(1)
            fire_gather(1)
            wait_gather(0)
            scale_rows(0)
            d = pltpu.async_copy(rows0, bins.at[dstb0], sems0, add=True)
            d.wait()
            @pl.when(i < NH - 1)
            def _():
                fire_idx(2 * i + 2, 0)
                wait_idx(0)
                fire_gather(0)
            wait_gather(1)
            scale_rows(1)
            d1 = pltpu.async_copy(rows1, bins.at[dstb1], sems1, add=True)
            d1.wait()
            @pl.when(i < NH - 1)
            def _():
                fire_idx(2 * i + 3, 1)
            return 0
        lax.fori_loop(0, NH, body, 0)
        plsc.subcore_barrier()
        def flush(j, _):
            pltpu.sync_copy(bins.at[pl.ds(sid * RPT + j * ZR, ZR)], zbuf)
            pltpu.sync_copy(zbuf, out_hbm.at[cid, pl.ds(sid * RPT + j * ZR, ZR)])
            return 0
        lax.fori_loop(0, RPT // ZR, flush, 0)
    return k


def _make_scat_kernel(ep, nbins, stage_y=False, ny=0):
    ETILE = ep // NW
    NCHUNK = ETILE // CHUNK
    RPT = nbins // NS
    ZR = 64

    NH = NCHUNK // 2
    ytab_scratch = [pltpu.VMEM_SHARED((nbins, EH), _f32)] if stage_y else []

    @functools.partial(
        pl.kernel, mesh=_mesh(), compiler_params=_SC_PARAMS,
        out_type=jax.ShapeDtypeStruct((NC, nbins, EH), _f32),
        scratch_types=[
            pltpu.VMEM((ZR, EH), _f32),
            pltpu.VMEM((CHUNK,), _i32), pltpu.VMEM((CHUNK,), _i32),
            pltpu.VMEM((CHUNK,), _i32), pltpu.VMEM((CHUNK,), _i32),
            pltpu.VMEM((CHUNK, EH), _f32), pltpu.VMEM((CHUNK, EH), _f32),
            pltpu.VMEM_SHARED((nbins, EH), _f32),
        ] + ytab_scratch + [
            pltpu.SemaphoreType.DMA, pltpu.SemaphoreType.DMA,
            pltpu.SemaphoreType.DMA, pltpu.SemaphoreType.DMA,
            pltpu.SemaphoreType.DMA, pltpu.SemaphoreType.DMA,
        ],
    )
    def k(y_hbm, row_hbm, col_hbm, out_hbm, zbuf, rowb0, rowb1, colb0, colb1,
          rows0, rows1, bins, *rest):
        if stage_y:
            ytab = rest[0]
            semi0, semi1, semg0, semg1, sems0, sems1 = rest[1:]
        else:
            ytab = y_hbm
            semi0, semi1, semg0, semg1, sems0, sems1 = rest
        cid = lax.axis_index("c")
        sid = lax.axis_index("s")
        wid = sid * NC + cid
        base = wid * ETILE
        rowb = (rowb0, rowb1)
        colb = (colb0, colb1)
        rows = (rows0, rows1)
        semi = (semi0, semi1)
        semg = (semg0, semg1)
        sems = (sems0, sems1)
        _fill_zero_2d(zbuf, ZR)
        for j in range(RPT // ZR):
            pltpu.sync_copy(zbuf, bins.at[pl.ds(sid * RPT + j * ZR, ZR)])
        plsc.subcore_barrier()

        def fire_idx(t, b):
            eb = (wid + t * NW) * CHUNK
            pltpu.async_copy(row_hbm.at[pl.ds(eb, CHUNK)], rowb[b], semi[b])
            pltpu.async_copy(col_hbm.at[pl.ds(eb, CHUNK)], colb[b], semi[b])

        def wait_idx(b):
            pltpu.make_async_copy(row_hbm.at[pl.ds(0, CHUNK)], rowb[b],
                                  semi[b]).wait()
            pltpu.make_async_copy(col_hbm.at[pl.ds(0, CHUNK)], colb[b],
                                  semi[b]).wait()

        def fire_gather(b):
            pltpu.async_copy(ytab.at[rowb[b]], rows[b], semg[b])

        def wait_gather(b):
            pltpu.make_async_copy(ytab.at[rowb[b]], rows[b], semg[b]).wait()

        if stage_y:
            nfull = ny // 80
            tail = ny - nfull * 80
            def stage(i, _):
                ci = sid + i * NS
                @pl.when(ci < nfull)
                def _():
                    r0 = ci * 80
                    pltpu.sync_copy(y_hbm.at[pl.ds(r0, 80)],
                                    rows0.at[pl.ds(0, 80)])
                    pltpu.sync_copy(rows0.at[pl.ds(0, 80)],
                                    ytab.at[pl.ds(r0, 80)])
                return 0
            lax.fori_loop(0, (nfull + NS - 1) // NS, stage, 0)
            if tail:
                @pl.when(sid == NS - 1)
                def _():
                    r0 = nfull * 80
                    pltpu.sync_copy(y_hbm.at[pl.ds(r0, tail)],
                                    rows1.at[pl.ds(0, tail)])
                    pltpu.sync_copy(rows1.at[pl.ds(0, tail)],
                                    ytab.at[pl.ds(r0, tail)])
            plsc.subcore_barrier()

        fire_idx(0, 0)
        fire_idx(1, 1)
        wait_idx(0)
        fire_gather(0)

        def body(i, _):
            wait_idx(1)
            fire_gather(1)
            wait_gather(0)
            d = pltpu.async_copy(rows0, bins.at[colb0], sems0, add=True)
            d.wait()
            @pl.when(i < NH - 1)
            def _():
                fire_idx(2 * i + 2, 0)
            @pl.when(i < NH - 1)
            def _():
                wait_idx(0)
                fire_gather(0)
            wait_gather(1)
            d1 = pltpu.async_copy(rows1, bins.at[colb1], sems1, add=True)
            d1.wait()
            @pl.when(i < NH - 1)
            def _():
                fire_idx(2 * i + 3, 1)
            return 0
        lax.fori_loop(0, NH, body, 0)
        plsc.subcore_barrier()
        def flush(j, _):
            pltpu.sync_copy(bins.at[pl.ds(sid * RPT + j * ZR, ZR)], zbuf)
            pltpu.sync_copy(zbuf, out_hbm.at[cid, pl.ds(sid * RPT + j * ZR, ZR)])
            return 0
        lax.fori_loop(0, RPT // ZR, flush, 0)
    return k


def _make_gather_kernel(ni, feat):
    KPT = ni // NW
    TR = next(t for t in (128, 80, 64, 32, 16, 8) if KPT % t == 0 and t <= KPT)
    NT = KPT // TR

    @functools.partial(
        pl.kernel, mesh=_mesh(), compiler_params=_SC_PARAMS,
        out_type=jax.ShapeDtypeStruct((ni, feat), _f32),
        scratch_types=[
            pltpu.VMEM((KPT,), _i32),
            pltpu.VMEM((TR, feat), _f32),
            pltpu.SemaphoreType.DMA,
        ],
    )
    def k(table_hbm, idx_hbm, out_hbm, idxb, rows, sem):
        cid = lax.axis_index("c")
        sid = lax.axis_index("s")
        wid = sid * NC + cid
        base = wid * KPT
        pltpu.sync_copy(idx_hbm.at[pl.ds(base, KPT)], idxb)
        for j in range(NT):
            pltpu.async_copy(table_hbm.at[idxb.at[pl.ds(j * TR, TR)]], rows,
                             sem).wait()
            pltpu.sync_copy(rows, out_hbm.at[pl.ds(base + j * TR, TR)])
    return k


def _make_merge_kernel():
    KPT = NB_MOV // NW
    TR = 32
    NT = KPT // TR
    HALF = NB_ENT // 2
    TRASH = NB_ENT - 8

    @functools.partial(
        pl.kernel, mesh=_mesh(), compiler_params=_SC_PARAMS,
        out_type=jax.ShapeDtypeStruct((NB_ENT, EH), _f32),
        scratch_types=[
            pltpu.VMEM((KPT,), _i32),
            pltpu.VMEM((NT, TR), _i32),
            pltpu.VMEM((KPT, EH), _f32),
            pltpu.VMEM((TR, EH), _f32),
            pltpu.VMEM((80, EH), _f32),
            pltpu.SemaphoreType.DMA,
        ],
    )
    def k(entA_hbm, mean_hbm, mi_hbm, out_hbm, mib, rib, meanb, rows, cbuf, sem):
        cid = lax.axis_index("c")
        sid = lax.axis_index("s")
        wid = sid * NC + cid
        def cp(i, _):
            ci = sid + i * NS
            nch = jnp.where(cid == 0, 64, 61)
            @pl.when(ci < nch)
            def _():
                r0 = cid * HALF + ci * 80
                pltpu.sync_copy(entA_hbm.at[pl.ds(r0, 80)], cbuf)
                pltpu.sync_copy(cbuf, out_hbm.at[pl.ds(r0, 80)])
            return 0
        lax.fori_loop(0, 4, cp, 0)
        plsc.subcore_barrier()
        base = wid * KPT
        pltpu.sync_copy(mi_hbm.at[pl.ds(base, KPT)], mib)
        pltpu.sync_copy(mean_hbm.at[pl.ds(base, KPT)], meanb)
        for c8 in range(KPT // LANE):
            sl = pl.ds((c8 % (TR // LANE)) * LANE, LANE)
            m16 = mib[pl.ds(c8 * LANE, LANE)]
            mine = jnp.where(cid == 0, m16 < HALF, m16 >= HALF)
            rib[c8 // (TR // LANE), sl] = jnp.where(mine, m16, TRASH)
        for j in range(NT):
            pltpu.async_copy(entA_hbm.at[mib.at[pl.ds(j * TR, TR)]], rows,
                             sem).wait()
            def addrow(i, _):
                for c8 in range(EH // LANE):
                    sl = pl.ds(c8 * LANE, LANE)
                    rows[i, sl] = rows[i, sl] + meanb[j * TR + i, sl]
                return 0
            lax.fori_loop(0, TR, addrow, 0)
            pltpu.sync_copy(rows, out_hbm.at[rib.at[j]])
    return k


def _dot(a, b):
    return jnp.dot(a, b, preferred_element_type=_f32)


def _tc_xr(node, basis, comp):
    def body(comp_ref, basis_ref, x_ref, o_ref):
        r = pl.program_id(0)
        w = comp_ref[r, 0] * basis_ref[0]
        for b in range(1, NREL):
            w = w + comp_ref[r, b] * basis_ref[b]
        o_ref[0] = _dot(x_ref[...], w)
    return pl.pallas_call(
        body,
        grid=(NREL,),
        in_specs=[
            pl.BlockSpec((NREL, NREL), lambda r: (0, 0)),
            pl.BlockSpec((NREL, EH, EH), lambda r: (0, 0, 0)),
            pl.BlockSpec((N, EH), lambda r: (0, 0)),
        ],
        out_specs=pl.BlockSpec((1, N, EH), lambda r: (r, 0, 0)),
        out_shape=jax.ShapeDtypeStruct((NREL, N, EH), _f32),
    )(comp, basis, node)


def _tc_transform_counts(cnt_seg, cnt_c, cnt_t, cnt_i):
    def body(s_ref, c_ref, t_ref, i_ref, inv_ref, dc_ref, dt_ref, di_ref):
        s = s_ref[0] + s_ref[1]
        inv_ref[...] = 1.0 / jnp.maximum(s, 1.0)
        dc_ref[...] = lax.rsqrt(c_ref[0] + c_ref[1] + 1.0)
        dt_ref[...] = lax.rsqrt(t_ref[0] + t_ref[1] + 1.0)
        di_ref[...] = lax.rsqrt(i_ref[0] + i_ref[1] + 1.0)
    r = lambda a: a.reshape(NC, -1, 128)
    outs = pl.pallas_call(
        body,
        out_shape=(
            jax.ShapeDtypeStruct((NB_SEG // 128, 128), _f32),
            jax.ShapeDtypeStruct((NB_ENT // 128, 128), _f32),
            jax.ShapeDtypeStruct((NB_MOV // 128, 128), _f32),
            jax.ShapeDtypeStruct((NB_MOV // 128, 128), _f32),
        ),
    )(r(cnt_seg), r(cnt_c), r(cnt_t), r(cnt_i))
    inv, dc, dt, di = outs
    return (inv.reshape(-1), dc.reshape(-1)[:N], dt.reshape(-1)[:NM],
            di.reshape(-1)[:NM])


def _tc_ent0(node, p0, p1, root_w, root_b, dinv_c):
    def body(x_ref, p0_ref, p1_ref, w_ref, b_ref, d_ref, e_ref, y_ref):
        e = p0_ref[...] + p1_ref[...] + _dot(x_ref[...], w_ref[...]) \
            + b_ref[...] + x_ref[...]
        e_ref[...] = e
        y_ref[...] = e * d_ref[...]
    blk = 2000
    return pl.pallas_call(
        body,
        grid=(N // blk,),
        in_specs=[
            pl.BlockSpec((blk, EH), lambda i: (i, 0)),
            pl.BlockSpec((blk, EH), lambda i: (i, 0)),
            pl.BlockSpec((blk, EH), lambda i: (i, 0)),
            pl.BlockSpec((EH, EH), lambda i: (0, 0)),
            pl.BlockSpec((1, EH), lambda i: (0, 0)),
            pl.BlockSpec((blk, 1), lambda i: (i, 0)),
        ],
        out_specs=[pl.BlockSpec((blk, EH), lambda i: (i, 0))] * 2,
        out_shape=[jax.ShapeDtypeStruct((N, EH), _f32)] * 2,
    )(node, p0, p1, root_w, root_b[None], dinv_c)


def _tc_gcn_step(n, s0, s1, y, dinv2):
    def body(s0_ref, s1_ref, y_ref, d_ref, o_ref, yn_ref):
        d = d_ref[...]
        o = d * (s0_ref[...] + s1_ref[...] + y_ref[...])
        o_ref[...] = o
        yn_ref[...] = d * o
    blk = 2000 if n == N else 1000
    return pl.pallas_call(
        body,
        grid=(n // blk,),
        in_specs=[pl.BlockSpec((blk, EH), lambda i: (i, 0))] * 3
        + [pl.BlockSpec((blk, 1), lambda i: (i, 0))],
        out_specs=[pl.BlockSpec((blk, EH), lambda i: (i, 0))] * 2,
        out_shape=[jax.ShapeDtypeStruct((n, EH), _f32)] * 2,
    )(s0, s1, y, dinv2)


def _tc_combine4(n, s0, s1, y, dinv2, a, b, c):
    def body(s0_ref, s1_ref, y_ref, d_ref, a_ref, b_ref, c_ref, o_ref):
        last = d_ref[...] * (s0_ref[...] + s1_ref[...] + y_ref[...])
        o_ref[...] = (a_ref[...] + b_ref[...] + c_ref[...] + last) * 0.25
    blk = 2000 if n == N else 1000
    return pl.pallas_call(
        body,
        grid=(n // blk,),
        in_specs=[pl.BlockSpec((blk, EH), lambda i: (i, 0))] * 3
        + [pl.BlockSpec((blk, 1), lambda i: (i, 0))]
        + [pl.BlockSpec((blk, EH), lambda i: (i, 0))] * 3,
        out_specs=pl.BlockSpec((blk, EH), lambda i: (i, 0)),
        out_shape=jax.ShapeDtypeStruct((n, EH), _f32),
    )(s0, s1, y, dinv2, a, b, c)


def _tc_prep_ti(nf, dt2, di2):
    def body(x_ref, dt_ref, di_ref, yt_ref, yi_ref):
        yt_ref[...] = x_ref[...] * dt_ref[...]
        yi_ref[...] = x_ref[...] * di_ref[...]
    blk = 1000
    return pl.pallas_call(
        body,
        grid=(NM // blk,),
        in_specs=[pl.BlockSpec((blk, EH), lambda i: (i, 0)),
                  pl.BlockSpec((blk, 1), lambda i: (i, 0)),
                  pl.BlockSpec((blk, 1), lambda i: (i, 0))],
        out_specs=[pl.BlockSpec((blk, EH), lambda i: (i, 0))] * 2,
        out_shape=[jax.ShapeDtypeStruct((NM, EH), _f32)] * 2,
    )(nf, dt2, di2)


def _tc_ent_mlp(entA, w1, b1, w2, b2, ew, eb):
    def body(x_ref, w1_ref, b1_ref, w2_ref, b2_ref, ew_ref, eb_ref, o_ref):
        x = x_ref[...]
        h = jnp.maximum(_dot(x, w1_ref[...]) + b1_ref[...], 0.0)
        m = _dot(h, w2_ref[...]) + b2_ref[...] + x
        o_ref[...] = _dot(m, ew_ref[...]) + eb_ref[...]
    blk = 2000
    return pl.pallas_call(
        body,
        grid=(N // blk,),
        in_specs=[
            pl.BlockSpec((blk, EH), lambda i: (i, 0)),
            pl.BlockSpec((EH, EH // 2), lambda i: (0, 0)),
            pl.BlockSpec((1, EH // 2), lambda i: (0, 0)),
            pl.BlockSpec((EH // 2, EH), lambda i: (0, 0)),
            pl.BlockSpec((1, EH), lambda i: (0, 0)),
            pl.BlockSpec((EH, HID), lambda i: (0, 0)),
            pl.BlockSpec((1, HID), lambda i: (0, 0)),
        ],
        out_specs=pl.BlockSpec((blk, HID), lambda i: (i, 0)),
        out_shape=jax.ShapeDtypeStruct((N, HID), _f32),
    )(entA, w1, b1[None], w2, b2[None], ew, eb[None])


def _tc_tok_mlp(tok, w1, b1, w2, b2, tw, tb):
    def body(t_ref, w1_ref, b1_ref, w2_ref, b2_ref, tw_ref, tb_ref, o_ref):
        t = t_ref[0]
        h = jnp.maximum(_dot(t, w1_ref[...]) + b1_ref[...], 0.0)
        m = _dot(h, w2_ref[...]) + b2_ref[...] + t
        o_ref[0] = _dot(m, tw_ref[...]) + tb_ref[...]
    return pl.pallas_call(
        body,
        grid=(BB,),
        in_specs=[
            pl.BlockSpec((1, LT, TOK), lambda i: (i, 0, 0)),
            pl.BlockSpec((TOK, TOK // 2), lambda i: (0, 0)),
            pl.BlockSpec((1, TOK // 2), lambda i: (0, 0)),
            pl.BlockSpec((TOK // 2, TOK), lambda i: (0, 0)),
            pl.BlockSpec((1, TOK), lambda i: (0, 0)),
            pl.BlockSpec((TOK, HID), lambda i: (0, 0)),
            pl.BlockSpec((1, HID), lambda i: (0, 0)),
        ],
        out_specs=pl.BlockSpec((1, LT, HID), lambda i: (i, 0, 0)),
        out_shape=jax.ShapeDtypeStruct((BB, LT, HID), _f32),
    )(tok, w1, b1[None], w2, b2[None], tw, tb[None])


def _tc_attn(tt, e, cross_w):
    def body(t_ref, e_ref, w_ref, o_ref):
        t = t_ref[0]
        e2 = e_ref[0]
        q = _dot(t, w_ref[...])
        a = lax.dot_general(q, e2, (((1,), (1,)), ((), ())),
                            preferred_element_type=_f32) * (1.0 / HID)
        a = a - jnp.max(a, axis=1, keepdims=True)
        ex = jnp.exp(a)
        ew = ex / jnp.sum(ex, axis=1, keepdims=True)
        o_ref[0] = _dot(ew, e2) + t
    return pl.pallas_call(
        body,
        grid=(BB,),
        in_specs=[
            pl.BlockSpec((1, LT, HID), lambda i: (i, 0, 0)),
            pl.BlockSpec((1, LE, HID), lambda i: (i, 0, 0)),
            pl.BlockSpec((HID, HID), lambda i: (0, 0)),
        ],
        out_specs=pl.BlockSpec((1, LT, HID), lambda i: (i, 0, 0)),
        out_shape=jax.ShapeDtypeStruct((BB, LT, HID), _f32),
    )(tt, e, cross_w)


def _pad1(a, n, val):
    return jnp.concatenate([a, jnp.full((n - a.shape[0],), val, a.dtype)])


def _pad_spread(a, n, lo, nspread):
    npad = n - a.shape[0]
    pad = lo + (jnp.arange(npad, dtype=a.dtype) % nspread)
    return jnp.concatenate([a, pad])


def kernel(node_embeds, basis, comp, root_w, root_b, ep1_w1, ep1_b1, ep1_w2,
           ep1_b2, ep2_w, ep2_b, tp1_w1, tp1_b1, tp1_w2, tp1_b2, tp2_w, tp2_b,
           cross_w, token_embeds, entity_ids, edge_index, edge_type,
           edge_index_c, edge_index_t_s, edge_index_i_s, movie_indices):
    kg_src = _pad1(edge_index[0], EP_KG, 0)
    kg_dst = _pad_spread(edge_index[1], EP_KG, N, 16)
    kg_et = _pad1(edge_type, EP_KG, 0)
    c_row = _pad1(edge_index_c[0], EP_C, 0)
    c_col = _pad_spread(edge_index_c[1], EP_C, N, NB_ENT - N)
    t_row = _pad1(edge_index_t_s[0], EP_S, 0)
    t_col = _pad_spread(edge_index_t_s[1], EP_S, NM, NB_MOV - NM)
    i_row = _pad1(edge_index_i_s[0], EP_S, 0)
    i_col = _pad_spread(edge_index_i_s[1], EP_S, NM, NB_MOV - NM)
    mi_pad = _pad1(movie_indices, NB_MOV, 0)
    eids = entity_ids.reshape(-1)

    cnt_seg, cnt_c, cnt_t, cnt_i = _make_cnt_kernel()(
        kg_dst, kg_et, c_col, t_col, i_col)
    inv, dinv_c, dinv_t, dinv_i = _tc_transform_counts(
        cnt_seg, cnt_c, cnt_t, cnt_i)
    dc2, dt2, di2 = dinv_c[:, None], dinv_t[:, None], dinv_i[:, None]

    xr = _tc_xr(node_embeds, basis, comp).reshape(NREL * N, EH)
    scale, gidx = _make_scale_kernel()(inv, kg_src, kg_dst, kg_et)
    rg = _make_rgcn_kernel()(xr, scale, gidx, kg_dst)
    ent0, y1c = _tc_ent0(node_embeds, rg[0, :N], rg[1, :N], root_w, root_b, dc2)

    scat_c = _make_scat_kernel(EP_C, NB_ENT)
    s1 = scat_c(y1c, c_row, c_col)
    c1, y2c = _tc_gcn_step(N, s1[0, :N], s1[1, :N], y1c, dc2)
    s2 = scat_c(y2c, c_row, c_col)
    c2, y3c = _tc_gcn_step(N, s2[0, :N], s2[1, :N], y2c, dc2)
    s3 = scat_c(y3c, c_row, c_col)
    entA = _tc_combine4(N, s3[0, :N], s3[1, :N], y3c, dc2, c1, c2, ent0)

    nf = _make_gather_kernel(NB_MOV, EH)(ent0, mi_pad)[:NM]
    y1t, y1i = _tc_prep_ti(nf, dt2, di2)
    scat_s = _make_scat_kernel(EP_S, NB_MOV, stage_y=True, ny=NM)
    st1 = scat_s(y1t, t_row, t_col)
    t1, y2t = _tc_gcn_step(NM, st1[0, :NM], st1[1, :NM], y1t, dt2)
    st2 = scat_s(y2t, t_row, t_col)
    t2, _ = _tc_gcn_step(NM, st2[0, :NM], st2[1, :NM], y2t, dt2)
    si1 = scat_s(y1i, i_row, i_col)
    i1, y2i = _tc_gcn_step(NM, si1[0, :NM], si1[1, :NM], y1i, di2)
    si2 = scat_s(y2i, i_row, i_col)
    mean = _tc_combine4(NM, si2[0, :NM], si2[1, :NM], y2i, di2, t1, t2, i1)

    mean_pad = jnp.concatenate(
        [mean, jnp.zeros((NB_MOV - NM, EH), _f32)], axis=0)
    entA2 = _make_merge_kernel()(entA, mean_pad, mi_pad)[:N]
    entF = _tc_ent_mlp(entA2, ep1_w1, ep1_b1, ep1_w2, ep1_b2, ep2_w, ep2_b)

    e = _make_gather_kernel(BB * LE, HID)(entF, eids).reshape(BB, LE, HID)
    tt = _tc_tok_mlp(token_embeds, tp1_w1, tp1_b1, tp1_w2, tp1_b2, tp2_w, tp2_b)
    return _tc_attn(tt, e, cross_w)


# speedup vs baseline: 1.3159x; 1.0092x over previous
import functools

import jax
import jax.numpy as jnp
from jax import lax
from jax.experimental import pallas as pl
from jax.experimental.pallas import tpu as pltpu
from jax.experimental.pallas import tpu_sc as plsc

N = 10000
NM = 5000
EH = 128
NREL = 8
HID = 256
TOK = 768
BB, LE, LT = 16, 32, 64

NC, NS, LANE = 2, 16, 16
NW = NC * NS

NB_ENT = 10240
NB_MOV = 5120
NB_SEG = 80128
EP_KG = 327680
EP_C = 163840
EP_S = 81920
CHUNK = 128

_f32 = jnp.float32
_i32 = jnp.int32


def _mesh():
    return plsc.VectorSubcoreMesh(core_axis_name="c", subcore_axis_name="s")


_SC_PARAMS = pltpu.CompilerParams(needs_layout_passes=False)


def _fill_zero_2d(buf, nrows):
    def body(i, _):
        for c8 in range(EH // LANE):
            buf[i, pl.ds(c8 * LANE, LANE)] = jnp.zeros((LANE,), _f32)
        return 0
    lax.fori_loop(0, nrows, body, 0)


def _fill_const_1d(buf, n, val):
    def body(i, _):
        buf[pl.ds(i * LANE, LANE)] = jnp.full((LANE,), val, _f32)
        return 0
    lax.fori_loop(0, n // LANE, body, 0)


def _make_cnt_kernel():
    ZROWS = NB_SEG // NS

    @functools.partial(
        pl.kernel, mesh=_mesh(), compiler_params=_SC_PARAMS,
        out_type=(
            jax.ShapeDtypeStruct((NC * NB_SEG,), _f32),
            jax.ShapeDtypeStruct((NC * NB_ENT,), _f32),
            jax.ShapeDtypeStruct((NC * NB_MOV,), _f32),
            jax.ShapeDtypeStruct((NC * NB_MOV,), _f32),
        ),
        scratch_types=[
            pltpu.VMEM((ZROWS,), _f32),
            pltpu.VMEM((CHUNK,), _f32),
            pltpu.VMEM((CHUNK,), _i32), pltpu.VMEM((CHUNK,), _i32),
            pltpu.VMEM((CHUNK,), _i32), pltpu.VMEM((CHUNK,), _i32),
            pltpu.VMEM((CHUNK,), _i32), pltpu.VMEM((CHUNK,), _i32),
            pltpu.VMEM_SHARED((NB_SEG,), _f32),
            pltpu.VMEM_SHARED((NB_ENT,), _f32),
            pltpu.VMEM_SHARED((NB_MOV,), _f32),
            pltpu.VMEM_SHARED((NB_MOV,), _f32),
            pltpu.SemaphoreType.DMA, pltpu.SemaphoreType.DMA,
            pltpu.SemaphoreType.DMA, pltpu.SemaphoreType.DMA,
        ],
    )
    def k(kg_dst, kg_et, c_col, t_col, i_col,
          out_seg, out_c, out_t, out_i,
          zbuf, ones, dstb0, dstb1, etb0, etb1, segb0, segb1,
          b_seg, b_c, b_t, b_i, semi0, semi1, sems0, sems1):
        cid = lax.axis_index("c")
        sid = lax.axis_index("s")
        wid = sid * NC + cid
        dstb = (dstb0, dstb1)
        etb = (etb0, etb1)
        segb = (segb0, segb1)
        semi = (semi0, semi1)
        sems = (sems0, sems1)
        _fill_const_1d(zbuf, ZROWS, 0.0)
        _fill_const_1d(ones, CHUNK, 1.0)
        for bins, nb in ((b_seg, NB_SEG), (b_c, NB_ENT), (b_t, NB_MOV), (b_i, NB_MOV)):
            per = nb // NS
            pltpu.sync_copy(zbuf.at[pl.ds(0, per)], bins.at[pl.ds(sid * per, per)])
        plsc.subcore_barrier()

        def run_phase(idx_hbms, bins, ep, with_seg):
            nch = ep // NW // CHUNK
            nh = nch // 2

            def fire_idx(t, b):
                eb = (wid + t * NW) * CHUNK
                for h, dst in zip(idx_hbms, (dstb[b], etb[b])):
                    pltpu.async_copy(h.at[pl.ds(eb, CHUNK)], dst, semi[b])

            def wait_idx(b):
                for h, dst in zip(idx_hbms, (dstb[b], etb[b])):
                    pltpu.make_async_copy(h.at[pl.ds(0, CHUNK)], dst,
                                          semi[b]).wait()

            def half(i, b):
                wait_idx(b)
                if with_seg:
                    for c8 in range(CHUNK // LANE):
                        sl = pl.ds(c8 * LANE, LANE)
                        segb[b][sl] = dstb[b][sl] * NREL + etb[b][sl]
                    key = segb[b]
                else:
                    key = dstb[b]
                d = pltpu.async_copy(ones, bins.at[key], sems[b], add=True)
                d.wait()
                @pl.when(i < nh - 1)
                def _():
                    fire_idx(2 * i + 2 + b, b)

            fire_idx(0, 0)
            fire_idx(1, 1)
            def body(i, _):
                half(i, 0)
                half(i, 1)
                return 0
            lax.fori_loop(0, nh, body, 0)

        run_phase((kg_dst, kg_et), b_seg, EP_KG, True)
        run_phase((c_col,), b_c, EP_C, False)
        run_phase((t_col,), b_t, EP_S, False)
        run_phase((i_col,), b_i, EP_S, False)
        plsc.subcore_barrier()
        for bins, out, nb in ((b_seg, out_seg, NB_SEG), (b_c, out_c, NB_ENT),
                              (b_t, out_t, NB_MOV), (b_i, out_i, NB_MOV)):
            per = nb // NS
            pltpu.sync_copy(bins.at[pl.ds(sid * per, per)], zbuf.at[pl.ds(0, per)])
            pltpu.sync_copy(zbuf.at[pl.ds(0, per)],
                            out.at[pl.ds(cid * nb + sid * per, per)])
    return k


def _make_scale_kernel():
    ETILE = EP_KG // NW
    NCHUNK = ETILE // CHUNK

    @functools.partial(
        pl.kernel, mesh=_mesh(), compiler_params=_SC_PARAMS,
        out_type=(
            jax.ShapeDtypeStruct((EP_KG,), _f32),
            jax.ShapeDtypeStruct((EP_KG,), _i32),
        ),
        scratch_types=[
            pltpu.VMEM((NB_SEG,), _f32),
            pltpu.VMEM((CHUNK,), _i32), pltpu.VMEM((CHUNK,), _i32),
            pltpu.VMEM((CHUNK,), _i32), pltpu.VMEM((CHUNK,), _i32),
            pltpu.VMEM((CHUNK,), _i32), pltpu.VMEM((CHUNK,), _i32),
            pltpu.VMEM((CHUNK,), _i32), pltpu.VMEM((CHUNK,), _i32),
            pltpu.VMEM((CHUNK,), _f32), pltpu.VMEM((CHUNK,), _f32),
            pltpu.SemaphoreType.DMA, pltpu.SemaphoreType.DMA,
            pltpu.SemaphoreType.DMA, pltpu.SemaphoreType.DMA,
        ],
    )
    def k(inv_hbm, src_hbm, dst_hbm, et_hbm, scale_out, gidx_out,
          invt, srcb0, srcb1, dstb0, dstb1, etb0, etb1, gidxb0, gidxb1,
          scaleb0, scaleb1, semi0, semi1, semo0, semo1):
        cid = lax.axis_index("c")
        sid = lax.axis_index("s")
        wid = sid * NC + cid
        srcb = (srcb0, srcb1)
        dstb = (dstb0, dstb1)
        etb = (etb0, etb1)
        gidxb = (gidxb0, gidxb1)
        scaleb = (scaleb0, scaleb1)
        semi = (semi0, semi1)
        semo = (semo0, semo1)
        pltpu.sync_copy(inv_hbm, invt)
        NH = NCHUNK // 2

        def fire_idx(t, b):
            eb = wid * ETILE + t * CHUNK
            pltpu.async_copy(src_hbm.at[pl.ds(eb, CHUNK)], srcb[b], semi[b])
            pltpu.async_copy(dst_hbm.at[pl.ds(eb, CHUNK)], dstb[b], semi[b])
            pltpu.async_copy(et_hbm.at[pl.ds(eb, CHUNK)], etb[b], semi[b])

        def wait_idx(b):
            for h, d in ((src_hbm, srcb[b]), (dst_hbm, dstb[b]),
                         (et_hbm, etb[b])):
                pltpu.make_async_copy(h.at[pl.ds(0, CHUNK)], d, semi[b]).wait()

        def wait_out(b):
            pltpu.make_async_copy(scaleb[b], scale_out.at[pl.ds(0, CHUNK)],
                                  semo[b]).wait()
            pltpu.make_async_copy(gidxb[b], gidx_out.at[pl.ds(0, CHUNK)],
                                  semo[b]).wait()

        def half(i, b):
            t = 2 * i + b
            wait_idx(b)
            @pl.when(i > 0)
            def _():
                wait_out(b)
            for c8 in range(CHUNK // LANE):
                sl = pl.ds(c8 * LANE, LANE)
                e16 = etb[b][sl]
                gidxb[b][sl] = e16 * N + srcb[b][sl]
                scaleb[b][sl] = plsc.load_gather(
                    invt, [dstb[b][sl] * NREL + e16])
            eb = wid * ETILE + t * CHUNK
            pltpu.async_copy(scaleb[b], scale_out.at[pl.ds(eb, CHUNK)], semo[b])
            pltpu.async_copy(gidxb[b], gidx_out.at[pl.ds(eb, CHUNK)], semo[b])
            @pl.when(i < NH - 1)
            def _():
                fire_idx(t + 2, b)

        fire_idx(0, 0)
        fire_idx(1, 1)
        def body(i, _):
            half(i, 0)
            half(i, 1)
            return 0
        lax.fori_loop(0, NH, body, 0)
        wait_out(0)
        wait_out(1)
    return k


def _make_rgcn_kernel():
    ETILE = EP_KG // NW
    NCHUNK = ETILE // CHUNK
    RPT = NB_ENT // NS
    ZR = 64

    NH = NCHUNK // 2

    @functools.partial(
        pl.kernel, mesh=_mesh(), compiler_params=_SC_PARAMS,
        out_type=jax.ShapeDtypeStruct((NC, NB_ENT, EH), _f32),
        scratch_types=[
            pltpu.VMEM((ZR, EH), _f32),
            pltpu.VMEM((CHUNK,), _i32), pltpu.VMEM((CHUNK,), _i32),
            pltpu.VMEM((CHUNK,), _i32), pltpu.VMEM((CHUNK,), _i32),
            pltpu.VMEM((CHUNK,), _f32), pltpu.VMEM((CHUNK,), _f32),
            pltpu.VMEM((CHUNK, EH), _f32), pltpu.VMEM((CHUNK, EH), _f32),
            pltpu.VMEM_SHARED((NB_ENT, EH), _f32),
            pltpu.SemaphoreType.DMA, pltpu.SemaphoreType.DMA,
            pltpu.SemaphoreType.DMA, pltpu.SemaphoreType.DMA,
            pltpu.SemaphoreType.DMA, pltpu.SemaphoreType.DMA,
        ],
    )
    def k(xr_hbm, scale_hbm, gidx_hbm, dst_hbm, out_hbm,
          zbuf, dstb0, dstb1, gidxb0, gidxb1, scaleb0, scaleb1, rows0, rows1,
          bins, semi0, semi1, semg0, semg1, sems0, sems1):
        cid = lax.axis_index("c")
        sid = lax.axis_index("s")
        wid = sid * NC + cid
        base = wid * ETILE
        dstb = (dstb0, dstb1)
        gidxb = (gidxb0, gidxb1)
        scaleb = (scaleb0, scaleb1)
        rows = (rows0, rows1)
        semi = (semi0, semi1)
        semg = (semg0, semg1)
        _fill_zero_2d(zbuf, ZR)
        for j in range(RPT // ZR):
            pltpu.async_copy(zbuf, bins.at[pl.ds(sid * RPT + j * ZR, ZR)],
                             semi0)
        for j in range(RPT // ZR):
            pltpu.make_async_copy(zbuf, bins.at[pl.ds(0, ZR)], semi0).wait()
        plsc.subcore_barrier()

        def fire_idx(t, b):
            eb = (wid + t * NW) * CHUNK
            pltpu.async_copy(dst_hbm.at[pl.ds(eb, CHUNK)], dstb[b], semi[b])
            pltpu.async_copy(gidx_hbm.at[pl.ds(eb, CHUNK)], gidxb[b], semi[b])
            pltpu.async_copy(scale_hbm.at[pl.ds(eb, CHUNK)], scaleb[b], semi[b])

        def wait_idx(b):
            pltpu.make_async_copy(dst_hbm.at[pl.ds(0, CHUNK)], dstb[b],
                                  semi[b]).wait()
            pltpu.make_async_copy(gidx_hbm.at[pl.ds(0, CHUNK)], gidxb[b],
                                  semi[b]).wait()
            pltpu.make_async_copy(scale_hbm.at[pl.ds(0, CHUNK)], scaleb[b],
                                  semi[b]).wait()

        def fire_gather(b):
            pltpu.async_copy(xr_hbm.at[gidxb[b]], rows[b], semg[b])

        def wait_gather(b):
            pltpu.make_async_copy(xr_hbm.at[gidxb[b]], rows[b], semg[b]).wait()

        def scale_rows(b):
            def mul_body(j, _):
                splat = plsc.load_gather(
                    scaleb[b], [jnp.broadcast_to(j, (LANE,))])
                for c8 in range(EH // LANE):
                    sl = pl.ds(c8 * LANE, LANE)
                    rows[b][j, sl] = rows[b][j, sl] * splat
                return 0
            lax.fori_loop(0, CHUNK, mul_body, 0)

        fire_idx(0, 0)
        fire_idx(1, 1)
        wait_idx(0)
        fire_gather(0)

        def body(i, _):
            wait_idx(1)
            fire_gather(1)
            wait_gather(0)
            scale_rows(0)
            d = pltpu.async_copy(rows0, bins.at[dstb0], sems0, add=True)
            d.wait()
            @pl.when(i < NH - 1)
            def _():
                fire_idx(2 * i + 2, 0)
                wait_idx(0)
                fire_gather(0)
            wait_gather(1)
            scale_rows(1)
            d1 = pltpu.async_copy(rows1, bins.at[dstb1], sems1, add=True)
            d1.wait()
            @pl.when(i < NH - 1)
            def _():
                fire_idx(2 * i + 3, 1)
            return 0
        lax.fori_loop(0, NH, body, 0)
        plsc.subcore_barrier()
        FCH = 64
        nfl = RPT // FCH
        rowsbufs = (rows0, rows1)
        semg = (semg0, semg1)
        sems = (sems0, sems1)
        def fl_load(j, b):
            pltpu.async_copy(bins.at[pl.ds(sid * RPT + j * FCH, FCH)],
                             rowsbufs[b].at[pl.ds(0, FCH)], semg[b])
        def fl_wait_load(b):
            pltpu.make_async_copy(bins.at[pl.ds(0, FCH)],
                                  rowsbufs[b].at[pl.ds(0, FCH)],
                                  semg[b]).wait()
        def fl_store(j, b):
            pltpu.async_copy(rowsbufs[b].at[pl.ds(0, FCH)],
                             out_hbm.at[cid, pl.ds(sid * RPT + j * FCH, FCH)],
                             sems[b])
        def fl_wait_store(b):
            pltpu.make_async_copy(rowsbufs[b].at[pl.ds(0, FCH)],
                                  out_hbm.at[cid, pl.ds(0, FCH)],
                                  sems[b]).wait()
        fl_load(0, 0)
        for j in range(nfl):
            b = j % 2
            fl_wait_load(b)
            fl_store(j, b)
            if j + 1 < nfl:
                if j >= 1:
                    fl_wait_store(1 - b)
                fl_load(j + 1, 1 - b)
        fl_wait_store((nfl - 1) % 2)
        if nfl >= 2:
            fl_wait_store(nfl % 2)
    return k


def _make_scat_kernel(ep, nbins, stage_y=False, ny=0):
    ETILE = ep // NW
    NCHUNK = ETILE // CHUNK
    RPT = nbins // NS
    ZR = 64

    NH = NCHUNK // 2
    ytab_scratch = [pltpu.VMEM_SHARED((nbins, EH), _f32)] if stage_y else []

    @functools.partial(
        pl.kernel, mesh=_mesh(), compiler_params=_SC_PARAMS,
        out_type=jax.ShapeDtypeStruct((NC, nbins, EH), _f32),
        scratch_types=[
            pltpu.VMEM((ZR, EH), _f32),
            pltpu.VMEM((CHUNK,), _i32), pltpu.VMEM((CHUNK,), _i32),
            pltpu.VMEM((CHUNK,), _i32), pltpu.VMEM((CHUNK,), _i32),
            pltpu.VMEM((CHUNK, EH), _f32), pltpu.VMEM((CHUNK, EH), _f32),
            pltpu.VMEM_SHARED((nbins, EH), _f32),
        ] + ytab_scratch + [
            pltpu.SemaphoreType.DMA, pltpu.SemaphoreType.DMA,
            pltpu.SemaphoreType.DMA, pltpu.SemaphoreType.DMA,
            pltpu.SemaphoreType.DMA, pltpu.SemaphoreType.DMA,
        ],
    )
    def k(y_hbm, row_hbm, col_hbm, out_hbm, zbuf, rowb0, rowb1, colb0, colb1,
          rows0, rows1, bins, *rest):
        if stage_y:
            ytab = rest[0]
            semi0, semi1, semg0, semg1, sems0, sems1 = rest[1:]
        else:
            ytab = y_hbm
            semi0, semi1, semg0, semg1, sems0, sems1 = rest
        cid = lax.axis_index("c")
        sid = lax.axis_index("s")
        wid = sid * NC + cid
        base = wid * ETILE
        rowb = (rowb0, rowb1)
        colb = (colb0, colb1)
        rows = (rows0, rows1)
        semi = (semi0, semi1)
        semg = (semg0, semg1)
        sems = (sems0, sems1)
        _fill_zero_2d(zbuf, ZR)
        for j in range(RPT // ZR):
            pltpu.async_copy(zbuf, bins.at[pl.ds(sid * RPT + j * ZR, ZR)],
                             semi0)
        for j in range(RPT // ZR):
            pltpu.make_async_copy(zbuf, bins.at[pl.ds(0, ZR)], semi0).wait()
        plsc.subcore_barrier()

        def fire_idx(t, b):
            eb = (wid + t * NW) * CHUNK
            pltpu.async_copy(row_hbm.at[pl.ds(eb, CHUNK)], rowb[b], semi[b])
            pltpu.async_copy(col_hbm.at[pl.ds(eb, CHUNK)], colb[b], semi[b])

        def wait_idx(b):
            pltpu.make_async_copy(row_hbm.at[pl.ds(0, CHUNK)], rowb[b],
                                  semi[b]).wait()
            pltpu.make_async_copy(col_hbm.at[pl.ds(0, CHUNK)], colb[b],
                                  semi[b]).wait()

        def fire_gather(b):
            pltpu.async_copy(ytab.at[rowb[b]], rows[b], semg[b])

        def wait_gather(b):
            pltpu.make_async_copy(ytab.at[rowb[b]], rows[b], semg[b]).wait()

        if stage_y:
            nfull = ny // 80
            tail = ny - nfull * 80
            def stage(i, _):
                ci = sid + i * NS
                @pl.when(ci < nfull)
                def _():
                    r0 = ci * 80
                    pltpu.sync_copy(y_hbm.at[pl.ds(r0, 80)],
                                    rows0.at[pl.ds(0, 80)])
                    pltpu.sync_copy(rows0.at[pl.ds(0, 80)],
                                    ytab.at[pl.ds(r0, 80)])
                return 0
            lax.fori_loop(0, (nfull + NS - 1) // NS, stage, 0)
            if tail:
                @pl.when(sid == NS - 1)
                def _():
                    r0 = nfull * 80
                    pltpu.sync_copy(y_hbm.at[pl.ds(r0, tail)],
                                    rows1.at[pl.ds(0, tail)])
                    pltpu.sync_copy(rows1.at[pl.ds(0, tail)],
                                    ytab.at[pl.ds(r0, tail)])
            plsc.subcore_barrier()

        fire_idx(0, 0)
        fire_idx(1, 1)
        wait_idx(0)
        fire_gather(0)

        def body(i, _):
            wait_idx(1)
            fire_gather(1)
            wait_gather(0)
            d = pltpu.async_copy(rows0, bins.at[colb0], sems0, add=True)
            d.wait()
            @pl.when(i < NH - 1)
            def _():
                fire_idx(2 * i + 2, 0)
            @pl.when(i < NH - 1)
            def _():
                wait_idx(0)
                fire_gather(0)
            wait_gather(1)
            d1 = pltpu.async_copy(rows1, bins.at[colb1], sems1, add=True)
            d1.wait()
            @pl.when(i < NH - 1)
            def _():
                fire_idx(2 * i + 3, 1)
            return 0
        lax.fori_loop(0, NH, body, 0)
        plsc.subcore_barrier()
        FCH = 64
        nfl = RPT // FCH
        rowsbufs = (rows0, rows1)
        def fl_load(j, b):
            pltpu.async_copy(bins.at[pl.ds(sid * RPT + j * FCH, FCH)],
                             rowsbufs[b].at[pl.ds(0, FCH)], semg[b])
        def fl_wait_load(b):
            pltpu.make_async_copy(bins.at[pl.ds(0, FCH)],
                                  rowsbufs[b].at[pl.ds(0, FCH)],
                                  semg[b]).wait()
        def fl_store(j, b):
            pltpu.async_copy(rowsbufs[b].at[pl.ds(0, FCH)],
                             out_hbm.at[cid, pl.ds(sid * RPT + j * FCH, FCH)],
                             sems[b])
        def fl_wait_store(b):
            pltpu.make_async_copy(rowsbufs[b].at[pl.ds(0, FCH)],
                                  out_hbm.at[cid, pl.ds(0, FCH)],
                                  sems[b]).wait()
        fl_load(0, 0)
        for j in range(nfl):
            b = j % 2
            fl_wait_load(b)
            fl_store(j, b)
            if j + 1 < nfl:
                if j >= 1:
                    fl_wait_store(1 - b)
                fl_load(j + 1, 1 - b)
        fl_wait_store((nfl - 1) % 2)
        if nfl >= 2:
            fl_wait_store(nfl % 2)
    return k


def _make_gather_kernel(ni, feat):
    KPT = ni // NW
    TR = next(t for t in (128, 80, 64, 32, 16, 8) if KPT % t == 0 and t <= KPT)
    NT = KPT // TR

    @functools.partial(
        pl.kernel, mesh=_mesh(), compiler_params=_SC_PARAMS,
        out_type=jax.ShapeDtypeStruct((ni, feat), _f32),
        scratch_types=[
            pltpu.VMEM((KPT,), _i32),
            pltpu.VMEM((TR, feat), _f32),
            pltpu.SemaphoreType.DMA,
        ],
    )
    def k(table_hbm, idx_hbm, out_hbm, idxb, rows, sem):
        cid = lax.axis_index("c")
        sid = lax.axis_index("s")
        wid = sid * NC + cid
        base = wid * KPT
        pltpu.sync_copy(idx_hbm.at[pl.ds(base, KPT)], idxb)
        for j in range(NT):
            pltpu.async_copy(table_hbm.at[idxb.at[pl.ds(j * TR, TR)]], rows,
                             sem).wait()
            pltpu.sync_copy(rows, out_hbm.at[pl.ds(base + j * TR, TR)])
    return k


def _make_merge_kernel():
    KPT = NB_MOV // NW
    TR = 32
    NT = KPT // TR
    HALF = NB_ENT // 2
    TRASH = NB_ENT - 8

    @functools.partial(
        pl.kernel, mesh=_mesh(), compiler_params=_SC_PARAMS,
        out_type=jax.ShapeDtypeStruct((NB_ENT, EH), _f32),
        scratch_types=[
            pltpu.VMEM((KPT,), _i32),
            pltpu.VMEM((NT, TR), _i32),
            pltpu.VMEM((KPT, EH), _f32),
            pltpu.VMEM((TR, EH), _f32),
            pltpu.VMEM((80, EH), _f32),
            pltpu.SemaphoreType.DMA,
        ],
    )
    def k(entA_hbm, mean_hbm, mi_hbm, out_hbm, mib, rib, meanb, rows, cbuf, sem):
        cid = lax.axis_index("c")
        sid = lax.axis_index("s")
        wid = sid * NC + cid
        def cp(i, _):
            ci = sid + i * NS
            nch = jnp.where(cid == 0, 64, 61)
            @pl.when(ci < nch)
            def _():
                r0 = cid * HALF + ci * 80
                pltpu.sync_copy(entA_hbm.at[pl.ds(r0, 80)], cbuf)
                pltpu.sync_copy(cbuf, out_hbm.at[pl.ds(r0, 80)])
            return 0
        lax.fori_loop(0, 4, cp, 0)
        plsc.subcore_barrier()
        base = wid * KPT
        pltpu.sync_copy(mi_hbm.at[pl.ds(base, KPT)], mib)
        pltpu.sync_copy(mean_hbm.at[pl.ds(base, KPT)], meanb)
        for c8 in range(KPT // LANE):
            sl = pl.ds((c8 % (TR // LANE)) * LANE, LANE)
            m16 = mib[pl.ds(c8 * LANE, LANE)]
            mine = jnp.where(cid == 0, m16 < HALF, m16 >= HALF)
            rib[c8 // (TR // LANE), sl] = jnp.where(mine, m16, TRASH)
        for j in range(NT):
            pltpu.async_copy(entA_hbm.at[mib.at[pl.ds(j * TR, TR)]], rows,
                             sem).wait()
            def addrow(i, _):
                for c8 in range(EH // LANE):
                    sl = pl.ds(c8 * LANE, LANE)
                    rows[i, sl] = rows[i, sl] + meanb[j * TR + i, sl]
                return 0
            lax.fori_loop(0, TR, addrow, 0)
            pltpu.sync_copy(rows, out_hbm.at[rib.at[j]])
    return k


def _dot(a, b):
    return jnp.dot(a, b, preferred_element_type=_f32)


def _tc_xr(node, basis, comp):
    def body(comp_ref, basis_ref, x_ref, o_ref):
        r = pl.program_id(0)
        w = comp_ref[r, 0] * basis_ref[0]
        for b in range(1, NREL):
            w = w + comp_ref[r, b] * basis_ref[b]
        o_ref[0] = _dot(x_ref[...], w)
    return pl.pallas_call(
        body,
        grid=(NREL,),
        in_specs=[
            pl.BlockSpec((NREL, NREL), lambda r: (0, 0)),
            pl.BlockSpec((NREL, EH, EH), lambda r: (0, 0, 0)),
            pl.BlockSpec((N, EH), lambda r: (0, 0)),
        ],
        out_specs=pl.BlockSpec((1, N, EH), lambda r: (r, 0, 0)),
        out_shape=jax.ShapeDtypeStruct((NREL, N, EH), _f32),
    )(comp, basis, node)


def _tc_transform_counts(cnt_seg, cnt_c, cnt_t, cnt_i):
    def body(s_ref, c_ref, t_ref, i_ref, inv_ref, dc_ref, dt_ref, di_ref):
        s = s_ref[0] + s_ref[1]
        inv_ref[...] = 1.0 / jnp.maximum(s, 1.0)
        dc_ref[...] = lax.rsqrt(c_ref[0] + c_ref[1] + 1.0)
        dt_ref[...] = lax.rsqrt(t_ref[0] + t_ref[1] + 1.0)
        di_ref[...] = lax.rsqrt(i_ref[0] + i_ref[1] + 1.0)
    r = lambda a: a.reshape(NC, -1, 128)
    outs = pl.pallas_call(
        body,
        out_shape=(
            jax.ShapeDtypeStruct((NB_SEG // 128, 128), _f32),
            jax.ShapeDtypeStruct((NB_ENT // 128, 128), _f32),
            jax.ShapeDtypeStruct((NB_MOV // 128, 128), _f32),
            jax.ShapeDtypeStruct((NB_MOV // 128, 128), _f32),
        ),
    )(r(cnt_seg), r(cnt_c), r(cnt_t), r(cnt_i))
    inv, dc, dt, di = outs
    return (inv.reshape(-1), dc.reshape(-1)[:N], dt.reshape(-1)[:NM],
            di.reshape(-1)[:NM])


def _tc_ent0(node, p0, p1, root_w, root_b, dinv_c):
    def body(x_ref, p0_ref, p1_ref, w_ref, b_ref, d_ref, e_ref, y_ref):
        e = p0_ref[...] + p1_ref[...] + _dot(x_ref[...], w_ref[...]) \
            + b_ref[...] + x_ref[...]
        e_ref[...] = e
        y_ref[...] = e * d_ref[...]
    blk = 2000
    return pl.pallas_call(
        body,
        grid=(N // blk,),
        in_specs=[
            pl.BlockSpec((blk, EH), lambda i: (i, 0)),
            pl.BlockSpec((blk, EH), lambda i: (i, 0)),
            pl.BlockSpec((blk, EH), lambda i: (i, 0)),
            pl.BlockSpec((EH, EH), lambda i: (0, 0)),
            pl.BlockSpec((1, EH), lambda i: (0, 0)),
            pl.BlockSpec((blk, 1), lambda i: (i, 0)),
        ],
        out_specs=[pl.BlockSpec((blk, EH), lambda i: (i, 0))] * 2,
        out_shape=[jax.ShapeDtypeStruct((N, EH), _f32)] * 2,
    )(node, p0, p1, root_w, root_b[None], dinv_c)


def _tc_gcn_step(n, s0, s1, y, dinv2):
    def body(s0_ref, s1_ref, y_ref, d_ref, o_ref, yn_ref):
        d = d_ref[...]
        o = d * (s0_ref[...] + s1_ref[...] + y_ref[...])
        o_ref[...] = o
        yn_ref[...] = d * o
    blk = 2000 if n == N else 1000
    return pl.pallas_call(
        body,
        grid=(n // blk,),
        in_specs=[pl.BlockSpec((blk, EH), lambda i: (i, 0))] * 3
        + [pl.BlockSpec((blk, 1), lambda i: (i, 0))],
        out_specs=[pl.BlockSpec((blk, EH), lambda i: (i, 0))] * 2,
        out_shape=[jax.ShapeDtypeStruct((n, EH), _f32)] * 2,
    )(s0, s1, y, dinv2)


def _tc_combine4(n, s0, s1, y, dinv2, a, b, c):
    def body(s0_ref, s1_ref, y_ref, d_ref, a_ref, b_ref, c_ref, o_ref):
        last = d_ref[...] * (s0_ref[...] + s1_ref[...] + y_ref[...])
        o_ref[...] = (a_ref[...] + b_ref[...] + c_ref[...] + last) * 0.25
    blk = 2000 if n == N else 1000
    return pl.pallas_call(
        body,
        grid=(n // blk,),
        in_specs=[pl.BlockSpec((blk, EH), lambda i: (i, 0))] * 3
        + [pl.BlockSpec((blk, 1), lambda i: (i, 0))]
        + [pl.BlockSpec((blk, EH), lambda i: (i, 0))] * 3,
        out_specs=pl.BlockSpec((blk, EH), lambda i: (i, 0)),
        out_shape=jax.ShapeDtypeStruct((n, EH), _f32),
    )(s0, s1, y, dinv2, a, b, c)


def _tc_prep_ti(nf, dt2, di2):
    def body(x_ref, dt_ref, di_ref, yt_ref, yi_ref):
        yt_ref[...] = x_ref[...] * dt_ref[...]
        yi_ref[...] = x_ref[...] * di_ref[...]
    blk = 1000
    return pl.pallas_call(
        body,
        grid=(NM // blk,),
        in_specs=[pl.BlockSpec((blk, EH), lambda i: (i, 0)),
                  pl.BlockSpec((blk, 1), lambda i: (i, 0)),
                  pl.BlockSpec((blk, 1), lambda i: (i, 0))],
        out_specs=[pl.BlockSpec((blk, EH), lambda i: (i, 0))] * 2,
        out_shape=[jax.ShapeDtypeStruct((NM, EH), _f32)] * 2,
    )(nf, dt2, di2)


def _tc_ent_mlp(entA, w1, b1, w2, b2, ew, eb):
    def body(x_ref, w1_ref, b1_ref, w2_ref, b2_ref, ew_ref, eb_ref, o_ref):
        x = x_ref[...]
        h = jnp.maximum(_dot(x, w1_ref[...]) + b1_ref[...], 0.0)
        m = _dot(h, w2_ref[...]) + b2_ref[...] + x
        o_ref[...] = _dot(m, ew_ref[...]) + eb_ref[...]
    blk = 2000
    return pl.pallas_call(
        body,
        grid=(N // blk,),
        in_specs=[
            pl.BlockSpec((blk, EH), lambda i: (i, 0)),
            pl.BlockSpec((EH, EH // 2), lambda i: (0, 0)),
            pl.BlockSpec((1, EH // 2), lambda i: (0, 0)),
            pl.BlockSpec((EH // 2, EH), lambda i: (0, 0)),
            pl.BlockSpec((1, EH), lambda i: (0, 0)),
            pl.BlockSpec((EH, HID), lambda i: (0, 0)),
            pl.BlockSpec((1, HID), lambda i: (0, 0)),
        ],
        out_specs=pl.BlockSpec((blk, HID), lambda i: (i, 0)),
        out_shape=jax.ShapeDtypeStruct((N, HID), _f32),
    )(entA, w1, b1[None], w2, b2[None], ew, eb[None])


def _tc_tok_mlp(tok, w1, b1, w2, b2, tw, tb):
    def body(t_ref, w1_ref, b1_ref, w2_ref, b2_ref, tw_ref, tb_ref, o_ref):
        t = t_ref[0]
        h = jnp.maximum(_dot(t, w1_ref[...]) + b1_ref[...], 0.0)
        m = _dot(h, w2_ref[...]) + b2_ref[...] + t
        o_ref[0] = _dot(m, tw_ref[...]) + tb_ref[...]
    return pl.pallas_call(
        body,
        grid=(BB,),
        in_specs=[
            pl.BlockSpec((1, LT, TOK), lambda i: (i, 0, 0)),
            pl.BlockSpec((TOK, TOK // 2), lambda i: (0, 0)),
            pl.BlockSpec((1, TOK // 2), lambda i: (0, 0)),
            pl.BlockSpec((TOK // 2, TOK), lambda i: (0, 0)),
            pl.BlockSpec((1, TOK), lambda i: (0, 0)),
            pl.BlockSpec((TOK, HID), lambda i: (0, 0)),
            pl.BlockSpec((1, HID), lambda i: (0, 0)),
        ],
        out_specs=pl.BlockSpec((1, LT, HID), lambda i: (i, 0, 0)),
        out_shape=jax.ShapeDtypeStruct((BB, LT, HID), _f32),
    )(tok, w1, b1[None], w2, b2[None], tw, tb[None])


def _tc_attn(tt, e, cross_w):
    def body(t_ref, e_ref, w_ref, o_ref):
        t = t_ref[0]
        e2 = e_ref[0]
        q = _dot(t, w_ref[...])
        a = lax.dot_general(q, e2, (((1,), (1,)), ((), ())),
                            preferred_element_type=_f32) * (1.0 / HID)
        a = a - jnp.max(a, axis=1, keepdims=True)
        ex = jnp.exp(a)
        ew = ex / jnp.sum(ex, axis=1, keepdims=True)
        o_ref[0] = _dot(ew, e2) + t
    return pl.pallas_call(
        body,
        grid=(BB,),
        in_specs=[
            pl.BlockSpec((1, LT, HID), lambda i: (i, 0, 0)),
            pl.BlockSpec((1, LE, HID), lambda i: (i, 0, 0)),
            pl.BlockSpec((HID, HID), lambda i: (0, 0)),
        ],
        out_specs=pl.BlockSpec((1, LT, HID), lambda i: (i, 0, 0)),
        out_shape=jax.ShapeDtypeStruct((BB, LT, HID), _f32),
    )(tt, e, cross_w)


def _pad1(a, n, val):
    return jnp.concatenate([a, jnp.full((n - a.shape[0],), val, a.dtype)])


def _pad_spread(a, n, lo, nspread):
    npad = n - a.shape[0]
    pad = lo + (jnp.arange(npad, dtype=a.dtype) % nspread)
    return jnp.concatenate([a, pad])


def kernel(node_embeds, basis, comp, root_w, root_b, ep1_w1, ep1_b1, ep1_w2,
           ep1_b2, ep2_w, ep2_b, tp1_w1, tp1_b1, tp1_w2, tp1_b2, tp2_w, tp2_b,
           cross_w, token_embeds, entity_ids, edge_index, edge_type,
           edge_index_c, edge_index_t_s, edge_index_i_s, movie_indices):
    kg_src = _pad1(edge_index[0], EP_KG, 0)
    kg_dst = _pad_spread(edge_index[1], EP_KG, N, 16)
    kg_et = _pad1(edge_type, EP_KG, 0)
    c_row = _pad1(edge_index_c[0], EP_C, 0)
    c_col = _pad_spread(edge_index_c[1], EP_C, N, NB_ENT - N)
    t_row = _pad1(edge_index_t_s[0], EP_S, 0)
    t_col = _pad_spread(edge_index_t_s[1], EP_S, NM, NB_MOV - NM)
    i_row = _pad1(edge_index_i_s[0], EP_S, 0)
    i_col = _pad_spread(edge_index_i_s[1], EP_S, NM, NB_MOV - NM)
    mi_pad = _pad1(movie_indices, NB_MOV, 0)
    eids = entity_ids.reshape(-1)

    cnt_seg, cnt_c, cnt_t, cnt_i = _make_cnt_kernel()(
        kg_dst, kg_et, c_col, t_col, i_col)
    inv, dinv_c, dinv_t, dinv_i = _tc_transform_counts(
        cnt_seg, cnt_c, cnt_t, cnt_i)
    dc2, dt2, di2 = dinv_c[:, None], dinv_t[:, None], dinv_i[:, None]

    xr = _tc_xr(node_embeds, basis, comp).reshape(NREL * N, EH)
    scale, gidx = _make_scale_kernel()(inv, kg_src, kg_dst, kg_et)
    rg = _make_rgcn_kernel()(xr, scale, gidx, kg_dst)
    ent0, y1c = _tc_ent0(node_embeds, rg[0, :N], rg[1, :N], root_w, root_b, dc2)

    scat_c = _make_scat_kernel(EP_C, NB_ENT)
    s1 = scat_c(y1c, c_row, c_col)
    c1, y2c = _tc_gcn_step(N, s1[0, :N], s1[1, :N], y1c, dc2)
    s2 = scat_c(y2c, c_row, c_col)
    c2, y3c = _tc_gcn_step(N, s2[0, :N], s2[1, :N], y2c, dc2)
    s3 = scat_c(y3c, c_row, c_col)
    entA = _tc_combine4(N, s3[0, :N], s3[1, :N], y3c, dc2, c1, c2, ent0)

    nf = _make_gather_kernel(NB_MOV, EH)(ent0, mi_pad)[:NM]
    y1t, y1i = _tc_prep_ti(nf, dt2, di2)
    scat_s = _make_scat_kernel(EP_S, NB_MOV, stage_y=True, ny=NM)
    st1 = scat_s(y1t, t_row, t_col)
    t1, y2t = _tc_gcn_step(NM, st1[0, :NM], st1[1, :NM], y1t, dt2)
    st2 = scat_s(y2t, t_row, t_col)
    t2, _ = _tc_gcn_step(NM, st2[0, :NM], st2[1, :NM], y2t, dt2)
    si1 = scat_s(y1i, i_row, i_col)
    i1, y2i = _tc_gcn_step(NM, si1[0, :NM], si1[1, :NM], y1i, di2)
    si2 = scat_s(y2i, i_row, i_col)
    mean = _tc_combine4(NM, si2[0, :NM], si2[1, :NM], y2i, di2, t1, t2, i1)

    mean_pad = jnp.concatenate(
        [mean, jnp.zeros((NB_MOV - NM, EH), _f32)], axis=0)
    entA2 = _make_merge_kernel()(entA, mean_pad, mi_pad)[:N]
    entF = _tc_ent_mlp(entA2, ep1_w1, ep1_b1, ep1_w2, ep1_b2, ep2_w, ep2_b)

    e = _make_gather_kernel(BB * LE, HID)(entF, eids).reshape(BB, LE, HID)
    tt = _tc_tok_mlp(token_embeds, tp1_w1, tp1_b1, tp1_w2, tp1_b2, tp2_w, tp2_b)
    return _tc_attn(tt, e, cross_w)


# speedup vs baseline: 1.3197x; 1.0029x over previous
import functools

import jax
import jax.numpy as jnp
from jax import lax
from jax.experimental import pallas as pl
from jax.experimental.pallas import tpu as pltpu
from jax.experimental.pallas import tpu_sc as plsc

N = 10000
NM = 5000
EH = 128
NREL = 8
HID = 256
TOK = 768
BB, LE, LT = 16, 32, 64

NC, NS, LANE = 2, 16, 16
NW = NC * NS

NB_ENT = 10240
NB_MOV = 5120
NB_SEG = 80128
EP_KG = 327680
EP_C = 163840
EP_S = 81920
CHUNK = 128

_f32 = jnp.float32
_i32 = jnp.int32


def _mesh():
    return plsc.VectorSubcoreMesh(core_axis_name="c", subcore_axis_name="s")


_SC_PARAMS = pltpu.CompilerParams(needs_layout_passes=False)


def _fill_zero_2d(buf, nrows):
    def body(i, _):
        for c8 in range(EH // LANE):
            buf[i, pl.ds(c8 * LANE, LANE)] = jnp.zeros((LANE,), _f32)
        return 0
    lax.fori_loop(0, nrows, body, 0)


def _fill_const_1d(buf, n, val):
    def body(i, _):
        buf[pl.ds(i * LANE, LANE)] = jnp.full((LANE,), val, _f32)
        return 0
    lax.fori_loop(0, n // LANE, body, 0)


def _make_cnt_kernel():
    ZROWS = NB_SEG // NS

    @functools.partial(
        pl.kernel, mesh=_mesh(), compiler_params=_SC_PARAMS,
        out_type=(
            jax.ShapeDtypeStruct((NC * NB_SEG,), _f32),
            jax.ShapeDtypeStruct((NC * NB_ENT,), _f32),
            jax.ShapeDtypeStruct((NC * NB_MOV,), _f32),
            jax.ShapeDtypeStruct((NC * NB_MOV,), _f32),
        ),
        scratch_types=[
            pltpu.VMEM((ZROWS,), _f32),
            pltpu.VMEM((CHUNK,), _f32),
            pltpu.VMEM((CHUNK,), _i32), pltpu.VMEM((CHUNK,), _i32),
            pltpu.VMEM((CHUNK,), _i32), pltpu.VMEM((CHUNK,), _i32),
            pltpu.VMEM((CHUNK,), _i32), pltpu.VMEM((CHUNK,), _i32),
            pltpu.VMEM_SHARED((NB_SEG,), _f32),
            pltpu.VMEM_SHARED((NB_ENT,), _f32),
            pltpu.VMEM_SHARED((NB_MOV,), _f32),
            pltpu.VMEM_SHARED((NB_MOV,), _f32),
            pltpu.SemaphoreType.DMA, pltpu.SemaphoreType.DMA,
            pltpu.SemaphoreType.DMA, pltpu.SemaphoreType.DMA,
        ],
    )
    def k(kg_dst, kg_et, c_col, t_col, i_col,
          out_seg, out_c, out_t, out_i,
          zbuf, ones, dstb0, dstb1, etb0, etb1, segb0, segb1,
          b_seg, b_c, b_t, b_i, semi0, semi1, sems0, sems1):
        cid = lax.axis_index("c")
        sid = lax.axis_index("s")
        wid = sid * NC + cid
        dstb = (dstb0, dstb1)
        etb = (etb0, etb1)
        segb = (segb0, segb1)
        semi = (semi0, semi1)
        sems = (sems0, sems1)
        _fill_const_1d(zbuf, ZROWS, 0.0)
        _fill_const_1d(ones, CHUNK, 1.0)
        for bins, nb in ((b_seg, NB_SEG), (b_c, NB_ENT), (b_t, NB_MOV), (b_i, NB_MOV)):
            per = nb // NS
            pltpu.sync_copy(zbuf.at[pl.ds(0, per)], bins.at[pl.ds(sid * per, per)])
        plsc.subcore_barrier()

        def run_phase(idx_hbms, bins, ep, with_seg):
            nch = ep // NW // CHUNK
            nh = nch // 2

            def fire_idx(t, b):
                eb = (wid + t * NW) * CHUNK
                for h, dst in zip(idx_hbms, (dstb[b], etb[b])):
                    pltpu.async_copy(h.at[pl.ds(eb, CHUNK)], dst, semi[b])

            def wait_idx(b):
                for h, dst in zip(idx_hbms, (dstb[b], etb[b])):
                    pltpu.make_async_copy(h.at[pl.ds(0, CHUNK)], dst,
                                          semi[b]).wait()

            def half(i, b):
                wait_idx(b)
                if with_seg:
                    for c8 in range(CHUNK // LANE):
                        sl = pl.ds(c8 * LANE, LANE)
                        segb[b][sl] = dstb[b][sl] * NREL + etb[b][sl]
                    key = segb[b]
                else:
                    key = dstb[b]
                d = pltpu.async_copy(ones, bins.at[key], sems[b], add=True)
                d.wait()
                @pl.when(i < nh - 1)
                def _():
                    fire_idx(2 * i + 2 + b, b)

            fire_idx(0, 0)
            fire_idx(1, 1)
            def body(i, _):
                half(i, 0)
                half(i, 1)
                return 0
            lax.fori_loop(0, nh, body, 0)

        run_phase((kg_dst, kg_et), b_seg, EP_KG, True)
        run_phase((c_col,), b_c, EP_C, False)
        run_phase((t_col,), b_t, EP_S, False)
        run_phase((i_col,), b_i, EP_S, False)
        plsc.subcore_barrier()
        for bins, out, nb in ((b_seg, out_seg, NB_SEG), (b_c, out_c, NB_ENT),
                              (b_t, out_t, NB_MOV), (b_i, out_i, NB_MOV)):
            per = nb // NS
            pltpu.sync_copy(bins.at[pl.ds(sid * per, per)], zbuf.at[pl.ds(0, per)])
            pltpu.sync_copy(zbuf.at[pl.ds(0, per)],
                            out.at[pl.ds(cid * nb + sid * per, per)])
    return k


def _make_scale_kernel():
    ETILE = EP_KG // NW
    NCHUNK = ETILE // CHUNK

    @functools.partial(
        pl.kernel, mesh=_mesh(), compiler_params=_SC_PARAMS,
        out_type=(
            jax.ShapeDtypeStruct((EP_KG,), _f32),
            jax.ShapeDtypeStruct((EP_KG,), _i32),
        ),
        scratch_types=[
            pltpu.VMEM((NB_SEG,), _f32),
            pltpu.VMEM((CHUNK,), _i32), pltpu.VMEM((CHUNK,), _i32),
            pltpu.VMEM((CHUNK,), _i32), pltpu.VMEM((CHUNK,), _i32),
            pltpu.VMEM((CHUNK,), _i32), pltpu.VMEM((CHUNK,), _i32),
            pltpu.VMEM((CHUNK,), _i32), pltpu.VMEM((CHUNK,), _i32),
            pltpu.VMEM((CHUNK,), _f32), pltpu.VMEM((CHUNK,), _f32),
            pltpu.SemaphoreType.DMA, pltpu.SemaphoreType.DMA,
            pltpu.SemaphoreType.DMA, pltpu.SemaphoreType.DMA,
        ],
    )
    def k(inv_hbm, src_hbm, dst_hbm, et_hbm, scale_out, gidx_out,
          invt, srcb0, srcb1, dstb0, dstb1, etb0, etb1, gidxb0, gidxb1,
          scaleb0, scaleb1, semi0, semi1, semo0, semo1):
        cid = lax.axis_index("c")
        sid = lax.axis_index("s")
        wid = sid * NC + cid
        srcb = (srcb0, srcb1)
        dstb = (dstb0, dstb1)
        etb = (etb0, etb1)
        gidxb = (gidxb0, gidxb1)
        scaleb = (scaleb0, scaleb1)
        semi = (semi0, semi1)
        semo = (semo0, semo1)
        pltpu.sync_copy(inv_hbm, invt)
        NH = NCHUNK // 2

        def fire_idx(t, b):
            eb = wid * ETILE + t * CHUNK
            pltpu.async_copy(src_hbm.at[pl.ds(eb, CHUNK)], srcb[b], semi[b])
            pltpu.async_copy(dst_hbm.at[pl.ds(eb, CHUNK)], dstb[b], semi[b])
            pltpu.async_copy(et_hbm.at[pl.ds(eb, CHUNK)], etb[b], semi[b])

        def wait_idx(b):
            for h, d in ((src_hbm, srcb[b]), (dst_hbm, dstb[b]),
                         (et_hbm, etb[b])):
                pltpu.make_async_copy(h.at[pl.ds(0, CHUNK)], d, semi[b]).wait()

        def wait_out(b):
            pltpu.make_async_copy(scaleb[b], scale_out.at[pl.ds(0, CHUNK)],
                                  semo[b]).wait()
            pltpu.make_async_copy(gidxb[b], gidx_out.at[pl.ds(0, CHUNK)],
                                  semo[b]).wait()

        def half(i, b):
            t = 2 * i + b
            wait_idx(b)
            @pl.when(i > 0)
            def _():
                wait_out(b)
            for c8 in range(CHUNK // LANE):
                sl = pl.ds(c8 * LANE, LANE)
                e16 = etb[b][sl]
                gidxb[b][sl] = e16 * N + srcb[b][sl]
                scaleb[b][sl] = plsc.load_gather(
                    invt, [dstb[b][sl] * NREL + e16])
            eb = wid * ETILE + t * CHUNK
            pltpu.async_copy(scaleb[b], scale_out.at[pl.ds(eb, CHUNK)], semo[b])
            pltpu.async_copy(gidxb[b], gidx_out.at[pl.ds(eb, CHUNK)], semo[b])
            @pl.when(i < NH - 1)
            def _():
                fire_idx(t + 2, b)

        fire_idx(0, 0)
        fire_idx(1, 1)
        def body(i, _):
            half(i, 0)
            half(i, 1)
            return 0
        lax.fori_loop(0, NH, body, 0)
        wait_out(0)
        wait_out(1)
    return k


def _make_rgcn_kernel():
    ETILE = EP_KG // NW
    NCHUNK = ETILE // CHUNK
    RPT = NB_ENT // NS
    ZR = 64

    NH = NCHUNK // 2

    @functools.partial(
        pl.kernel, mesh=_mesh(), compiler_params=_SC_PARAMS,
        out_type=jax.ShapeDtypeStruct((NC, NB_ENT, EH), _f32),
        scratch_types=[
            pltpu.VMEM((ZR, EH), _f32),
            pltpu.VMEM((CHUNK,), _i32), pltpu.VMEM((CHUNK,), _i32),
            pltpu.VMEM((CHUNK,), _i32), pltpu.VMEM((CHUNK,), _i32),
            pltpu.VMEM((CHUNK,), _f32), pltpu.VMEM((CHUNK,), _f32),
            pltpu.VMEM((CHUNK, EH), _f32), pltpu.VMEM((CHUNK, EH), _f32),
            pltpu.VMEM_SHARED((NB_ENT, EH), _f32),
            pltpu.SemaphoreType.DMA, pltpu.SemaphoreType.DMA,
            pltpu.SemaphoreType.DMA, pltpu.SemaphoreType.DMA,
            pltpu.SemaphoreType.DMA, pltpu.SemaphoreType.DMA,
        ],
    )
    def k(xr_hbm, scale_hbm, gidx_hbm, dst_hbm, out_hbm,
          zbuf, dstb0, dstb1, gidxb0, gidxb1, scaleb0, scaleb1, rows0, rows1,
          bins, semi0, semi1, semg0, semg1, sems0, sems1):
        cid = lax.axis_index("c")
        sid = lax.axis_index("s")
        wid = sid * NC + cid
        base = wid * ETILE
        dstb = (dstb0, dstb1)
        gidxb = (gidxb0, gidxb1)
        scaleb = (scaleb0, scaleb1)
        rows = (rows0, rows1)
        semi = (semi0, semi1)
        semg = (semg0, semg1)
        _fill_zero_2d(zbuf, ZR)
        for j in range(RPT // ZR):
            pltpu.async_copy(zbuf, bins.at[pl.ds(sid * RPT + j * ZR, ZR)],
                             semi0)
        for j in range(RPT // ZR):
            pltpu.make_async_copy(zbuf, bins.at[pl.ds(0, ZR)], semi0).wait()
        plsc.subcore_barrier()

        def fire_idx(t, b):
            eb = (wid + t * NW) * CHUNK
            pltpu.async_copy(dst_hbm.at[pl.ds(eb, CHUNK)], dstb[b], semi[b])
            pltpu.async_copy(gidx_hbm.at[pl.ds(eb, CHUNK)], gidxb[b], semi[b])
            pltpu.async_copy(scale_hbm.at[pl.ds(eb, CHUNK)], scaleb[b], semi[b])

        def wait_idx(b):
            pltpu.make_async_copy(dst_hbm.at[pl.ds(0, CHUNK)], dstb[b],
                                  semi[b]).wait()
            pltpu.make_async_copy(gidx_hbm.at[pl.ds(0, CHUNK)], gidxb[b],
                                  semi[b]).wait()
            pltpu.make_async_copy(scale_hbm.at[pl.ds(0, CHUNK)], scaleb[b],
                                  semi[b]).wait()

        def fire_gather(b):
            pltpu.async_copy(xr_hbm.at[gidxb[b]], rows[b], semg[b])

        def wait_gather(b):
            pltpu.make_async_copy(xr_hbm.at[gidxb[b]], rows[b], semg[b]).wait()

        def scale_rows(b):
            def mul_body(j, _):
                splat = plsc.load_gather(
                    scaleb[b], [jnp.broadcast_to(j, (LANE,))])
                for c8 in range(EH // LANE):
                    sl = pl.ds(c8 * LANE, LANE)
                    rows[b][j, sl] = rows[b][j, sl] * splat
                return 0
            lax.fori_loop(0, CHUNK, mul_body, 0)

        fire_idx(0, 0)
        fire_idx(1, 1)
        wait_idx(0)
        fire_gather(0)

        def body(i, _):
            wait_idx(1)
            fire_gather(1)
            wait_gather(0)
            scale_rows(0)
            d = pltpu.async_copy(rows0, bins.at[dstb0], sems0, add=True)
            d.wait()
            @pl.when(i < NH - 1)
            def _():
                fire_idx(2 * i + 2, 0)
                wait_idx(0)
                fire_gather(0)
            wait_gather(1)
            scale_rows(1)
            d1 = pltpu.async_copy(rows1, bins.at[dstb1], sems1, add=True)
            d1.wait()
            @pl.when(i < NH - 1)
            def _():
                fire_idx(2 * i + 3, 1)
            return 0
        lax.fori_loop(0, NH, body, 0)
        plsc.subcore_barrier()
        FCH = 64
        nfl = RPT // FCH
        rowsbufs = (rows0, rows1)
        semg = (semg0, semg1)
        sems = (sems0, sems1)
        def fl_load(j, b):
            pltpu.async_copy(bins.at[pl.ds(sid * RPT + j * FCH, FCH)],
                             rowsbufs[b].at[pl.ds(0, FCH)], semg[b])
        def fl_wait_load(b):
            pltpu.make_async_copy(bins.at[pl.ds(0, FCH)],
                                  rowsbufs[b].at[pl.ds(0, FCH)],
                                  semg[b]).wait()
        def fl_store(j, b):
            pltpu.async_copy(rowsbufs[b].at[pl.ds(0, FCH)],
                             out_hbm.at[cid, pl.ds(sid * RPT + j * FCH, FCH)],
                             sems[b])
        def fl_wait_store(b):
            pltpu.make_async_copy(rowsbufs[b].at[pl.ds(0, FCH)],
                                  out_hbm.at[cid, pl.ds(0, FCH)],
                                  sems[b]).wait()
        fl_load(0, 0)
        for j in range(nfl):
            b = j % 2
            fl_wait_load(b)
            fl_store(j, b)
            if j + 1 < nfl:
                if j >= 1:
                    fl_wait_store(1 - b)
                fl_load(j + 1, 1 - b)
        fl_wait_store((nfl - 1) % 2)
        if nfl >= 2:
            fl_wait_store(nfl % 2)
    return k


def _make_scat_kernel(ep, nbins, stage_y=False, ny=0):
    ETILE = ep // NW
    NCHUNK = ETILE // CHUNK
    RPT = nbins // NS
    ZR = 64

    NH = NCHUNK // 2
    ytab_scratch = [pltpu.VMEM_SHARED((nbins, EH), _f32)] if stage_y else []

    @functools.partial(
        pl.kernel, mesh=_mesh(), compiler_params=_SC_PARAMS,
        out_type=jax.ShapeDtypeStruct((NC, nbins, EH), _f32),
        scratch_types=[
            pltpu.VMEM((ZR, EH), _f32),
            pltpu.VMEM((CHUNK,), _i32), pltpu.VMEM((CHUNK,), _i32),
            pltpu.VMEM((CHUNK,), _i32), pltpu.VMEM((CHUNK,), _i32),
            pltpu.VMEM((CHUNK, EH), _f32), pltpu.VMEM((CHUNK, EH), _f32),
            pltpu.VMEM_SHARED((nbins, EH), _f32),
        ] + ytab_scratch + [
            pltpu.SemaphoreType.DMA, pltpu.SemaphoreType.DMA,
            pltpu.SemaphoreType.DMA, pltpu.SemaphoreType.DMA,
            pltpu.SemaphoreType.DMA, pltpu.SemaphoreType.DMA,
        ],
    )
    def k(y_hbm, row_hbm, col_hbm, out_hbm, zbuf, rowb0, rowb1, colb0, colb1,
          rows0, rows1, bins, *rest):
        if stage_y:
            ytab = rest[0]
            semi0, semi1, semg0, semg1, sems0, sems1 = rest[1:]
        else:
            ytab = y_hbm
            semi0, semi1, semg0, semg1, sems0, sems1 = rest
        cid = lax.axis_index("c")
        sid = lax.axis_index("s")
        wid = sid * NC + cid
        base = wid * ETILE
        rowb = (rowb0, rowb1)
        colb = (colb0, colb1)
        rows = (rows0, rows1)
        semi = (semi0, semi1)
        semg = (semg0, semg1)
        sems = (sems0, sems1)
        _fill_zero_2d(zbuf, ZR)
        for j in range(RPT // ZR):
            pltpu.async_copy(zbuf, bins.at[pl.ds(sid * RPT + j * ZR, ZR)],
                             semi0)
        for j in range(RPT // ZR):
            pltpu.make_async_copy(zbuf, bins.at[pl.ds(0, ZR)], semi0).wait()
        plsc.subcore_barrier()

        def fire_idx(t, b):
            eb = (wid + t * NW) * CHUNK
            pltpu.async_copy(row_hbm.at[pl.ds(eb, CHUNK)], rowb[b], semi[b])
            pltpu.async_copy(col_hbm.at[pl.ds(eb, CHUNK)], colb[b], semi[b])

        def wait_idx(b):
            pltpu.make_async_copy(row_hbm.at[pl.ds(0, CHUNK)], rowb[b],
                                  semi[b]).wait()
            pltpu.make_async_copy(col_hbm.at[pl.ds(0, CHUNK)], colb[b],
                                  semi[b]).wait()

        def fire_gather(b):
            pltpu.async_copy(ytab.at[rowb[b]], rows[b], semg[b])

        def wait_gather(b):
            pltpu.make_async_copy(ytab.at[rowb[b]], rows[b], semg[b]).wait()

        if stage_y:
            nfull = ny // 80
            tail = ny - nfull * 80
            def stage(i, _):
                ci = sid + i * NS
                @pl.when(ci < nfull)
                def _():
                    r0 = ci * 80
                    pltpu.sync_copy(y_hbm.at[pl.ds(r0, 80)],
                                    rows0.at[pl.ds(0, 80)])
                    pltpu.sync_copy(rows0.at[pl.ds(0, 80)],
                                    ytab.at[pl.ds(r0, 80)])
                return 0
            lax.fori_loop(0, (nfull + NS - 1) // NS, stage, 0)
            if tail:
                @pl.when(sid == NS - 1)
                def _():
                    r0 = nfull * 80
                    pltpu.sync_copy(y_hbm.at[pl.ds(r0, tail)],
                                    rows1.at[pl.ds(0, tail)])
                    pltpu.sync_copy(rows1.at[pl.ds(0, tail)],
                                    ytab.at[pl.ds(r0, tail)])
            plsc.subcore_barrier()

        fire_idx(0, 0)
        fire_idx(1, 1)
        wait_idx(0)
        fire_gather(0)

        def body(i, _):
            wait_idx(1)
            fire_gather(1)
            wait_gather(0)
            d = pltpu.async_copy(rows0, bins.at[colb0], sems0, add=True)
            d.wait()
            @pl.when(i < NH - 1)
            def _():
                fire_idx(2 * i + 2, 0)
            @pl.when(i < NH - 1)
            def _():
                wait_idx(0)
                fire_gather(0)
            wait_gather(1)
            d1 = pltpu.async_copy(rows1, bins.at[colb1], sems1, add=True)
            d1.wait()
            @pl.when(i < NH - 1)
            def _():
                fire_idx(2 * i + 3, 1)
            return 0
        lax.fori_loop(0, NH, body, 0)
        plsc.subcore_barrier()
        FCH = 64
        nfl = RPT // FCH
        rowsbufs = (rows0, rows1)
        def fl_load(j, b):
            pltpu.async_copy(bins.at[pl.ds(sid * RPT + j * FCH, FCH)],
                             rowsbufs[b].at[pl.ds(0, FCH)], semg[b])
        def fl_wait_load(b):
            pltpu.make_async_copy(bins.at[pl.ds(0, FCH)],
                                  rowsbufs[b].at[pl.ds(0, FCH)],
                                  semg[b]).wait()
        def fl_store(j, b):
            pltpu.async_copy(rowsbufs[b].at[pl.ds(0, FCH)],
                             out_hbm.at[cid, pl.ds(sid * RPT + j * FCH, FCH)],
                             sems[b])
        def fl_wait_store(b):
            pltpu.make_async_copy(rowsbufs[b].at[pl.ds(0, FCH)],
                                  out_hbm.at[cid, pl.ds(0, FCH)],
                                  sems[b]).wait()
        fl_load(0, 0)
        for j in range(nfl):
            b = j % 2
            fl_wait_load(b)
            fl_store(j, b)
            if j + 1 < nfl:
                if j >= 1:
                    fl_wait_store(1 - b)
                fl_load(j + 1, 1 - b)
        fl_wait_store((nfl - 1) % 2)
        if nfl >= 2:
            fl_wait_store(nfl % 2)
    return k


def _make_gather_kernel(ni, feat):
    KPT = ni // NW
    TR = next(t for t in (128, 80, 64, 32, 16, 8) if KPT % t == 0 and t <= KPT)
    NT = KPT // TR

    @functools.partial(
        pl.kernel, mesh=_mesh(), compiler_params=_SC_PARAMS,
        out_type=jax.ShapeDtypeStruct((ni, feat), _f32),
        scratch_types=[
            pltpu.VMEM((KPT,), _i32),
            pltpu.VMEM((TR, feat), _f32),
            pltpu.SemaphoreType.DMA,
        ],
    )
    def k(table_hbm, idx_hbm, out_hbm, idxb, rows, sem):
        cid = lax.axis_index("c")
        sid = lax.axis_index("s")
        wid = sid * NC + cid
        base = wid * KPT
        pltpu.sync_copy(idx_hbm.at[pl.ds(base, KPT)], idxb)
        for j in range(NT):
            pltpu.async_copy(table_hbm.at[idxb.at[pl.ds(j * TR, TR)]], rows,
                             sem).wait()
            pltpu.sync_copy(rows, out_hbm.at[pl.ds(base + j * TR, TR)])
    return k


def _make_merge_kernel():
    KPT = NB_MOV // NW
    TR = 32
    NT = KPT // TR
    HALF = NB_ENT // 2
    TRASH = NB_ENT - 8

    @functools.partial(
        pl.kernel, mesh=_mesh(), compiler_params=_SC_PARAMS,
        out_type=jax.ShapeDtypeStruct((NB_ENT, EH), _f32),
        scratch_types=[
            pltpu.VMEM((KPT,), _i32),
            pltpu.VMEM((NT, TR), _i32),
            pltpu.VMEM((KPT, EH), _f32),
            pltpu.VMEM((TR, EH), _f32), pltpu.VMEM((TR, EH), _f32),
            pltpu.VMEM((80, EH), _f32), pltpu.VMEM((80, EH), _f32),
            pltpu.SemaphoreType.DMA, pltpu.SemaphoreType.DMA,
            pltpu.SemaphoreType.DMA, pltpu.SemaphoreType.DMA,
            pltpu.SemaphoreType.DMA,
        ],
    )
    def k(entA_hbm, mean_hbm, mi_hbm, out_hbm, mib, rib, meanb, rows0, rows1,
          cbuf0, cbuf1, semg0, semg1, semc0, semc1, semm):
        cid = lax.axis_index("c")
        sid = lax.axis_index("s")
        wid = sid * NC + cid
        base = wid * KPT
        rows = (rows0, rows1)
        cbuf = (cbuf0, cbuf1)
        semg = (semg0, semg1)
        semc = (semc0, semc1)
        pltpu.async_copy(mi_hbm.at[pl.ds(base, KPT)], mib, semm)
        pltpu.async_copy(mean_hbm.at[pl.ds(base, KPT)], meanb, semm)
        nch = jnp.where(cid == 0, 64, 61)
        def cp_load(i, b):
            ci = sid + i * NS
            @pl.when(ci < nch)
            def _():
                r0 = cid * HALF + ci * 80
                pltpu.async_copy(entA_hbm.at[pl.ds(r0, 80)], cbuf[b], semc[b])
        def cp_flush(i, b):
            ci = sid + i * NS
            @pl.when(ci < nch)
            def _():
                r0 = cid * HALF + ci * 80
                pltpu.make_async_copy(entA_hbm.at[pl.ds(0, 80)], cbuf[b],
                                      semc[b]).wait()
                d = pltpu.async_copy(cbuf[b], out_hbm.at[pl.ds(r0, 80)],
                                     semc[b])
                d.wait()
        cp_load(0, 0)
        for i in range(4):
            if i + 1 < 4:
                cp_load(i + 1, 1 - (i % 2))
            cp_flush(i, i % 2)
        pltpu.make_async_copy(mi_hbm.at[pl.ds(0, KPT)], mib, semm).wait()
        pltpu.make_async_copy(mean_hbm.at[pl.ds(0, KPT)], meanb, semm).wait()
        plsc.subcore_barrier()
        for c8 in range(KPT // LANE):
            sl = pl.ds((c8 % (TR // LANE)) * LANE, LANE)
            m16 = mib[pl.ds(c8 * LANE, LANE)]
            mine = jnp.where(cid == 0, m16 < HALF, m16 >= HALF)
            rib[c8 // (TR // LANE), sl] = jnp.where(mine, m16, TRASH)
        def up_gather(j, b):
            pltpu.async_copy(entA_hbm.at[mib.at[pl.ds(j * TR, TR)]], rows[b],
                             semg[b])
        up_gather(0, 0)
        scat_descs = [None, None]
        for j in range(NT):
            b = j % 2
            pltpu.make_async_copy(entA_hbm.at[mib.at[pl.ds(0, TR)]], rows[b],
                                  semg[b]).wait()
            if j + 1 < NT:
                if scat_descs[1 - b] is not None:
                    scat_descs[1 - b].wait()
                up_gather(j + 1, 1 - b)
            def addrow(i, _):
                for c8 in range(EH // LANE):
                    sl = pl.ds(c8 * LANE, LANE)
                    rows[b][i, sl] = rows[b][i, sl] + meanb[j * TR + i, sl]
                return 0
            lax.fori_loop(0, TR, addrow, 0)
            scat_descs[b] = pltpu.async_copy(rows[b], out_hbm.at[rib.at[j]],
                                             semc[b])
        scat_descs[0].wait()
        scat_descs[1].wait()
    return k


def _dot(a, b):
    return jnp.dot(a, b, preferred_element_type=_f32)


def _tc_xr(node, basis, comp):
    def body(comp_ref, basis_ref, x_ref, o_ref):
        r = pl.program_id(0)
        w = comp_ref[r, 0] * basis_ref[0]
        for b in range(1, NREL):
            w = w + comp_ref[r, b] * basis_ref[b]
        o_ref[0] = _dot(x_ref[...], w)
    return pl.pallas_call(
        body,
        grid=(NREL,),
        in_specs=[
            pl.BlockSpec((NREL, NREL), lambda r: (0, 0)),
            pl.BlockSpec((NREL, EH, EH), lambda r: (0, 0, 0)),
            pl.BlockSpec((N, EH), lambda r: (0, 0)),
        ],
        out_specs=pl.BlockSpec((1, N, EH), lambda r: (r, 0, 0)),
        out_shape=jax.ShapeDtypeStruct((NREL, N, EH), _f32),
    )(comp, basis, node)


def _tc_transform_counts(cnt_seg, cnt_c, cnt_t, cnt_i):
    def body(s_ref, c_ref, t_ref, i_ref, inv_ref, dc_ref, dt_ref, di_ref):
        s = s_ref[0] + s_ref[1]
        inv_ref[...] = 1.0 / jnp.maximum(s, 1.0)
        dc_ref[...] = lax.rsqrt(c_ref[0] + c_ref[1] + 1.0)
        dt_ref[...] = lax.rsqrt(t_ref[0] + t_ref[1] + 1.0)
        di_ref[...] = lax.rsqrt(i_ref[0] + i_ref[1] + 1.0)
    r = lambda a: a.reshape(NC, -1, 128)
    outs = pl.pallas_call(
        body,
        out_shape=(
            jax.ShapeDtypeStruct((NB_SEG // 128, 128), _f32),
            jax.ShapeDtypeStruct((NB_ENT // 128, 128), _f32),
            jax.ShapeDtypeStruct((NB_MOV // 128, 128), _f32),
            jax.ShapeDtypeStruct((NB_MOV // 128, 128), _f32),
        ),
    )(r(cnt_seg), r(cnt_c), r(cnt_t), r(cnt_i))
    inv, dc, dt, di = outs
    return (inv.reshape(-1), dc.reshape(-1)[:N], dt.reshape(-1)[:NM],
            di.reshape(-1)[:NM])


def _tc_ent0(node, p0, p1, root_w, root_b, dinv_c):
    def body(x_ref, p0_ref, p1_ref, w_ref, b_ref, d_ref, e_ref, y_ref):
        e = p0_ref[...] + p1_ref[...] + _dot(x_ref[...], w_ref[...]) \
            + b_ref[...] + x_ref[...]
        e_ref[...] = e
        y_ref[...] = e * d_ref[...]
    blk = 2000
    return pl.pallas_call(
        body,
        grid=(N // blk,),
        in_specs=[
            pl.BlockSpec((blk, EH), lambda i: (i, 0)),
            pl.BlockSpec((blk, EH), lambda i: (i, 0)),
            pl.BlockSpec((blk, EH), lambda i: (i, 0)),
            pl.BlockSpec((EH, EH), lambda i: (0, 0)),
            pl.BlockSpec((1, EH), lambda i: (0, 0)),
            pl.BlockSpec((blk, 1), lambda i: (i, 0)),
        ],
        out_specs=[pl.BlockSpec((blk, EH), lambda i: (i, 0))] * 2,
        out_shape=[jax.ShapeDtypeStruct((N, EH), _f32)] * 2,
    )(node, p0, p1, root_w, root_b[None], dinv_c)


def _tc_gcn_step(n, s0, s1, y, dinv2):
    def body(s0_ref, s1_ref, y_ref, d_ref, o_ref, yn_ref):
        d = d_ref[...]
        o = d * (s0_ref[...] + s1_ref[...] + y_ref[...])
        o_ref[...] = o
        yn_ref[...] = d * o
    blk = 2000 if n == N else 1000
    return pl.pallas_call(
        body,
        grid=(n // blk,),
        in_specs=[pl.BlockSpec((blk, EH), lambda i: (i, 0))] * 3
        + [pl.BlockSpec((blk, 1), lambda i: (i, 0))],
        out_specs=[pl.BlockSpec((blk, EH), lambda i: (i, 0))] * 2,
        out_shape=[jax.ShapeDtypeStruct((n, EH), _f32)] * 2,
    )(s0, s1, y, dinv2)


def _tc_combine4(n, s0, s1, y, dinv2, a, b, c):
    def body(s0_ref, s1_ref, y_ref, d_ref, a_ref, b_ref, c_ref, o_ref):
        last = d_ref[...] * (s0_ref[...] + s1_ref[...] + y_ref[...])
        o_ref[...] = (a_ref[...] + b_ref[...] + c_ref[...] + last) * 0.25
    blk = 2000 if n == N else 1000
    return pl.pallas_call(
        body,
        grid=(n // blk,),
        in_specs=[pl.BlockSpec((blk, EH), lambda i: (i, 0))] * 3
        + [pl.BlockSpec((blk, 1), lambda i: (i, 0))]
        + [pl.BlockSpec((blk, EH), lambda i: (i, 0))] * 3,
        out_specs=pl.BlockSpec((blk, EH), lambda i: (i, 0)),
        out_shape=jax.ShapeDtypeStruct((n, EH), _f32),
    )(s0, s1, y, dinv2, a, b, c)


def _tc_prep_ti(nf, dt2, di2):
    def body(x_ref, dt_ref, di_ref, yt_ref, yi_ref):
        yt_ref[...] = x_ref[...] * dt_ref[...]
        yi_ref[...] = x_ref[...] * di_ref[...]
    blk = 1000
    return pl.pallas_call(
        body,
        grid=(NM // blk,),
        in_specs=[pl.BlockSpec((blk, EH), lambda i: (i, 0)),
                  pl.BlockSpec((blk, 1), lambda i: (i, 0)),
                  pl.BlockSpec((blk, 1), lambda i: (i, 0))],
        out_specs=[pl.BlockSpec((blk, EH), lambda i: (i, 0))] * 2,
        out_shape=[jax.ShapeDtypeStruct((NM, EH), _f32)] * 2,
    )(nf, dt2, di2)


def _tc_ent_mlp(entA, w1, b1, w2, b2, ew, eb):
    def body(x_ref, w1_ref, b1_ref, w2_ref, b2_ref, ew_ref, eb_ref, o_ref):
        x = x_ref[...]
        h = jnp.maximum(_dot(x, w1_ref[...]) + b1_ref[...], 0.0)
        m = _dot(h, w2_ref[...]) + b2_ref[...] + x
        o_ref[...] = _dot(m, ew_ref[...]) + eb_ref[...]
    blk = 2000
    return pl.pallas_call(
        body,
        grid=(N // blk,),
        in_specs=[
            pl.BlockSpec((blk, EH), lambda i: (i, 0)),
            pl.BlockSpec((EH, EH // 2), lambda i: (0, 0)),
            pl.BlockSpec((1, EH // 2), lambda i: (0, 0)),
            pl.BlockSpec((EH // 2, EH), lambda i: (0, 0)),
            pl.BlockSpec((1, EH), lambda i: (0, 0)),
            pl.BlockSpec((EH, HID), lambda i: (0, 0)),
            pl.BlockSpec((1, HID), lambda i: (0, 0)),
        ],
        out_specs=pl.BlockSpec((blk, HID), lambda i: (i, 0)),
        out_shape=jax.ShapeDtypeStruct((N, HID), _f32),
    )(entA, w1, b1[None], w2, b2[None], ew, eb[None])


def _tc_tok_mlp(tok, w1, b1, w2, b2, tw, tb):
    def body(t_ref, w1_ref, b1_ref, w2_ref, b2_ref, tw_ref, tb_ref, o_ref):
        t = t_ref[0]
        h = jnp.maximum(_dot(t, w1_ref[...]) + b1_ref[...], 0.0)
        m = _dot(h, w2_ref[...]) + b2_ref[...] + t
        o_ref[0] = _dot(m, tw_ref[...]) + tb_ref[...]
    return pl.pallas_call(
        body,
        grid=(BB,),
        in_specs=[
            pl.BlockSpec((1, LT, TOK), lambda i: (i, 0, 0)),
            pl.BlockSpec((TOK, TOK // 2), lambda i: (0, 0)),
            pl.BlockSpec((1, TOK // 2), lambda i: (0, 0)),
            pl.BlockSpec((TOK // 2, TOK), lambda i: (0, 0)),
            pl.BlockSpec((1, TOK), lambda i: (0, 0)),
            pl.BlockSpec((TOK, HID), lambda i: (0, 0)),
            pl.BlockSpec((1, HID), lambda i: (0, 0)),
        ],
        out_specs=pl.BlockSpec((1, LT, HID), lambda i: (i, 0, 0)),
        out_shape=jax.ShapeDtypeStruct((BB, LT, HID), _f32),
    )(tok, w1, b1[None], w2, b2[None], tw, tb[None])


def _tc_attn(tt, e, cross_w):
    def body(t_ref, e_ref, w_ref, o_ref):
        t = t_ref[0]
        e2 = e_ref[0]
        q = _dot(t, w_ref[...])
        a = lax.dot_general(q, e2, (((1,), (1,)), ((), ())),
                            preferred_element_type=_f32) * (1.0 / HID)
        a = a - jnp.max(a, axis=1, keepdims=True)
        ex = jnp.exp(a)
        ew = ex / jnp.sum(ex, axis=1, keepdims=True)
        o_ref[0] = _dot(ew, e2) + t
    return pl.pallas_call(
        body,
        grid=(BB,),
        in_specs=[
            pl.BlockSpec((1, LT, HID), lambda i: (i, 0, 0)),
            pl.BlockSpec((1, LE, HID), lambda i: (i, 0, 0)),
            pl.BlockSpec((HID, HID), lambda i: (0, 0)),
        ],
        out_specs=pl.BlockSpec((1, LT, HID), lambda i: (i, 0, 0)),
        out_shape=jax.ShapeDtypeStruct((BB, LT, HID), _f32),
    )(tt, e, cross_w)


def _pad1(a, n, val):
    return jnp.concatenate([a, jnp.full((n - a.shape[0],), val, a.dtype)])


def _pad_spread(a, n, lo, nspread):
    npad = n - a.shape[0]
    pad = lo + (jnp.arange(npad, dtype=a.dtype) % nspread)
    return jnp.concatenate([a, pad])


def kernel(node_embeds, basis, comp, root_w, root_b, ep1_w1, ep1_b1, ep1_w2,
           ep1_b2, ep2_w, ep2_b, tp1_w1, tp1_b1, tp1_w2, tp1_b2, tp2_w, tp2_b,
           cross_w, token_embeds, entity_ids, edge_index, edge_type,
           edge_index_c, edge_index_t_s, edge_index_i_s, movie_indices):
    kg_src = _pad1(edge_index[0], EP_KG, 0)
    kg_dst = _pad_spread(edge_index[1], EP_KG, N, 16)
    kg_et = _pad1(edge_type, EP_KG, 0)
    c_row = _pad1(edge_index_c[0], EP_C, 0)
    c_col = _pad_spread(edge_index_c[1], EP_C, N, NB_ENT - N)
    t_row = _pad1(edge_index_t_s[0], EP_S, 0)
    t_col = _pad_spread(edge_index_t_s[1], EP_S, NM, NB_MOV - NM)
    i_row = _pad1(edge_index_i_s[0], EP_S, 0)
    i_col = _pad_spread(edge_index_i_s[1], EP_S, NM, NB_MOV - NM)
    mi_pad = _pad1(movie_indices, NB_MOV, 0)
    eids = entity_ids.reshape(-1)

    cnt_seg, cnt_c, cnt_t, cnt_i = _make_cnt_kernel()(
        kg_dst, kg_et, c_col, t_col, i_col)
    inv, dinv_c, dinv_t, dinv_i = _tc_transform_counts(
        cnt_seg, cnt_c, cnt_t, cnt_i)
    dc2, dt2, di2 = dinv_c[:, None], dinv_t[:, None], dinv_i[:, None]

    xr = _tc_xr(node_embeds, basis, comp).reshape(NREL * N, EH)
    scale, gidx = _make_scale_kernel()(inv, kg_src, kg_dst, kg_et)
    rg = _make_rgcn_kernel()(xr, scale, gidx, kg_dst)
    ent0, y1c = _tc_ent0(node_embeds, rg[0, :N], rg[1, :N], root_w, root_b, dc2)

    scat_c = _make_scat_kernel(EP_C, NB_ENT)
    s1 = scat_c(y1c, c_row, c_col)
    c1, y2c = _tc_gcn_step(N, s1[0, :N], s1[1, :N], y1c, dc2)
    s2 = scat_c(y2c, c_row, c_col)
    c2, y3c = _tc_gcn_step(N, s2[0, :N], s2[1, :N], y2c, dc2)
    s3 = scat_c(y3c, c_row, c_col)
    entA = _tc_combine4(N, s3[0, :N], s3[1, :N], y3c, dc2, c1, c2, ent0)

    nf = _make_gather_kernel(NB_MOV, EH)(ent0, mi_pad)[:NM]
    y1t, y1i = _tc_prep_ti(nf, dt2, di2)
    scat_s = _make_scat_kernel(EP_S, NB_MOV, stage_y=True, ny=NM)
    st1 = scat_s(y1t, t_row, t_col)
    t1, y2t = _tc_gcn_step(NM, st1[0, :NM], st1[1, :NM], y1t, dt2)
    st2 = scat_s(y2t, t_row, t_col)
    t2, _ = _tc_gcn_step(NM, st2[0, :NM], st2[1, :NM], y2t, dt2)
    si1 = scat_s(y1i, i_row, i_col)
    i1, y2i = _tc_gcn_step(NM, si1[0, :NM], si1[1, :NM], y1i, di2)
    si2 = scat_s(y2i, i_row, i_col)
    mean = _tc_combine4(NM, si2[0, :NM], si2[1, :NM], y2i, di2, t1, t2, i1)

    mean_pad = jnp.concatenate(
        [mean, jnp.zeros((NB_MOV - NM, EH), _f32)], axis=0)
    entA2 = _make_merge_kernel()(entA, mean_pad, mi_pad)[:N]
    entF = _tc_ent_mlp(entA2, ep1_w1, ep1_b1, ep1_w2, ep1_b2, ep2_w, ep2_b)

    e = _make_gather_kernel(BB * LE, HID)(entF, eids).reshape(BB, LE, HID)
    tt = _tc_tok_mlp(token_embeds, tp1_w1, tp1_b1, tp1_w2, tp1_b2, tp2_w, tp2_b)
    return _tc_attn(tt, e, cross_w)
